# Initial kernel scaffold; baseline (speedup 1.0000x reference)
import jax
import jax.numpy as jnp
from jax.experimental import pallas as pl


def kernel(Q_info, edge_index, edge_type, q, y, diff, device, rgcn_weight, rgcn_root, rgcn_bias, emb_diff, emb_answer, W1, b1, W2, b2, W_ih, W_hh, b_ih, b_hh, W3, b3):
    raise NotImplementedError("write your pallas kernel here")


# speedup vs baseline: 6.1044x; 6.1044x over previous
import functools

import jax
import jax.numpy as jnp
from jax import lax
from jax.experimental import pallas as pl
from jax.experimental.pallas import tpu as pltpu
from jax.experimental.pallas import tpu_sc as plsc

_N = 10000
_R = 4
_D = 128
_E = 160000
_SEG = _N * _R
_NC = 2
_NS = 16
_NW = _NC * _NS
_EPW = _E // _NW
_Q = 20000
_BS = 1600
_H = 256
_RPW = _N // _NS


def _mesh():
  return plsc.VectorSubcoreMesh(
      core_axis_name="c", subcore_axis_name="s",
      num_cores=_NC, num_subcores=_NS)


@functools.partial(
    pl.kernel,
    out_type=jax.ShapeDtypeStruct((_NW, _SEG), jnp.float32),
    mesh=_mesh(),
    scratch_types=[
        pltpu.VMEM((_EPW,), jnp.int32),
        pltpu.VMEM((_EPW,), jnp.int32),
        pltpu.VMEM((_SEG,), jnp.float32),
    ],
    compiler_params=pltpu.CompilerParams(
        needs_layout_passes=False, use_tc_tiling_on_sc=False),
)
def _count_kernel(dst_hbm, rel_hbm, out_hbm, dst_v, comb_v, cnt_v):
  wid = lax.axis_index("s") * _NC + lax.axis_index("c")
  base = wid * _EPW
  pltpu.sync_copy(dst_hbm.at[pl.ds(base, _EPW)], dst_v)
  pltpu.sync_copy(rel_hbm.at[pl.ds(base, _EPW)], comb_v)

  def zbody(i, _):
    cnt_v[pl.ds(i * 16, 16)] = jnp.zeros((16,), jnp.float32)
    return 0
  lax.fori_loop(0, _SEG // 16, zbody, 0)

  def cbody(i, _):
    sl = pl.ds(i * 16, 16)
    comb_v[sl] = dst_v[sl] * _R + comb_v[sl]
    return 0
  lax.fori_loop(0, _EPW // 16, cbody, 0)

  def hbody(i, _):
    c16 = comb_v[pl.ds(i * 16, 16)]
    cnts, lastm = plsc.scan_count(c16)
    plsc.addupdate_scatter(cnt_v, [c16], cnts.astype(jnp.float32), mask=lastm)
    return 0
  lax.fori_loop(0, _EPW // 16, hbody, 0)

  pltpu.sync_copy(cnt_v, out_hbm.at[wid])


def _inv_tc(cnt_parts):
  def body(c_ref, o_ref):
    s = jnp.sum(c_ref[...], axis=0, keepdims=True)
    o_ref[...] = 1.0 / jnp.maximum(s, 1.0)

  out = pl.pallas_call(
      body,
      out_shape=jax.ShapeDtypeStruct((1, _SEG), jnp.float32),
  )(cnt_parts)
  return out.reshape(_SEG)


@functools.partial(
    pl.kernel,
    out_type=jax.ShapeDtypeStruct((_E,), jnp.float32),
    mesh=_mesh(),
    scratch_types=[
        pltpu.VMEM((_EPW,), jnp.int32),
        pltpu.VMEM((_EPW,), jnp.int32),
        pltpu.VMEM((_SEG,), jnp.float32),
        pltpu.VMEM((_EPW,), jnp.float32),
    ],
    compiler_params=pltpu.CompilerParams(
        needs_layout_passes=False, use_tc_tiling_on_sc=False),
)
def _scale_kernel(dst_hbm, rel_hbm, inv_hbm, out_hbm, dst_v, rel_v, inv_v,
                  s_v):
  wid = lax.axis_index("s") * _NC + lax.axis_index("c")
  base = wid * _EPW
  pltpu.sync_copy(dst_hbm.at[pl.ds(base, _EPW)], dst_v)
  pltpu.sync_copy(rel_hbm.at[pl.ds(base, _EPW)], rel_v)
  pltpu.sync_copy(inv_hbm, inv_v)

  def body(i, _):
    sl = pl.ds(i * 16, 16)
    comb = dst_v[sl] * _R + rel_v[sl]
    s_v[sl] = plsc.load_gather(inv_v, [comb])
    return 0
  lax.fori_loop(0, _EPW // 16, body, 0)

  pltpu.sync_copy(s_v, out_hbm.at[pl.ds(base, _EPW)])


_CH = 128
_NFULL = (_EPW // _CH)
_TAIL = _EPW - _NFULL * _CH


@functools.partial(
    pl.kernel,
    out_type=jax.ShapeDtypeStruct((_NC, _N, _D), jnp.float32),
    mesh=_mesh(),
    scratch_types=[
        pltpu.VMEM((_EPW,), jnp.int32),
        pltpu.VMEM((_EPW,), jnp.int32),
        pltpu.VMEM((_EPW,), jnp.int32),
        pltpu.VMEM((_EPW + 16,), jnp.float32),
        pltpu.VMEM((1, _CH), jnp.int32),
        pltpu.VMEM((1, _CH), jnp.int32),
        pltpu.VMEM((_CH, _D), jnp.float32),
        pltpu.VMEM((_D,), jnp.float32),
        pltpu.VMEM_SHARED((_N, _D), jnp.float32),
    ],
    compiler_params=pltpu.CompilerParams(
        needs_layout_passes=False, use_tc_tiling_on_sc=False),
)
def _scatter_kernel(src_hbm, dst_hbm, rel_hbm, w_hbm, s_hbm, root_hbm,
                    bias_hbm, out_hbm, src_v, dst_v, rel_v, s_v, gidx_v,
                    sidx_v, rows_v, bias_v, agg_sh):
  cid = lax.axis_index("c")
  sid = lax.axis_index("s")
  wid = sid * _NC + cid
  base = wid * _EPW

  pltpu.sync_copy(src_hbm.at[pl.ds(base, _EPW)], src_v)
  pltpu.sync_copy(dst_hbm.at[pl.ds(base, _EPW)], dst_v)
  pltpu.sync_copy(rel_hbm.at[pl.ds(base, _EPW)], rel_v)
  pltpu.sync_copy(s_hbm.at[pl.ds(base, _EPW)], s_v.at[pl.ds(0, _EPW)])
  pltpu.sync_copy(bias_hbm, bias_v)

  zrow = jnp.zeros((16,), jnp.float32)

  def zero_rows(nrows):
    def zb(j, _):
      for t in range(_D // 16):
        rows_v[j, pl.ds(t * 16, 16)] = zrow
      return 0
    lax.fori_loop(0, nrows, zb, 0)

  nchunk = _RPW // 125
  @pl.when(cid == 1)
  def _():
    zero_rows(125)
    for k in range(nchunk):
      r0 = sid * _RPW + k * 125
      pltpu.sync_copy(rows_v.at[pl.ds(0, 125)], agg_sh.at[pl.ds(r0, 125)])

  @pl.when(cid == 0)
  def _():
    for k in range(nchunk):
      r0 = sid * _RPW + k * 125
      pltpu.sync_copy(root_hbm.at[pl.ds(r0, 125)], rows_v.at[pl.ds(0, 125)])

      def ab(j, _):
        for t in range(_D // 16):
          sl = pl.ds(t * 16, 16)
          rows_v[j, sl] = rows_v[j, sl] + bias_v[sl]
        return 0
      lax.fori_loop(0, 125, ab, 0)
      pltpu.sync_copy(rows_v.at[pl.ds(0, 125)], agg_sh.at[pl.ds(r0, 125)])

  plsc.subcore_barrier()

  def do_chunk(off, nedges):
    nv = nedges // 16

    def ib(i, _):
      sl = pl.ds(off + i * 16, 16)
      so = pl.ds(i * 16, 16)
      gidx_v[0, so] = rel_v[sl] * _N + src_v[sl]
      sidx_v[0, so] = dst_v[sl]
      return 0
    lax.fori_loop(0, nv, ib, 0)

    if nedges == _CH:
      pltpu.sync_copy(w_hbm.at[gidx_v.at[0]], rows_v)
    else:
      pltpu.sync_copy(w_hbm.at[gidx_v.at[0, pl.ds(0, nedges)]],
                      rows_v.at[pl.ds(0, nedges)])

    def sb(j, _):
      s = s_v[pl.ds(off + j, 16)][0]
      bv = jnp.full((16,), s, jnp.float32)
      for t in range(_D // 16):
        sl = pl.ds(t * 16, 16)
        rows_v[j, sl] = rows_v[j, sl] * bv
      return 0
    lax.fori_loop(0, nedges, sb, 0)

    if nedges == _CH:
      pltpu.sync_copy(rows_v, agg_sh.at[sidx_v.at[0]], add=True)
    else:
      pltpu.sync_copy(rows_v.at[pl.ds(0, nedges)],
                      agg_sh.at[sidx_v.at[0, pl.ds(0, nedges)]], add=True)

  def mb(c, _):
    do_chunk(c * _CH, _CH)
    return 0
  lax.fori_loop(0, _NFULL, mb, 0)
  do_chunk(_NFULL * _CH, _TAIL)

  plsc.subcore_barrier()

  for k in range(nchunk):
    r0 = sid * _RPW + k * 125
    pltpu.sync_copy(agg_sh.at[pl.ds(r0, 125)], rows_v.at[pl.ds(0, 125)])
    pltpu.sync_copy(rows_v.at[pl.ds(0, 125)], out_hbm.at[cid, pl.ds(r0, 125)])


_GT = 25
_GPW = _BS // _GT


@functools.partial(
    pl.kernel,
    out_type=(
        jax.ShapeDtypeStruct((_BS, _D), jnp.float32),
        jax.ShapeDtypeStruct((_BS, 64), jnp.float32),
        jax.ShapeDtypeStruct((_BS,), jnp.float32),
        jax.ShapeDtypeStruct((_BS, _H), jnp.float32),
        jax.ShapeDtypeStruct((_BS,), jnp.float32),
    ),
    mesh=_mesh(),
    scratch_types=[
        pltpu.VMEM((_Q,), jnp.int32),
        pltpu.VMEM((_Q,), jnp.int32),
        pltpu.VMEM((_N,), jnp.float32),
        pltpu.VMEM((_GPW,), jnp.int32),
        pltpu.VMEM((_GPW,), jnp.int32),
        pltpu.VMEM((1, _GPW), jnp.int32),
        pltpu.VMEM((1, _GPW), jnp.int32),
        pltpu.VMEM((1, _GPW), jnp.int32),
        pltpu.VMEM((_GPW,), jnp.float32),
        pltpu.VMEM((_GPW,), jnp.float32),
        pltpu.VMEM((_GPW, _D), jnp.float32),
        pltpu.VMEM((_GPW, _D), jnp.float32),
        pltpu.VMEM((_GPW, 64), jnp.float32),
        pltpu.VMEM((_GPW, _H), jnp.float32),
    ],
    compiler_params=pltpu.CompilerParams(
        needs_layout_passes=False, use_tc_tiling_on_sc=False),
)
def _gather_kernel(qi_hbm, qf_hbm, qr_hbm, agg0_hbm, agg1_hbm, diff_hbm,
                   embd_hbm, b3_hbm, w3_hbm,
                   conc_hbm, dif_hbm, dqs_hbm, w3g_hbm, b3g_hbm,
                   qi_v, diff_v, b3_v, q_v, qr_v, idx_v, idxs_v, dr_v,
                   dqs_v, b3g_v, ca_v, cb_v, db_v, wb_v):
  wid = lax.axis_index("s") * _NC + lax.axis_index("c")

  @pl.when(wid < _GT)
  def _():
    base = wid * _GPW
    pltpu.sync_copy(qi_hbm, qi_v)
    pltpu.sync_copy(diff_hbm, diff_v)
    pltpu.sync_copy(b3_hbm, b3_v)
    pltpu.sync_copy(qf_hbm.at[pl.ds(base, _GPW)], q_v)
    pltpu.sync_copy(qr_hbm.at[pl.ds(base, _GPW)], qr_v)

    def ib(i, _):
      sl = pl.ds(i * 16, 16)
      qv = q_v[sl]
      qs = qr_v[sl]
      n16 = plsc.load_gather(qi_v, [qv])
      ns16 = plsc.load_gather(qi_v, [qs])
      idx_v[0, sl] = n16
      idxs_v[0, sl] = ns16
      dr_v[0, sl] = plsc.load_gather(diff_v, [qv])
      dqs_v[sl] = plsc.load_gather(diff_v, [qs]).astype(jnp.float32)
      b3g_v[sl] = plsc.load_gather(b3_v, [ns16])
      return 0
    lax.fori_loop(0, _GPW // 16, ib, 0)

    pltpu.sync_copy(agg0_hbm.at[idx_v.at[0]], ca_v)
    pltpu.sync_copy(agg1_hbm.at[idx_v.at[0]], cb_v)

    def ab(j, _):
      for t in range(_D // 16):
        sl = pl.ds(t * 16, 16)
        ca_v[j, sl] = ca_v[j, sl] + cb_v[j, sl]
      return 0
    lax.fori_loop(0, _GPW, ab, 0)

    pltpu.sync_copy(embd_hbm.at[dr_v.at[0]], db_v)
    pltpu.sync_copy(w3_hbm.at[idxs_v.at[0]], wb_v)

    pltpu.sync_copy(ca_v, conc_hbm.at[pl.ds(base, _GPW)])
    pltpu.sync_copy(db_v, dif_hbm.at[pl.ds(base, _GPW)])
    pltpu.sync_copy(wb_v, w3g_hbm.at[pl.ds(base, _GPW)])
    pltpu.sync_copy(dqs_v, dqs_hbm.at[pl.ds(base, _GPW)])
    pltpu.sync_copy(b3g_v, b3g_hbm.at[pl.ds(base, _GPW)])


def _dense_tc(conc, dif, yf, ea, W1, b1, W2, b2, Wih, Whh, bih, bhh, w3g3,
              b3g2, dqs2):
  def body(conc_ref, dif_ref, yf_ref, ea_ref, W1_ref, b1_ref, W2_ref, b2_ref,
           Wih_ref, Whh_ref, bih_ref, bhh_ref, w3g_ref, b3g_ref, dqs_ref,
           out_ref, res_ref):
    x1 = jnp.concatenate([conc_ref[...], dif_ref[...]], axis=1)
    text = lax.dot_general(x1, W1_ref[...], (((1,), (1,)), ((), ())),
                           preferred_element_type=jnp.float32) + b1_ref[...]
    a0 = ea_ref[0:1, :]
    a1 = ea_ref[1:2, :]
    ans = a0 + yf_ref[...] * (a1 - a0)
    x2 = jnp.concatenate([text, ans], axis=1)
    X = lax.dot_general(x2, W2_ref[...], (((1,), (1,)), ((), ())),
                        preferred_element_type=jnp.float32) + b2_ref[...]
    bgv = bih_ref[...] + bhh_ref[...]
    Wih = Wih_ref[...]
    Whh = Whh_ref[...]

    h = jnp.zeros((50, _H), jnp.float32)
    c = jnp.zeros((50, _H), jnp.float32)
    hs = []
    for t in range(32):
      xt = X[t * 50:(t + 1) * 50, :]
      g = (lax.dot_general(xt, Wih, (((1,), (1,)), ((), ())),
                           preferred_element_type=jnp.float32)
           + lax.dot_general(h, Whh, (((1,), (1,)), ((), ())),
                             preferred_element_type=jnp.float32) + bgv)
      i_ = jax.nn.sigmoid(g[:, 0:_H])
      f_ = jax.nn.sigmoid(g[:, _H:2 * _H])
      gg = jnp.tanh(g[:, 2 * _H:3 * _H])
      o_ = jax.nn.sigmoid(g[:, 3 * _H:4 * _H])
      c = f_ * c + i_ * gg
      h = o_ * jnp.tanh(c)
      hs.append(h)

    outv = jnp.stack(hs, axis=0)
    out_ref[...] = outv
    pr = jnp.sum(outv * w3g_ref[...], axis=2) + b3g_ref[...]
    ev = jax.nn.sigmoid(pr)
    res_ref[...] = jax.nn.sigmoid(ev - (dqs_ref[...] * 0.2 + 0.2))

  return pl.pallas_call(
      body,
      out_shape=(
          jax.ShapeDtypeStruct((32, 50, _H), jnp.float32),
          jax.ShapeDtypeStruct((32, 50), jnp.float32),
      ),
  )(conc, dif, yf, ea, W1, b1, W2, b2, Wih, Whh, bih, bhh, w3g3, b3g2, dqs2)


_CT = 1024


def _e_tc(out, W3, b3r):
  def body(o_ref, w_ref, b_ref, e_ref):
    e_ref[...] = jax.nn.sigmoid(
        lax.dot_general(o_ref[...], w_ref[...], (((1,), (1,)), ((), ())),
                        preferred_element_type=jnp.float32) + b_ref[...])

  grid = pl.cdiv(_N, _CT)
  return pl.pallas_call(
      body,
      grid=(grid,),
      in_specs=[
          pl.BlockSpec((_BS, _H), lambda j: (0, 0)),
          pl.BlockSpec((_CT, _H), lambda j: (j, 0)),
          pl.BlockSpec((1, _CT), lambda j: (0, j)),
      ],
      out_specs=pl.BlockSpec((_BS, _CT), lambda j: (0, j)),
      out_shape=jax.ShapeDtypeStruct((_BS, _N), jnp.float32),
  )(out, W3, b3r)


def kernel(Q_info, edge_index, edge_type, q, y, diff, device, rgcn_weight,
           rgcn_root, rgcn_bias, emb_diff, emb_answer, W1, b1, W2, b2,
           W_ih, W_hh, b_ih, b_hh, W3, b3):
  src = edge_index[0].astype(jnp.int32)
  dst = edge_index[1].astype(jnp.int32)
  rel = edge_type.astype(jnp.int32)

  cnt_parts = _count_kernel(dst, rel)
  inv = _inv_tc(cnt_parts)

  s_all = _scale_kernel(dst, rel, inv)
  wflat = rgcn_weight.reshape(_SEG, _D)
  agg = _scatter_kernel(src, dst, rel, wflat, s_all, rgcn_root, rgcn_bias)

  qf = q.reshape(-1).astype(jnp.int32)
  qr = jnp.roll(qf, -1)
  conc, dif, dqs, w3g, b3g = _gather_kernel(
      Q_info.astype(jnp.int32), qf, qr, agg[0], agg[1],
      diff.astype(jnp.int32), emb_diff, b3, W3)

  yf = y.reshape(_BS, 1).astype(jnp.float32)
  out3, res_full = _dense_tc(
      conc, dif, yf, emb_answer, W1, b1.reshape(1, -1), W2, b2.reshape(1, -1),
      W_ih, W_hh, b_ih.reshape(1, -1), b_hh.reshape(1, -1),
      w3g.reshape(32, 50, _H), b3g.reshape(32, 50), dqs.reshape(32, 50))

  e = _e_tc(out3.reshape(_BS, _H), W3, b3.reshape(1, _N))

  res = res_full[:, :49]
  return (res, e.reshape(32, 50, _N))


# speedup vs baseline: 7.0828x; 1.1603x over previous
import functools

import jax
import jax.numpy as jnp
from jax import lax
from jax.experimental import pallas as pl
from jax.experimental.pallas import tpu as pltpu
from jax.experimental.pallas import tpu_sc as plsc

_N = 10000
_R = 4
_D = 128
_E = 160000
_SEG = _N * _R
_NC = 2
_NS = 16
_NW = _NC * _NS
_EPW = _E // _NW
_Q = 20000
_BS = 1600
_H = 256
_RPW = _N // _NS


def _mesh():
  return plsc.VectorSubcoreMesh(
      core_axis_name="c", subcore_axis_name="s",
      num_cores=_NC, num_subcores=_NS)


@functools.partial(
    pl.kernel,
    out_type=jax.ShapeDtypeStruct((_NW, _SEG), jnp.float32),
    mesh=_mesh(),
    scratch_types=[
        pltpu.VMEM((_EPW,), jnp.int32),
        pltpu.VMEM((_EPW,), jnp.int32),
        pltpu.VMEM((_SEG,), jnp.float32),
    ],
    compiler_params=pltpu.CompilerParams(
        needs_layout_passes=False, use_tc_tiling_on_sc=False),
)
def _count_kernel(dst_hbm, rel_hbm, out_hbm, dst_v, comb_v, cnt_v):
  wid = lax.axis_index("s") * _NC + lax.axis_index("c")
  base = wid * _EPW
  pltpu.sync_copy(dst_hbm.at[pl.ds(base, _EPW)], dst_v)
  pltpu.sync_copy(rel_hbm.at[pl.ds(base, _EPW)], comb_v)

  def zbody(i, _):
    cnt_v[pl.ds(i * 16, 16)] = jnp.zeros((16,), jnp.float32)
    return 0
  lax.fori_loop(0, _SEG // 16, zbody, 0)

  def cbody(i, _):
    sl = pl.ds(i * 16, 16)
    comb_v[sl] = dst_v[sl] * _R + comb_v[sl]
    return 0
  lax.fori_loop(0, _EPW // 16, cbody, 0)

  def hbody(i, _):
    c16 = comb_v[pl.ds(i * 16, 16)]
    cnts, lastm = plsc.scan_count(c16)
    plsc.addupdate_scatter(cnt_v, [c16], cnts.astype(jnp.float32), mask=lastm)
    return 0
  lax.fori_loop(0, _EPW // 16, hbody, 0)

  pltpu.sync_copy(cnt_v, out_hbm.at[wid])


def _inv_tc(cnt_parts):
  def body(c_ref, o_ref):
    s = jnp.sum(c_ref[...], axis=0, keepdims=True)
    o_ref[...] = 1.0 / jnp.maximum(s, 1.0)

  out = pl.pallas_call(
      body,
      out_shape=jax.ShapeDtypeStruct((1, _SEG), jnp.float32),
  )(cnt_parts)
  return out.reshape(_SEG)


@functools.partial(
    pl.kernel,
    out_type=jax.ShapeDtypeStruct((_E,), jnp.float32),
    mesh=_mesh(),
    scratch_types=[
        pltpu.VMEM((_EPW,), jnp.int32),
        pltpu.VMEM((_EPW,), jnp.int32),
        pltpu.VMEM((_SEG,), jnp.float32),
        pltpu.VMEM((_EPW,), jnp.float32),
    ],
    compiler_params=pltpu.CompilerParams(
        needs_layout_passes=False, use_tc_tiling_on_sc=False),
)
def _scale_kernel(dst_hbm, rel_hbm, inv_hbm, out_hbm, dst_v, rel_v, inv_v,
                  s_v):
  wid = lax.axis_index("s") * _NC + lax.axis_index("c")
  base = wid * _EPW
  pltpu.sync_copy(dst_hbm.at[pl.ds(base, _EPW)], dst_v)
  pltpu.sync_copy(rel_hbm.at[pl.ds(base, _EPW)], rel_v)
  pltpu.sync_copy(inv_hbm, inv_v)

  def body(i, _):
    sl = pl.ds(i * 16, 16)
    comb = dst_v[sl] * _R + rel_v[sl]
    s_v[sl] = plsc.load_gather(inv_v, [comb])
    return 0
  lax.fori_loop(0, _EPW // 16, body, 0)

  pltpu.sync_copy(s_v, out_hbm.at[pl.ds(base, _EPW)])


_CH = 128
_NFULL = (_EPW // _CH)
_TAIL = _EPW - _NFULL * _CH


@functools.partial(
    pl.kernel,
    out_type=jax.ShapeDtypeStruct((_NC, _N, _D), jnp.float32),
    mesh=_mesh(),
    scratch_types=[
        pltpu.VMEM((_EPW,), jnp.int32),
        pltpu.VMEM((_EPW,), jnp.int32),
        pltpu.VMEM((_EPW,), jnp.int32),
        pltpu.VMEM((_EPW + 16,), jnp.float32),
        pltpu.VMEM((1, _CH), jnp.int32),
        pltpu.VMEM((1, _CH), jnp.int32),
        pltpu.VMEM((_CH, _D), jnp.float32),
        pltpu.VMEM((_D,), jnp.float32),
        pltpu.VMEM_SHARED((_N, _D), jnp.float32),
    ],
    compiler_params=pltpu.CompilerParams(
        needs_layout_passes=False, use_tc_tiling_on_sc=False),
)
def _scatter_kernel(src_hbm, dst_hbm, rel_hbm, w_hbm, s_hbm, root_hbm,
                    bias_hbm, out_hbm, src_v, dst_v, rel_v, s_v, gidx_v,
                    sidx_v, rows_v, bias_v, agg_sh):
  cid = lax.axis_index("c")
  sid = lax.axis_index("s")
  wid = sid * _NC + cid
  base = wid * _EPW

  pltpu.sync_copy(src_hbm.at[pl.ds(base, _EPW)], src_v)
  pltpu.sync_copy(dst_hbm.at[pl.ds(base, _EPW)], dst_v)
  pltpu.sync_copy(rel_hbm.at[pl.ds(base, _EPW)], rel_v)
  pltpu.sync_copy(s_hbm.at[pl.ds(base, _EPW)], s_v.at[pl.ds(0, _EPW)])
  pltpu.sync_copy(bias_hbm, bias_v)

  zrow = jnp.zeros((16,), jnp.float32)

  def zero_rows(nrows):
    def zb(j, _):
      for t in range(_D // 16):
        rows_v[j, pl.ds(t * 16, 16)] = zrow
      return 0
    lax.fori_loop(0, nrows, zb, 0)

  nchunk = _RPW // 125
  @pl.when(cid == 1)
  def _():
    zero_rows(125)
    for k in range(nchunk):
      r0 = sid * _RPW + k * 125
      pltpu.sync_copy(rows_v.at[pl.ds(0, 125)], agg_sh.at[pl.ds(r0, 125)])

  @pl.when(cid == 0)
  def _():
    for k in range(nchunk):
      r0 = sid * _RPW + k * 125
      pltpu.sync_copy(root_hbm.at[pl.ds(r0, 125)], rows_v.at[pl.ds(0, 125)])

      def ab(j, _):
        for t in range(_D // 16):
          sl = pl.ds(t * 16, 16)
          rows_v[j, sl] = rows_v[j, sl] + bias_v[sl]
        return 0
      lax.fori_loop(0, 125, ab, 0)
      pltpu.sync_copy(rows_v.at[pl.ds(0, 125)], agg_sh.at[pl.ds(r0, 125)])

  plsc.subcore_barrier()

  def do_chunk(off, nedges):
    nv = nedges // 16

    def ib(i, _):
      sl = pl.ds(off + i * 16, 16)
      so = pl.ds(i * 16, 16)
      gidx_v[0, so] = rel_v[sl] * _N + src_v[sl]
      sidx_v[0, so] = dst_v[sl]
      return 0
    lax.fori_loop(0, nv, ib, 0)

    if nedges == _CH:
      pltpu.sync_copy(w_hbm.at[gidx_v.at[0]], rows_v)
    else:
      pltpu.sync_copy(w_hbm.at[gidx_v.at[0, pl.ds(0, nedges)]],
                      rows_v.at[pl.ds(0, nedges)])

    def sb(j, _):
      s = s_v[pl.ds(off + j, 16)][0]
      bv = jnp.full((16,), s, jnp.float32)
      for t in range(_D // 16):
        sl = pl.ds(t * 16, 16)
        rows_v[j, sl] = rows_v[j, sl] * bv
      return 0
    lax.fori_loop(0, nedges, sb, 0)

    if nedges == _CH:
      pltpu.sync_copy(rows_v, agg_sh.at[sidx_v.at[0]], add=True)
    else:
      pltpu.sync_copy(rows_v.at[pl.ds(0, nedges)],
                      agg_sh.at[sidx_v.at[0, pl.ds(0, nedges)]], add=True)

  def mb(c, _):
    do_chunk(c * _CH, _CH)
    return 0
  lax.fori_loop(0, _NFULL, mb, 0)
  do_chunk(_NFULL * _CH, _TAIL)

  plsc.subcore_barrier()

  for k in range(nchunk):
    r0 = sid * _RPW + k * 125
    pltpu.sync_copy(agg_sh.at[pl.ds(r0, 125)], rows_v.at[pl.ds(0, 125)])
    pltpu.sync_copy(rows_v.at[pl.ds(0, 125)], out_hbm.at[cid, pl.ds(r0, 125)])


_GT = 25
_GPW = _BS // _GT


@functools.partial(
    pl.kernel,
    out_type=(
        jax.ShapeDtypeStruct((_BS, _D), jnp.float32),
        jax.ShapeDtypeStruct((_BS, 64), jnp.float32),
        jax.ShapeDtypeStruct((_BS,), jnp.float32),
        jax.ShapeDtypeStruct((_BS, _H), jnp.float32),
        jax.ShapeDtypeStruct((_BS,), jnp.float32),
    ),
    mesh=_mesh(),
    scratch_types=[
        pltpu.VMEM((_Q,), jnp.int32),
        pltpu.VMEM((_Q,), jnp.int32),
        pltpu.VMEM((_N,), jnp.float32),
        pltpu.VMEM((_GPW,), jnp.int32),
        pltpu.VMEM((_GPW,), jnp.int32),
        pltpu.VMEM((1, _GPW), jnp.int32),
        pltpu.VMEM((1, _GPW), jnp.int32),
        pltpu.VMEM((1, _GPW), jnp.int32),
        pltpu.VMEM((_GPW,), jnp.float32),
        pltpu.VMEM((_GPW,), jnp.float32),
        pltpu.VMEM((_GPW, _D), jnp.float32),
        pltpu.VMEM((_GPW, _D), jnp.float32),
        pltpu.VMEM((_GPW, 64), jnp.float32),
        pltpu.VMEM((_GPW, _H), jnp.float32),
    ],
    compiler_params=pltpu.CompilerParams(
        needs_layout_passes=False, use_tc_tiling_on_sc=False),
)
def _gather_kernel(qi_hbm, qf_hbm, qr_hbm, agg0_hbm, agg1_hbm, diff_hbm,
                   embd_hbm, b3_hbm, w3_hbm,
                   conc_hbm, dif_hbm, dqs_hbm, w3g_hbm, b3g_hbm,
                   qi_v, diff_v, b3_v, q_v, qr_v, idx_v, idxs_v, dr_v,
                   dqs_v, b3g_v, ca_v, cb_v, db_v, wb_v):
  wid = lax.axis_index("s") * _NC + lax.axis_index("c")

  @pl.when(wid < _GT)
  def _():
    base = wid * _GPW
    pltpu.sync_copy(qi_hbm, qi_v)
    pltpu.sync_copy(diff_hbm, diff_v)
    pltpu.sync_copy(b3_hbm, b3_v)
    pltpu.sync_copy(qf_hbm.at[pl.ds(base, _GPW)], q_v)
    pltpu.sync_copy(qr_hbm.at[pl.ds(base, _GPW)], qr_v)

    def ib(i, _):
      sl = pl.ds(i * 16, 16)
      qv = q_v[sl]
      qs = qr_v[sl]
      n16 = plsc.load_gather(qi_v, [qv])
      ns16 = plsc.load_gather(qi_v, [qs])
      idx_v[0, sl] = n16
      idxs_v[0, sl] = ns16
      dr_v[0, sl] = plsc.load_gather(diff_v, [qv])
      dqs_v[sl] = plsc.load_gather(diff_v, [qs]).astype(jnp.float32)
      b3g_v[sl] = plsc.load_gather(b3_v, [ns16])
      return 0
    lax.fori_loop(0, _GPW // 16, ib, 0)

    pltpu.sync_copy(agg0_hbm.at[idx_v.at[0]], ca_v)
    pltpu.sync_copy(agg1_hbm.at[idx_v.at[0]], cb_v)

    def ab(j, _):
      for t in range(_D // 16):
        sl = pl.ds(t * 16, 16)
        ca_v[j, sl] = ca_v[j, sl] + cb_v[j, sl]
      return 0
    lax.fori_loop(0, _GPW, ab, 0)

    pltpu.sync_copy(embd_hbm.at[dr_v.at[0]], db_v)
    pltpu.sync_copy(w3_hbm.at[idxs_v.at[0]], wb_v)

    pltpu.sync_copy(ca_v, conc_hbm.at[pl.ds(base, _GPW)])
    pltpu.sync_copy(db_v, dif_hbm.at[pl.ds(base, _GPW)])
    pltpu.sync_copy(wb_v, w3g_hbm.at[pl.ds(base, _GPW)])
    pltpu.sync_copy(dqs_v, dqs_hbm.at[pl.ds(base, _GPW)])
    pltpu.sync_copy(b3g_v, b3g_hbm.at[pl.ds(base, _GPW)])


def _dense_tc(conc, dif, yf, ea, W1, b1, W2, b2, Wih, Whh, bih, bhh, w3g3,
              b3g2, dqs2):
  def body(conc_ref, dif_ref, yf_ref, ea_ref, W1_ref, b1_ref, W2_ref, b2_ref,
           Wih_ref, Whh_ref, bih_ref, bhh_ref, w3g_ref, b3g_ref, dqs_ref,
           out_ref, res_ref):
    x1 = jnp.concatenate([conc_ref[...], dif_ref[...]], axis=1)
    text = lax.dot_general(x1, W1_ref[...], (((1,), (1,)), ((), ())),
                           preferred_element_type=jnp.float32) + b1_ref[...]
    a0 = ea_ref[0:1, :]
    a1 = ea_ref[1:2, :]
    ans = a0 + yf_ref[...] * (a1 - a0)
    x2 = jnp.concatenate([text, ans], axis=1)
    X = lax.dot_general(x2, W2_ref[...], (((1,), (1,)), ((), ())),
                        preferred_element_type=jnp.float32) + b2_ref[...]
    bgv = bih_ref[...] + bhh_ref[...]
    Wih = Wih_ref[...]
    Whh = Whh_ref[...]

    h = jnp.zeros((50, _H), jnp.float32)
    c = jnp.zeros((50, _H), jnp.float32)
    hs = []
    for t in range(32):
      xt = X[t * 50:(t + 1) * 50, :]
      g = (lax.dot_general(xt, Wih, (((1,), (1,)), ((), ())),
                           preferred_element_type=jnp.float32)
           + lax.dot_general(h, Whh, (((1,), (1,)), ((), ())),
                             preferred_element_type=jnp.float32) + bgv)
      i_ = jax.nn.sigmoid(g[:, 0:_H])
      f_ = jax.nn.sigmoid(g[:, _H:2 * _H])
      gg = jnp.tanh(g[:, 2 * _H:3 * _H])
      o_ = jax.nn.sigmoid(g[:, 3 * _H:4 * _H])
      c = f_ * c + i_ * gg
      h = o_ * jnp.tanh(c)
      hs.append(h)

    outv = jnp.stack(hs, axis=0)
    out_ref[...] = outv
    pr = jnp.sum(outv * w3g_ref[...], axis=2) + b3g_ref[...]
    ev = jax.nn.sigmoid(pr)
    res_ref[...] = jax.nn.sigmoid(ev - (dqs_ref[...] * 0.2 + 0.2))

  return pl.pallas_call(
      body,
      out_shape=(
          jax.ShapeDtypeStruct((32, 50, _H), jnp.float32),
          jax.ShapeDtypeStruct((32, 50), jnp.float32),
      ),
  )(conc, dif, yf, ea, W1, b1, W2, b2, Wih, Whh, bih, bhh, w3g3, b3g2, dqs2)


_CT = 1024


def _e_tc(out, W3, b3r):
  def body(o_ref, w_ref, b_ref, e_ref):
    blk = jax.nn.sigmoid(
        lax.dot_general(o_ref[...], w_ref[...], (((1,), (1,)), ((), ())),
                        preferred_element_type=jnp.float32) + b_ref[...])
    for b in range(32):
      e_ref[b] = blk[b * 50:(b + 1) * 50, :]

  grid = pl.cdiv(_N, _CT)
  return pl.pallas_call(
      body,
      grid=(grid,),
      in_specs=[
          pl.BlockSpec((_BS, _H), lambda j: (0, 0)),
          pl.BlockSpec((_CT, _H), lambda j: (j, 0)),
          pl.BlockSpec((1, _CT), lambda j: (0, j)),
      ],
      out_specs=pl.BlockSpec((32, 50, _CT), lambda j: (0, 0, j)),
      out_shape=jax.ShapeDtypeStruct((32, 50, _N), jnp.float32),
  )(out, W3, b3r)


def kernel(Q_info, edge_index, edge_type, q, y, diff, device, rgcn_weight,
           rgcn_root, rgcn_bias, emb_diff, emb_answer, W1, b1, W2, b2,
           W_ih, W_hh, b_ih, b_hh, W3, b3):
  src = edge_index[0].astype(jnp.int32)
  dst = edge_index[1].astype(jnp.int32)
  rel = edge_type.astype(jnp.int32)

  cnt_parts = _count_kernel(dst, rel)
  inv = _inv_tc(cnt_parts)

  s_all = _scale_kernel(dst, rel, inv)
  wflat = rgcn_weight.reshape(_SEG, _D)
  agg = _scatter_kernel(src, dst, rel, wflat, s_all, rgcn_root, rgcn_bias)

  qf = q.reshape(-1).astype(jnp.int32)
  qr = jnp.roll(qf, -1)
  conc, dif, dqs, w3g, b3g = _gather_kernel(
      Q_info.astype(jnp.int32), qf, qr, agg[0], agg[1],
      diff.astype(jnp.int32), emb_diff, b3, W3)

  yf = y.reshape(_BS, 1).astype(jnp.float32)
  out3, res_full = _dense_tc(
      conc, dif, yf, emb_answer, W1, b1.reshape(1, -1), W2, b2.reshape(1, -1),
      W_ih, W_hh, b_ih.reshape(1, -1), b_hh.reshape(1, -1),
      w3g.reshape(32, 50, _H), b3g.reshape(32, 50), dqs.reshape(32, 50))

  e = _e_tc(out3.reshape(_BS, _H), W3, b3.reshape(1, _N))

  res = res_full[:, :49]
  return (res, e)


# speedup vs baseline: 7.2704x; 1.0265x over previous
import functools

import jax
import jax.numpy as jnp
from jax import lax
from jax.experimental import pallas as pl
from jax.experimental.pallas import tpu as pltpu
from jax.experimental.pallas import tpu_sc as plsc

_N = 10000
_R = 4
_D = 128
_E = 160000
_SEG = _N * _R
_NC = 2
_NS = 16
_NW = _NC * _NS
_EPW = _E // _NW
_Q = 20000
_BS = 1600
_H = 256
_RPW = _N // _NS


def _mesh():
  return plsc.VectorSubcoreMesh(
      core_axis_name="c", subcore_axis_name="s",
      num_cores=_NC, num_subcores=_NS)


@functools.partial(
    pl.kernel,
    out_type=jax.ShapeDtypeStruct((_NW, _SEG), jnp.float32),
    mesh=_mesh(),
    scratch_types=[
        pltpu.VMEM((_EPW,), jnp.int32),
        pltpu.VMEM((_EPW,), jnp.int32),
        pltpu.VMEM((_SEG,), jnp.float32),
    ],
    compiler_params=pltpu.CompilerParams(
        needs_layout_passes=False, use_tc_tiling_on_sc=False),
)
def _count_kernel(dst_hbm, rel_hbm, out_hbm, dst_v, comb_v, cnt_v):
  wid = lax.axis_index("s") * _NC + lax.axis_index("c")
  base = wid * _EPW
  pltpu.sync_copy(dst_hbm.at[pl.ds(base, _EPW)], dst_v)
  pltpu.sync_copy(rel_hbm.at[pl.ds(base, _EPW)], comb_v)

  def zbody(i, _):
    cnt_v[pl.ds(i * 16, 16)] = jnp.zeros((16,), jnp.float32)
    return 0
  lax.fori_loop(0, _SEG // 16, zbody, 0)

  def cbody(i, _):
    sl = pl.ds(i * 16, 16)
    comb_v[sl] = dst_v[sl] * _R + comb_v[sl]
    return 0
  lax.fori_loop(0, _EPW // 16, cbody, 0)

  def hbody(i, _):
    c16 = comb_v[pl.ds(i * 16, 16)]
    cnts, lastm = plsc.scan_count(c16)
    plsc.addupdate_scatter(cnt_v, [c16], cnts.astype(jnp.float32), mask=lastm)
    return 0
  lax.fori_loop(0, _EPW // 16, hbody, 0)

  pltpu.sync_copy(cnt_v, out_hbm.at[wid])


def _inv_tc(cnt_parts):
  def body(c_ref, o_ref):
    s = jnp.sum(c_ref[...], axis=0, keepdims=True)
    o_ref[...] = 1.0 / jnp.maximum(s, 1.0)

  out = pl.pallas_call(
      body,
      out_shape=jax.ShapeDtypeStruct((1, _SEG), jnp.float32),
  )(cnt_parts)
  return out.reshape(_SEG)


@functools.partial(
    pl.kernel,
    out_type=jax.ShapeDtypeStruct((_E,), jnp.float32),
    mesh=_mesh(),
    scratch_types=[
        pltpu.VMEM((_EPW,), jnp.int32),
        pltpu.VMEM((_EPW,), jnp.int32),
        pltpu.VMEM((_SEG,), jnp.float32),
        pltpu.VMEM((_EPW,), jnp.float32),
    ],
    compiler_params=pltpu.CompilerParams(
        needs_layout_passes=False, use_tc_tiling_on_sc=False),
)
def _scale_kernel(dst_hbm, rel_hbm, inv_hbm, out_hbm, dst_v, rel_v, inv_v,
                  s_v):
  wid = lax.axis_index("s") * _NC + lax.axis_index("c")
  base = wid * _EPW
  pltpu.sync_copy(dst_hbm.at[pl.ds(base, _EPW)], dst_v)
  pltpu.sync_copy(rel_hbm.at[pl.ds(base, _EPW)], rel_v)
  pltpu.sync_copy(inv_hbm, inv_v)

  def body(i, _):
    sl = pl.ds(i * 16, 16)
    comb = dst_v[sl] * _R + rel_v[sl]
    s_v[sl] = plsc.load_gather(inv_v, [comb])
    return 0
  lax.fori_loop(0, _EPW // 16, body, 0)

  pltpu.sync_copy(s_v, out_hbm.at[pl.ds(base, _EPW)])


_CH = 96
_NFULL = (_EPW // _CH)
_TAIL = _EPW - _NFULL * _CH
_NPAIR = _NFULL // 2


@functools.partial(
    pl.kernel,
    out_type=jax.ShapeDtypeStruct((_NC, _N, _D), jnp.float32),
    mesh=_mesh(),
    scratch_types=[
        pltpu.VMEM((_EPW + 16,), jnp.int32),
        pltpu.VMEM((_EPW + 16,), jnp.int32),
        pltpu.VMEM((_EPW + 16,), jnp.int32),
        pltpu.VMEM((_EPW + 16,), jnp.float32),
        pltpu.VMEM((2, _CH), jnp.int32),
        pltpu.VMEM((2, _CH), jnp.int32),
        pltpu.VMEM((_CH, _D), jnp.float32),
        pltpu.VMEM((_CH, _D), jnp.float32),
        pltpu.VMEM((_D,), jnp.float32),
        pltpu.SemaphoreType.DMA,
        pltpu.SemaphoreType.DMA,
        pltpu.VMEM_SHARED((_N, _D), jnp.float32),
    ],
    compiler_params=pltpu.CompilerParams(
        needs_layout_passes=False, use_tc_tiling_on_sc=False),
)
def _scatter_kernel(src_hbm, dst_hbm, rel_hbm, w_hbm, s_hbm, root_hbm,
                    bias_hbm, out_hbm, src_v, dst_v, rel_v, s_v, gidx_v,
                    sidx_v, rows_v, rows1_v, bias_v, g0_sem, g1_sem, agg_sh):
  cid = lax.axis_index("c")
  sid = lax.axis_index("s")
  wid = sid * _NC + cid
  base = wid * _EPW

  pltpu.sync_copy(src_hbm.at[pl.ds(base, _EPW)], src_v.at[pl.ds(0, _EPW)])
  pltpu.sync_copy(dst_hbm.at[pl.ds(base, _EPW)], dst_v.at[pl.ds(0, _EPW)])
  pltpu.sync_copy(rel_hbm.at[pl.ds(base, _EPW)], rel_v.at[pl.ds(0, _EPW)])
  pltpu.sync_copy(s_hbm.at[pl.ds(base, _EPW)], s_v.at[pl.ds(0, _EPW)])
  pltpu.sync_copy(bias_hbm, bias_v)

  zrow = jnp.zeros((16,), jnp.float32)
  ich = [(_CH * k, _CH) for k in range(_RPW // _CH)]
  ich.append((_CH * (_RPW // _CH), _RPW - _CH * (_RPW // _CH)))

  def zero_rows(nrows):
    def zb(j, _):
      for t in range(_D // 16):
        rows_v[j, pl.ds(t * 16, 16)] = zrow
      return 0
    lax.fori_loop(0, nrows, zb, 0)

  @pl.when(cid == 1)
  def _():
    zero_rows(_CH)
    for ro, nr in ich:
      r0 = sid * _RPW + ro
      pltpu.sync_copy(rows_v.at[pl.ds(0, nr)], agg_sh.at[pl.ds(r0, nr)])

  @pl.when(cid == 0)
  def _():
    for ro, nr in ich:
      r0 = sid * _RPW + ro
      pltpu.sync_copy(root_hbm.at[pl.ds(r0, nr)], rows_v.at[pl.ds(0, nr)])

      def ab(j, _):
        for t in range(_D // 16):
          sl = pl.ds(t * 16, 16)
          rows_v[j, sl] = rows_v[j, sl] + bias_v[sl]
        return 0
      lax.fori_loop(0, nr, ab, 0)
      pltpu.sync_copy(rows_v.at[pl.ds(0, nr)], agg_sh.at[pl.ds(r0, nr)])

  plsc.subcore_barrier()

  def build_idx(off, bsel):
    def ib(i, _):
      sl = pl.ds(off + i * 16, 16)
      so = pl.ds(i * 16, 16)
      gidx_v[bsel, so] = rel_v[sl] * _N + src_v[sl]
      sidx_v[bsel, so] = dst_v[sl]
      return 0
    lax.fori_loop(0, _CH // 16, ib, 0)

  def scale_rows(rv, off, nedges):
    def sbody(j, _):
      s = s_v[pl.ds(off + j, 16)][0]
      bv = jnp.full((16,), s, jnp.float32)
      for t in range(_D // 16):
        sl = pl.ds(t * 16, 16)
        rv[j, sl] = rv[j, sl] * bv
      return 0
    lax.fori_loop(0, nedges, sbody, 0)

  def pair(k, _):
    off0 = (2 * k) * _CH
    off1 = (2 * k + 1) * _CH
    build_idx(off0, 0)
    d0 = pltpu.async_copy(w_hbm.at[gidx_v.at[0]], rows_v, g0_sem)
    build_idx(off1, 1)
    d1 = pltpu.async_copy(w_hbm.at[gidx_v.at[1]], rows1_v, g1_sem)
    d0.wait()
    scale_rows(rows_v, off0, _CH)
    pltpu.sync_copy(rows_v, agg_sh.at[sidx_v.at[0]], add=True)
    d1.wait()
    scale_rows(rows1_v, off1, _CH)
    pltpu.sync_copy(rows1_v, agg_sh.at[sidx_v.at[1]], add=True)
    return 0
  lax.fori_loop(0, _NPAIR, pair, 0)

  toff = _NFULL * _CH
  gidx_v[0, pl.ds(0, 16)] = rel_v[pl.ds(toff, 16)] * _N + src_v[pl.ds(toff, 16)]
  sidx_v[0, pl.ds(0, 16)] = dst_v[pl.ds(toff, 16)]
  pltpu.sync_copy(w_hbm.at[gidx_v.at[0, pl.ds(0, _TAIL)]],
                  rows_v.at[pl.ds(0, _TAIL)])
  scale_rows(rows_v, toff, _TAIL)
  pltpu.sync_copy(rows_v.at[pl.ds(0, _TAIL)],
                  agg_sh.at[sidx_v.at[0, pl.ds(0, _TAIL)]], add=True)

  plsc.subcore_barrier()

  for ro, nr in ich:
    r0 = sid * _RPW + ro
    pltpu.sync_copy(agg_sh.at[pl.ds(r0, nr)], rows_v.at[pl.ds(0, nr)])
    pltpu.sync_copy(rows_v.at[pl.ds(0, nr)], out_hbm.at[cid, pl.ds(r0, nr)])


_GT = 25
_GPW = _BS // _GT


@functools.partial(
    pl.kernel,
    out_type=(
        jax.ShapeDtypeStruct((_BS, _D), jnp.float32),
        jax.ShapeDtypeStruct((_BS, 64), jnp.float32),
        jax.ShapeDtypeStruct((_BS,), jnp.float32),
        jax.ShapeDtypeStruct((_BS, _H), jnp.float32),
        jax.ShapeDtypeStruct((_BS,), jnp.float32),
    ),
    mesh=_mesh(),
    scratch_types=[
        pltpu.VMEM((_Q,), jnp.int32),
        pltpu.VMEM((_Q,), jnp.int32),
        pltpu.VMEM((_N,), jnp.float32),
        pltpu.VMEM((_GPW,), jnp.int32),
        pltpu.VMEM((_GPW,), jnp.int32),
        pltpu.VMEM((1, _GPW), jnp.int32),
        pltpu.VMEM((1, _GPW), jnp.int32),
        pltpu.VMEM((1, _GPW), jnp.int32),
        pltpu.VMEM((_GPW,), jnp.float32),
        pltpu.VMEM((_GPW,), jnp.float32),
        pltpu.VMEM((_GPW, _D), jnp.float32),
        pltpu.VMEM((_GPW, _D), jnp.float32),
        pltpu.VMEM((_GPW, 64), jnp.float32),
        pltpu.VMEM((_GPW, _H), jnp.float32),
    ],
    compiler_params=pltpu.CompilerParams(
        needs_layout_passes=False, use_tc_tiling_on_sc=False),
)
def _gather_kernel(qi_hbm, qf_hbm, qr_hbm, agg0_hbm, agg1_hbm, diff_hbm,
                   embd_hbm, b3_hbm, w3_hbm,
                   conc_hbm, dif_hbm, dqs_hbm, w3g_hbm, b3g_hbm,
                   qi_v, diff_v, b3_v, q_v, qr_v, idx_v, idxs_v, dr_v,
                   dqs_v, b3g_v, ca_v, cb_v, db_v, wb_v):
  wid = lax.axis_index("s") * _NC + lax.axis_index("c")

  @pl.when(wid < _GT)
  def _():
    base = wid * _GPW
    pltpu.sync_copy(qi_hbm, qi_v)
    pltpu.sync_copy(diff_hbm, diff_v)
    pltpu.sync_copy(b3_hbm, b3_v)
    pltpu.sync_copy(qf_hbm.at[pl.ds(base, _GPW)], q_v)
    pltpu.sync_copy(qr_hbm.at[pl.ds(base, _GPW)], qr_v)

    def ib(i, _):
      sl = pl.ds(i * 16, 16)
      qv = q_v[sl]
      qs = qr_v[sl]
      n16 = plsc.load_gather(qi_v, [qv])
      ns16 = plsc.load_gather(qi_v, [qs])
      idx_v[0, sl] = n16
      idxs_v[0, sl] = ns16
      dr_v[0, sl] = plsc.load_gather(diff_v, [qv])
      dqs_v[sl] = plsc.load_gather(diff_v, [qs]).astype(jnp.float32)
      b3g_v[sl] = plsc.load_gather(b3_v, [ns16])
      return 0
    lax.fori_loop(0, _GPW // 16, ib, 0)

    pltpu.sync_copy(agg0_hbm.at[idx_v.at[0]], ca_v)
    pltpu.sync_copy(agg1_hbm.at[idx_v.at[0]], cb_v)

    def ab(j, _):
      for t in range(_D // 16):
        sl = pl.ds(t * 16, 16)
        ca_v[j, sl] = ca_v[j, sl] + cb_v[j, sl]
      return 0
    lax.fori_loop(0, _GPW, ab, 0)

    pltpu.sync_copy(embd_hbm.at[dr_v.at[0]], db_v)
    pltpu.sync_copy(w3_hbm.at[idxs_v.at[0]], wb_v)

    pltpu.sync_copy(ca_v, conc_hbm.at[pl.ds(base, _GPW)])
    pltpu.sync_copy(db_v, dif_hbm.at[pl.ds(base, _GPW)])
    pltpu.sync_copy(wb_v, w3g_hbm.at[pl.ds(base, _GPW)])
    pltpu.sync_copy(dqs_v, dqs_hbm.at[pl.ds(base, _GPW)])
    pltpu.sync_copy(b3g_v, b3g_hbm.at[pl.ds(base, _GPW)])


def _dense_tc(conc, dif, yf, ea, W1, b1, W2, b2, Wih, Whh, bih, bhh, w3g3,
              b3g2, dqs2):
  def body(conc_ref, dif_ref, yf_ref, ea_ref, W1_ref, b1_ref, W2_ref, b2_ref,
           Wih_ref, Whh_ref, bih_ref, bhh_ref, w3g_ref, b3g_ref, dqs_ref,
           out_ref, res_ref):
    x1 = jnp.concatenate([conc_ref[...], dif_ref[...]], axis=1)
    text = lax.dot_general(x1, W1_ref[...], (((1,), (1,)), ((), ())),
                           preferred_element_type=jnp.float32) + b1_ref[...]
    a0 = ea_ref[0:1, :]
    a1 = ea_ref[1:2, :]
    ans = a0 + yf_ref[...] * (a1 - a0)
    x2 = jnp.concatenate([text, ans], axis=1)
    X = lax.dot_general(x2, W2_ref[...], (((1,), (1,)), ((), ())),
                        preferred_element_type=jnp.float32) + b2_ref[...]
    bgv = bih_ref[...] + bhh_ref[...]
    Wih = Wih_ref[...]
    Whh = Whh_ref[...]

    h = jnp.zeros((50, _H), jnp.float32)
    c = jnp.zeros((50, _H), jnp.float32)
    hs = []
    for t in range(32):
      xt = X[t * 50:(t + 1) * 50, :]
      g = (lax.dot_general(xt, Wih, (((1,), (1,)), ((), ())),
                           preferred_element_type=jnp.float32)
           + lax.dot_general(h, Whh, (((1,), (1,)), ((), ())),
                             preferred_element_type=jnp.float32) + bgv)
      i_ = jax.nn.sigmoid(g[:, 0:_H])
      f_ = jax.nn.sigmoid(g[:, _H:2 * _H])
      gg = jnp.tanh(g[:, 2 * _H:3 * _H])
      o_ = jax.nn.sigmoid(g[:, 3 * _H:4 * _H])
      c = f_ * c + i_ * gg
      h = o_ * jnp.tanh(c)
      hs.append(h)

    outv = jnp.stack(hs, axis=0)
    out_ref[...] = outv
    pr = jnp.sum(outv * w3g_ref[...], axis=2) + b3g_ref[...]
    ev = jax.nn.sigmoid(pr)
    res_ref[...] = jax.nn.sigmoid(ev - (dqs_ref[...] * 0.2 + 0.2))

  return pl.pallas_call(
      body,
      out_shape=(
          jax.ShapeDtypeStruct((32, 50, _H), jnp.float32),
          jax.ShapeDtypeStruct((32, 50), jnp.float32),
      ),
  )(conc, dif, yf, ea, W1, b1, W2, b2, Wih, Whh, bih, bhh, w3g3, b3g2, dqs2)


_CT = 1024


def _e_tc(out, W3, b3r):
  def body(o_ref, w_ref, b_ref, e_ref):
    blk = jax.nn.sigmoid(
        lax.dot_general(o_ref[...], w_ref[...], (((1,), (1,)), ((), ())),
                        preferred_element_type=jnp.float32) + b_ref[...])
    for b in range(32):
      e_ref[b] = blk[b * 50:(b + 1) * 50, :]

  grid = pl.cdiv(_N, _CT)
  return pl.pallas_call(
      body,
      grid=(grid,),
      in_specs=[
          pl.BlockSpec((_BS, _H), lambda j: (0, 0)),
          pl.BlockSpec((_CT, _H), lambda j: (j, 0)),
          pl.BlockSpec((1, _CT), lambda j: (0, j)),
      ],
      out_specs=pl.BlockSpec((32, 50, _CT), lambda j: (0, 0, j)),
      out_shape=jax.ShapeDtypeStruct((32, 50, _N), jnp.float32),
  )(out, W3, b3r)


def kernel(Q_info, edge_index, edge_type, q, y, diff, device, rgcn_weight,
           rgcn_root, rgcn_bias, emb_diff, emb_answer, W1, b1, W2, b2,
           W_ih, W_hh, b_ih, b_hh, W3, b3):
  src = edge_index[0].astype(jnp.int32)
  dst = edge_index[1].astype(jnp.int32)
  rel = edge_type.astype(jnp.int32)

  cnt_parts = _count_kernel(dst, rel)
  inv = _inv_tc(cnt_parts)

  s_all = _scale_kernel(dst, rel, inv)
  wflat = rgcn_weight.reshape(_SEG, _D)
  agg = _scatter_kernel(src, dst, rel, wflat, s_all, rgcn_root, rgcn_bias)

  qf = q.reshape(-1).astype(jnp.int32)
  qr = jnp.roll(qf, -1)
  conc, dif, dqs, w3g, b3g = _gather_kernel(
      Q_info.astype(jnp.int32), qf, qr, agg[0], agg[1],
      diff.astype(jnp.int32), emb_diff, b3, W3)

  yf = y.reshape(_BS, 1).astype(jnp.float32)
  out3, res_full = _dense_tc(
      conc, dif, yf, emb_answer, W1, b1.reshape(1, -1), W2, b2.reshape(1, -1),
      W_ih, W_hh, b_ih.reshape(1, -1), b_hh.reshape(1, -1),
      w3g.reshape(32, 50, _H), b3g.reshape(32, 50), dqs.reshape(32, 50))

  e = _e_tc(out3.reshape(_BS, _H), W3, b3.reshape(1, _N))

  res = res_full[:, :49]
  return (res, e)


# speedup vs baseline: 8.1230x; 1.1173x over previous
import functools

import jax
import jax.numpy as jnp
from jax import lax
from jax.experimental import pallas as pl
from jax.experimental.pallas import tpu as pltpu
from jax.experimental.pallas import tpu_sc as plsc

_N = 10000
_R = 4
_D = 128
_E = 160000
_SEG = _N * _R
_NC = 2
_NS = 16
_NW = _NC * _NS
_EPW = _E // _NW
_Q = 20000
_BS = 1600
_H = 256
_RPW = _N // _NS


def _mesh():
  return plsc.VectorSubcoreMesh(
      core_axis_name="c", subcore_axis_name="s",
      num_cores=_NC, num_subcores=_NS)


@functools.partial(
    pl.kernel,
    out_type=jax.ShapeDtypeStruct((_NW, _SEG), jnp.float32),
    mesh=_mesh(),
    scratch_types=[
        pltpu.VMEM((_EPW,), jnp.int32),
        pltpu.VMEM((_EPW,), jnp.int32),
        pltpu.VMEM((_SEG,), jnp.float32),
    ],
    compiler_params=pltpu.CompilerParams(
        needs_layout_passes=False, use_tc_tiling_on_sc=False),
)
def _count_kernel(dst_hbm, rel_hbm, out_hbm, dst_v, comb_v, cnt_v):
  wid = lax.axis_index("s") * _NC + lax.axis_index("c")
  base = wid * _EPW
  pltpu.sync_copy(dst_hbm.at[pl.ds(base, _EPW)], dst_v)
  pltpu.sync_copy(rel_hbm.at[pl.ds(base, _EPW)], comb_v)

  def zbody(i, _):
    cnt_v[pl.ds(i * 16, 16)] = jnp.zeros((16,), jnp.float32)
    return 0
  lax.fori_loop(0, _SEG // 16, zbody, 0)

  def cbody(i, _):
    sl = pl.ds(i * 16, 16)
    comb_v[sl] = dst_v[sl] * _R + comb_v[sl]
    return 0
  lax.fori_loop(0, _EPW // 16, cbody, 0)

  def hbody(i, _):
    c16 = comb_v[pl.ds(i * 16, 16)]
    cnts, lastm = plsc.scan_count(c16)
    plsc.addupdate_scatter(cnt_v, [c16], cnts.astype(jnp.float32), mask=lastm)
    return 0
  lax.fori_loop(0, _EPW // 16, hbody, 0)

  pltpu.sync_copy(cnt_v, out_hbm.at[wid])


def _inv_tc(cnt_parts):
  def body(c_ref, o_ref):
    s = jnp.sum(c_ref[...], axis=0, keepdims=True)
    o_ref[...] = 1.0 / jnp.maximum(s, 1.0)

  out = pl.pallas_call(
      body,
      out_shape=jax.ShapeDtypeStruct((1, _SEG), jnp.float32),
  )(cnt_parts)
  return out.reshape(_SEG)


@functools.partial(
    pl.kernel,
    out_type=jax.ShapeDtypeStruct((_E,), jnp.float32),
    mesh=_mesh(),
    scratch_types=[
        pltpu.VMEM((_EPW,), jnp.int32),
        pltpu.VMEM((_EPW,), jnp.int32),
        pltpu.VMEM((_SEG,), jnp.float32),
        pltpu.VMEM((_EPW,), jnp.float32),
    ],
    compiler_params=pltpu.CompilerParams(
        needs_layout_passes=False, use_tc_tiling_on_sc=False),
)
def _scale_kernel(dst_hbm, rel_hbm, inv_hbm, out_hbm, dst_v, rel_v, inv_v,
                  s_v):
  wid = lax.axis_index("s") * _NC + lax.axis_index("c")
  base = wid * _EPW
  pltpu.sync_copy(dst_hbm.at[pl.ds(base, _EPW)], dst_v)
  pltpu.sync_copy(rel_hbm.at[pl.ds(base, _EPW)], rel_v)
  pltpu.sync_copy(inv_hbm, inv_v)

  def body(i, _):
    sl = pl.ds(i * 16, 16)
    comb = dst_v[sl] * _R + rel_v[sl]
    s_v[sl] = plsc.load_gather(inv_v, [comb])
    return 0
  lax.fori_loop(0, _EPW // 16, body, 0)

  pltpu.sync_copy(s_v, out_hbm.at[pl.ds(base, _EPW)])


_CH = 96
_NFULL = (_EPW // _CH)
_TAIL = _EPW - _NFULL * _CH
_NPAIR = _NFULL // 2


@functools.partial(
    pl.kernel,
    out_type=jax.ShapeDtypeStruct((_NC, _N, _D), jnp.float32),
    mesh=_mesh(),
    scratch_types=[
        pltpu.VMEM((_EPW + 16,), jnp.int32),
        pltpu.VMEM((_EPW + 16,), jnp.int32),
        pltpu.VMEM((_EPW + 16,), jnp.int32),
        pltpu.VMEM((_EPW + 16,), jnp.float32),
        pltpu.VMEM((2, _CH), jnp.int32),
        pltpu.VMEM((2, _CH), jnp.int32),
        pltpu.VMEM((_CH, _D), jnp.float32),
        pltpu.VMEM((_CH, _D), jnp.float32),
        pltpu.VMEM((_D,), jnp.float32),
        pltpu.SemaphoreType.DMA,
        pltpu.SemaphoreType.DMA,
        pltpu.VMEM_SHARED((_N, _D), jnp.float32),
    ],
    compiler_params=pltpu.CompilerParams(
        needs_layout_passes=False, use_tc_tiling_on_sc=False),
)
def _scatter_kernel(src_hbm, dst_hbm, rel_hbm, w_hbm, s_hbm, root_hbm,
                    bias_hbm, out_hbm, src_v, dst_v, rel_v, s_v, gidx_v,
                    sidx_v, rows_v, rows1_v, bias_v, g0_sem, g1_sem, agg_sh):
  cid = lax.axis_index("c")
  sid = lax.axis_index("s")
  wid = sid * _NC + cid
  base = wid * _EPW

  pltpu.sync_copy(src_hbm.at[pl.ds(base, _EPW)], src_v.at[pl.ds(0, _EPW)])
  pltpu.sync_copy(dst_hbm.at[pl.ds(base, _EPW)], dst_v.at[pl.ds(0, _EPW)])
  pltpu.sync_copy(rel_hbm.at[pl.ds(base, _EPW)], rel_v.at[pl.ds(0, _EPW)])
  pltpu.sync_copy(s_hbm.at[pl.ds(base, _EPW)], s_v.at[pl.ds(0, _EPW)])
  pltpu.sync_copy(bias_hbm, bias_v)

  zrow = jnp.zeros((16,), jnp.float32)
  ich = [(_CH * k, _CH) for k in range(_RPW // _CH)]
  ich.append((_CH * (_RPW // _CH), _RPW - _CH * (_RPW // _CH)))

  def zero_rows(nrows):
    @plsc.parallel_loop(0, nrows, unroll=4)
    def zb(j):
      for t in range(_D // 16):
        rows_v[j, pl.ds(t * 16, 16)] = zrow

  @pl.when(cid == 1)
  def _():
    zero_rows(_CH)
    for ro, nr in ich:
      r0 = sid * _RPW + ro
      pltpu.sync_copy(rows_v.at[pl.ds(0, nr)], agg_sh.at[pl.ds(r0, nr)])

  @pl.when(cid == 0)
  def _():
    for ro, nr in ich:
      r0 = sid * _RPW + ro
      pltpu.sync_copy(root_hbm.at[pl.ds(r0, nr)], rows_v.at[pl.ds(0, nr)])

      @plsc.parallel_loop(0, nr, unroll=4)
      def ab(j):
        for t in range(_D // 16):
          sl = pl.ds(t * 16, 16)
          rows_v[j, sl] = rows_v[j, sl] + bias_v[sl]
      pltpu.sync_copy(rows_v.at[pl.ds(0, nr)], agg_sh.at[pl.ds(r0, nr)])

  plsc.subcore_barrier()

  def build_idx(off, bsel):
    @plsc.parallel_loop(0, _CH // 16, unroll=3)
    def ib(i):
      sl = pl.ds(off + i * 16, 16)
      so = pl.ds(i * 16, 16)
      gidx_v[bsel, so] = rel_v[sl] * _N + src_v[sl]
      sidx_v[bsel, so] = dst_v[sl]

  def scale_rows(rv, off, nedges):
    @plsc.parallel_loop(0, nedges, unroll=4)
    def sbody(j):
      s = s_v[pl.ds(off + j, 16)][0]
      bv = jnp.full((16,), s, jnp.float32)
      for t in range(_D // 16):
        sl = pl.ds(t * 16, 16)
        rv[j, sl] = rv[j, sl] * bv

  def pair(k, _):
    off0 = (2 * k) * _CH
    off1 = (2 * k + 1) * _CH
    build_idx(off0, 0)
    d0 = pltpu.async_copy(w_hbm.at[gidx_v.at[0]], rows_v, g0_sem)
    build_idx(off1, 1)
    d1 = pltpu.async_copy(w_hbm.at[gidx_v.at[1]], rows1_v, g1_sem)
    d0.wait()
    scale_rows(rows_v, off0, _CH)
    pltpu.sync_copy(rows_v, agg_sh.at[sidx_v.at[0]], add=True)
    d1.wait()
    scale_rows(rows1_v, off1, _CH)
    pltpu.sync_copy(rows1_v, agg_sh.at[sidx_v.at[1]], add=True)
    return 0
  lax.fori_loop(0, _NPAIR, pair, 0)

  toff = _NFULL * _CH
  gidx_v[0, pl.ds(0, 16)] = rel_v[pl.ds(toff, 16)] * _N + src_v[pl.ds(toff, 16)]
  sidx_v[0, pl.ds(0, 16)] = dst_v[pl.ds(toff, 16)]
  pltpu.sync_copy(w_hbm.at[gidx_v.at[0, pl.ds(0, _TAIL)]],
                  rows_v.at[pl.ds(0, _TAIL)])
  scale_rows(rows_v, toff, _TAIL)
  pltpu.sync_copy(rows_v.at[pl.ds(0, _TAIL)],
                  agg_sh.at[sidx_v.at[0, pl.ds(0, _TAIL)]], add=True)

  plsc.subcore_barrier()

  for ro, nr in ich:
    r0 = sid * _RPW + ro
    pltpu.sync_copy(agg_sh.at[pl.ds(r0, nr)], rows_v.at[pl.ds(0, nr)])
    pltpu.sync_copy(rows_v.at[pl.ds(0, nr)], out_hbm.at[cid, pl.ds(r0, nr)])


_GT = 25
_GPW = _BS // _GT


@functools.partial(
    pl.kernel,
    out_type=(
        jax.ShapeDtypeStruct((_BS, _D), jnp.float32),
        jax.ShapeDtypeStruct((_BS, 64), jnp.float32),
        jax.ShapeDtypeStruct((_BS,), jnp.float32),
        jax.ShapeDtypeStruct((_BS, _H), jnp.float32),
        jax.ShapeDtypeStruct((_BS,), jnp.float32),
    ),
    mesh=_mesh(),
    scratch_types=[
        pltpu.VMEM((_Q,), jnp.int32),
        pltpu.VMEM((_Q,), jnp.int32),
        pltpu.VMEM((_N,), jnp.float32),
        pltpu.VMEM((_GPW,), jnp.int32),
        pltpu.VMEM((_GPW,), jnp.int32),
        pltpu.VMEM((1, _GPW), jnp.int32),
        pltpu.VMEM((1, _GPW), jnp.int32),
        pltpu.VMEM((1, _GPW), jnp.int32),
        pltpu.VMEM((_GPW,), jnp.float32),
        pltpu.VMEM((_GPW,), jnp.float32),
        pltpu.VMEM((_GPW, _D), jnp.float32),
        pltpu.VMEM((_GPW, _D), jnp.float32),
        pltpu.VMEM((_GPW, 64), jnp.float32),
        pltpu.VMEM((_GPW, _H), jnp.float32),
    ],
    compiler_params=pltpu.CompilerParams(
        needs_layout_passes=False, use_tc_tiling_on_sc=False),
)
def _gather_kernel(qi_hbm, qf_hbm, qr_hbm, agg0_hbm, agg1_hbm, diff_hbm,
                   embd_hbm, b3_hbm, w3_hbm,
                   conc_hbm, dif_hbm, dqs_hbm, w3g_hbm, b3g_hbm,
                   qi_v, diff_v, b3_v, q_v, qr_v, idx_v, idxs_v, dr_v,
                   dqs_v, b3g_v, ca_v, cb_v, db_v, wb_v):
  wid = lax.axis_index("s") * _NC + lax.axis_index("c")

  @pl.when(wid < _GT)
  def _():
    base = wid * _GPW
    pltpu.sync_copy(qi_hbm, qi_v)
    pltpu.sync_copy(diff_hbm, diff_v)
    pltpu.sync_copy(b3_hbm, b3_v)
    pltpu.sync_copy(qf_hbm.at[pl.ds(base, _GPW)], q_v)
    pltpu.sync_copy(qr_hbm.at[pl.ds(base, _GPW)], qr_v)

    @plsc.parallel_loop(0, _GPW // 16, unroll=2)
    def ib(i):
      sl = pl.ds(i * 16, 16)
      qv = q_v[sl]
      qs = qr_v[sl]
      n16 = plsc.load_gather(qi_v, [qv])
      ns16 = plsc.load_gather(qi_v, [qs])
      idx_v[0, sl] = n16
      idxs_v[0, sl] = ns16
      dr_v[0, sl] = plsc.load_gather(diff_v, [qv])
      dqs_v[sl] = plsc.load_gather(diff_v, [qs]).astype(jnp.float32)
      b3g_v[sl] = plsc.load_gather(b3_v, [ns16])

    pltpu.sync_copy(agg0_hbm.at[idx_v.at[0]], ca_v)
    pltpu.sync_copy(agg1_hbm.at[idx_v.at[0]], cb_v)

    @plsc.parallel_loop(0, _GPW, unroll=4)
    def ab(j):
      for t in range(_D // 16):
        sl = pl.ds(t * 16, 16)
        ca_v[j, sl] = ca_v[j, sl] + cb_v[j, sl]

    pltpu.sync_copy(embd_hbm.at[dr_v.at[0]], db_v)
    pltpu.sync_copy(w3_hbm.at[idxs_v.at[0]], wb_v)

    pltpu.sync_copy(ca_v, conc_hbm.at[pl.ds(base, _GPW)])
    pltpu.sync_copy(db_v, dif_hbm.at[pl.ds(base, _GPW)])
    pltpu.sync_copy(wb_v, w3g_hbm.at[pl.ds(base, _GPW)])
    pltpu.sync_copy(dqs_v, dqs_hbm.at[pl.ds(base, _GPW)])
    pltpu.sync_copy(b3g_v, b3g_hbm.at[pl.ds(base, _GPW)])


def _dense_tc(conc, dif, yf, ea, W1, b1, W2, b2, Wih, Whh, bih, bhh, w3g3,
              b3g2, dqs2):
  def body(conc_ref, dif_ref, yf_ref, ea_ref, W1_ref, b1_ref, W2_ref, b2_ref,
           Wih_ref, Whh_ref, bih_ref, bhh_ref, w3g_ref, b3g_ref, dqs_ref,
           out_ref, res_ref):
    x1 = jnp.concatenate([conc_ref[...], dif_ref[...]], axis=1)
    text = lax.dot_general(x1, W1_ref[...], (((1,), (1,)), ((), ())),
                           preferred_element_type=jnp.float32) + b1_ref[...]
    a0 = ea_ref[0:1, :]
    a1 = ea_ref[1:2, :]
    ans = a0 + yf_ref[...] * (a1 - a0)
    x2 = jnp.concatenate([text, ans], axis=1)
    X = lax.dot_general(x2, W2_ref[...], (((1,), (1,)), ((), ())),
                        preferred_element_type=jnp.float32) + b2_ref[...]
    bgv = bih_ref[...] + bhh_ref[...]
    Wih = Wih_ref[...]
    Whh = Whh_ref[...]

    h = jnp.zeros((50, _H), jnp.float32)
    c = jnp.zeros((50, _H), jnp.float32)
    hs = []
    for t in range(32):
      xt = X[t * 50:(t + 1) * 50, :]
      g = (lax.dot_general(xt, Wih, (((1,), (1,)), ((), ())),
                           preferred_element_type=jnp.float32)
           + lax.dot_general(h, Whh, (((1,), (1,)), ((), ())),
                             preferred_element_type=jnp.float32) + bgv)
      i_ = jax.nn.sigmoid(g[:, 0:_H])
      f_ = jax.nn.sigmoid(g[:, _H:2 * _H])
      gg = jnp.tanh(g[:, 2 * _H:3 * _H])
      o_ = jax.nn.sigmoid(g[:, 3 * _H:4 * _H])
      c = f_ * c + i_ * gg
      h = o_ * jnp.tanh(c)
      hs.append(h)

    outv = jnp.stack(hs, axis=0)
    out_ref[...] = outv
    pr = jnp.sum(outv * w3g_ref[...], axis=2) + b3g_ref[...]
    ev = jax.nn.sigmoid(pr)
    res_ref[...] = jax.nn.sigmoid(ev - (dqs_ref[...] * 0.2 + 0.2))

  return pl.pallas_call(
      body,
      out_shape=(
          jax.ShapeDtypeStruct((32, 50, _H), jnp.float32),
          jax.ShapeDtypeStruct((32, 50), jnp.float32),
      ),
  )(conc, dif, yf, ea, W1, b1, W2, b2, Wih, Whh, bih, bhh, w3g3, b3g2, dqs2)


_CT = 1024


def _e_tc(out, W3, b3r):
  def body(o_ref, w_ref, b_ref, e_ref):
    blk = jax.nn.sigmoid(
        lax.dot_general(o_ref[...], w_ref[...], (((1,), (1,)), ((), ())),
                        preferred_element_type=jnp.float32) + b_ref[...])
    for b in range(32):
      e_ref[b] = blk[b * 50:(b + 1) * 50, :]

  grid = pl.cdiv(_N, _CT)
  return pl.pallas_call(
      body,
      grid=(grid,),
      in_specs=[
          pl.BlockSpec((_BS, _H), lambda j: (0, 0)),
          pl.BlockSpec((_CT, _H), lambda j: (j, 0)),
          pl.BlockSpec((1, _CT), lambda j: (0, j)),
      ],
      out_specs=pl.BlockSpec((32, 50, _CT), lambda j: (0, 0, j)),
      out_shape=jax.ShapeDtypeStruct((32, 50, _N), jnp.float32),
  )(out, W3, b3r)


def kernel(Q_info, edge_index, edge_type, q, y, diff, device, rgcn_weight,
           rgcn_root, rgcn_bias, emb_diff, emb_answer, W1, b1, W2, b2,
           W_ih, W_hh, b_ih, b_hh, W3, b3):
  src = edge_index[0].astype(jnp.int32)
  dst = edge_index[1].astype(jnp.int32)
  rel = edge_type.astype(jnp.int32)

  cnt_parts = _count_kernel(dst, rel)
  inv = _inv_tc(cnt_parts)

  s_all = _scale_kernel(dst, rel, inv)
  wflat = rgcn_weight.reshape(_SEG, _D)
  agg = _scatter_kernel(src, dst, rel, wflat, s_all, rgcn_root, rgcn_bias)

  qf = q.reshape(-1).astype(jnp.int32)
  qr = jnp.roll(qf, -1)
  conc, dif, dqs, w3g, b3g = _gather_kernel(
      Q_info.astype(jnp.int32), qf, qr, agg[0], agg[1],
      diff.astype(jnp.int32), emb_diff, b3, W3)

  yf = y.reshape(_BS, 1).astype(jnp.float32)
  out3, res_full = _dense_tc(
      conc, dif, yf, emb_answer, W1, b1.reshape(1, -1), W2, b2.reshape(1, -1),
      W_ih, W_hh, b_ih.reshape(1, -1), b_hh.reshape(1, -1),
      w3g.reshape(32, 50, _H), b3g.reshape(32, 50), dqs.reshape(32, 50))

  e = _e_tc(out3.reshape(_BS, _H), W3, b3.reshape(1, _N))

  res = res_full[:, :49]
  return (res, e)


# speedup vs baseline: 8.1681x; 1.0056x over previous
import functools

import jax
import jax.numpy as jnp
from jax import lax
from jax.experimental import pallas as pl
from jax.experimental.pallas import tpu as pltpu
from jax.experimental.pallas import tpu_sc as plsc

_N = 10000
_R = 4
_D = 128
_E = 160000
_SEG = _N * _R
_NC = 2
_NS = 16
_NW = _NC * _NS
_EPW = _E // _NW
_Q = 20000
_BS = 1600
_H = 256
_RPW = _N // _NS


def _mesh():
  return plsc.VectorSubcoreMesh(
      core_axis_name="c", subcore_axis_name="s",
      num_cores=_NC, num_subcores=_NS)


@functools.partial(
    pl.kernel,
    out_type=jax.ShapeDtypeStruct((_NW, _SEG), jnp.float32),
    mesh=_mesh(),
    scratch_types=[
        pltpu.VMEM((_EPW,), jnp.int32),
        pltpu.VMEM((_EPW,), jnp.int32),
        pltpu.VMEM((_SEG,), jnp.float32),
    ],
    compiler_params=pltpu.CompilerParams(
        needs_layout_passes=False, use_tc_tiling_on_sc=False),
)
def _count_kernel(dst_hbm, rel_hbm, out_hbm, dst_v, comb_v, cnt_v):
  wid = lax.axis_index("s") * _NC + lax.axis_index("c")
  base = wid * _EPW
  pltpu.sync_copy(dst_hbm.at[pl.ds(base, _EPW)], dst_v)
  pltpu.sync_copy(rel_hbm.at[pl.ds(base, _EPW)], comb_v)

  def zbody(i, _):
    cnt_v[pl.ds(i * 16, 16)] = jnp.zeros((16,), jnp.float32)
    return 0
  lax.fori_loop(0, _SEG // 16, zbody, 0)

  def cbody(i, _):
    sl = pl.ds(i * 16, 16)
    comb_v[sl] = dst_v[sl] * _R + comb_v[sl]
    return 0
  lax.fori_loop(0, _EPW // 16, cbody, 0)

  def hbody(i, _):
    c16 = comb_v[pl.ds(i * 16, 16)]
    cnts, lastm = plsc.scan_count(c16)
    plsc.addupdate_scatter(cnt_v, [c16], cnts.astype(jnp.float32), mask=lastm)
    return 0
  lax.fori_loop(0, _EPW // 16, hbody, 0)

  pltpu.sync_copy(cnt_v, out_hbm.at[wid])


def _inv_tc(cnt_parts):
  def body(c_ref, o_ref):
    s = jnp.sum(c_ref[...], axis=0, keepdims=True)
    o_ref[...] = 1.0 / jnp.maximum(s, 1.0)

  out = pl.pallas_call(
      body,
      out_shape=jax.ShapeDtypeStruct((1, _SEG), jnp.float32),
  )(cnt_parts)
  return out.reshape(_SEG)


@functools.partial(
    pl.kernel,
    out_type=jax.ShapeDtypeStruct((_E,), jnp.float32),
    mesh=_mesh(),
    scratch_types=[
        pltpu.VMEM((_EPW,), jnp.int32),
        pltpu.VMEM((_EPW,), jnp.int32),
        pltpu.VMEM((_SEG,), jnp.float32),
        pltpu.VMEM((_EPW,), jnp.float32),
    ],
    compiler_params=pltpu.CompilerParams(
        needs_layout_passes=False, use_tc_tiling_on_sc=False),
)
def _scale_kernel(dst_hbm, rel_hbm, inv_hbm, out_hbm, dst_v, rel_v, inv_v,
                  s_v):
  wid = lax.axis_index("s") * _NC + lax.axis_index("c")
  base = wid * _EPW
  pltpu.sync_copy(dst_hbm.at[pl.ds(base, _EPW)], dst_v)
  pltpu.sync_copy(rel_hbm.at[pl.ds(base, _EPW)], rel_v)
  pltpu.sync_copy(inv_hbm, inv_v)

  def body(i, _):
    sl = pl.ds(i * 16, 16)
    comb = dst_v[sl] * _R + rel_v[sl]
    s_v[sl] = plsc.load_gather(inv_v, [comb])
    return 0
  lax.fori_loop(0, _EPW // 16, body, 0)

  pltpu.sync_copy(s_v, out_hbm.at[pl.ds(base, _EPW)])


_CH = 96
_NFULL = (_EPW // _CH)
_TAIL = _EPW - _NFULL * _CH
_NPAIR = _NFULL // 2


@functools.partial(
    pl.kernel,
    out_type=jax.ShapeDtypeStruct((_NC, _N, _D), jnp.float32),
    mesh=_mesh(),
    scratch_types=[
        pltpu.VMEM((_EPW + 16,), jnp.int32),
        pltpu.VMEM((_EPW + 16,), jnp.int32),
        pltpu.VMEM((_EPW + 16,), jnp.int32),
        pltpu.VMEM((_EPW + 16,), jnp.float32),
        pltpu.VMEM((2, _CH), jnp.int32),
        pltpu.VMEM((2, _CH), jnp.int32),
        pltpu.VMEM((_CH, _D), jnp.float32),
        pltpu.VMEM((_CH, _D), jnp.float32),
        pltpu.VMEM((_D,), jnp.float32),
        pltpu.SemaphoreType.DMA,
        pltpu.SemaphoreType.DMA,
        pltpu.VMEM_SHARED((_N, _D), jnp.float32),
    ],
    compiler_params=pltpu.CompilerParams(
        needs_layout_passes=False, use_tc_tiling_on_sc=False),
)
def _scatter_kernel(src_hbm, dst_hbm, rel_hbm, w_hbm, s_hbm, root_hbm,
                    bias_hbm, out_hbm, src_v, dst_v, rel_v, s_v, gidx_v,
                    sidx_v, rows_v, rows1_v, bias_v, g0_sem, g1_sem, agg_sh):
  cid = lax.axis_index("c")
  sid = lax.axis_index("s")
  wid = sid * _NC + cid
  base = wid * _EPW

  pltpu.sync_copy(src_hbm.at[pl.ds(base, _EPW)], src_v.at[pl.ds(0, _EPW)])
  pltpu.sync_copy(dst_hbm.at[pl.ds(base, _EPW)], dst_v.at[pl.ds(0, _EPW)])
  pltpu.sync_copy(rel_hbm.at[pl.ds(base, _EPW)], rel_v.at[pl.ds(0, _EPW)])
  pltpu.sync_copy(s_hbm.at[pl.ds(base, _EPW)], s_v.at[pl.ds(0, _EPW)])
  pltpu.sync_copy(bias_hbm, bias_v)

  zrow = jnp.zeros((16,), jnp.float32)
  ich = [(_CH * k, _CH) for k in range(_RPW // _CH)]
  ich.append((_CH * (_RPW // _CH), _RPW - _CH * (_RPW // _CH)))

  def zero_rows(nrows):
    @plsc.parallel_loop(0, nrows, unroll=4)
    def zb(j):
      for t in range(_D // 16):
        rows_v[j, pl.ds(t * 16, 16)] = zrow

  @pl.when(cid == 1)
  def _():
    zero_rows(_CH)
    for ro, nr in ich:
      r0 = sid * _RPW + ro
      pltpu.sync_copy(rows_v.at[pl.ds(0, nr)], agg_sh.at[pl.ds(r0, nr)])

  @pl.when(cid == 0)
  def _():
    for ro, nr in ich:
      r0 = sid * _RPW + ro
      pltpu.sync_copy(root_hbm.at[pl.ds(r0, nr)], rows_v.at[pl.ds(0, nr)])

      @plsc.parallel_loop(0, nr, unroll=4)
      def ab(j):
        for t in range(_D // 16):
          sl = pl.ds(t * 16, 16)
          rows_v[j, sl] = rows_v[j, sl] + bias_v[sl]
      pltpu.sync_copy(rows_v.at[pl.ds(0, nr)], agg_sh.at[pl.ds(r0, nr)])

  plsc.subcore_barrier()

  def build_idx(off, bsel):
    @plsc.parallel_loop(0, _CH // 16, unroll=3)
    def ib(i):
      sl = pl.ds(off + i * 16, 16)
      so = pl.ds(i * 16, 16)
      gidx_v[bsel, so] = rel_v[sl] * _N + src_v[sl]
      sidx_v[bsel, so] = dst_v[sl]

  def scale_rows(rv, off, nedges):
    @plsc.parallel_loop(0, nedges, unroll=4)
    def sbody(j):
      s = s_v[pl.ds(off + j, 16)][0]
      bv = jnp.full((16,), s, jnp.float32)
      for t in range(_D // 16):
        sl = pl.ds(t * 16, 16)
        rv[j, sl] = rv[j, sl] * bv

  def pair(k, _):
    off0 = (2 * k) * _CH
    off1 = (2 * k + 1) * _CH
    build_idx(off0, 0)
    d0 = pltpu.async_copy(w_hbm.at[gidx_v.at[0]], rows_v, g0_sem)
    build_idx(off1, 1)
    d1 = pltpu.async_copy(w_hbm.at[gidx_v.at[1]], rows1_v, g1_sem)
    d0.wait()
    scale_rows(rows_v, off0, _CH)
    pltpu.sync_copy(rows_v, agg_sh.at[sidx_v.at[0]], add=True)
    d1.wait()
    scale_rows(rows1_v, off1, _CH)
    pltpu.sync_copy(rows1_v, agg_sh.at[sidx_v.at[1]], add=True)
    return 0
  lax.fori_loop(0, _NPAIR, pair, 0)

  toff = _NFULL * _CH
  gidx_v[0, pl.ds(0, 16)] = rel_v[pl.ds(toff, 16)] * _N + src_v[pl.ds(toff, 16)]
  sidx_v[0, pl.ds(0, 16)] = dst_v[pl.ds(toff, 16)]
  pltpu.sync_copy(w_hbm.at[gidx_v.at[0, pl.ds(0, _TAIL)]],
                  rows_v.at[pl.ds(0, _TAIL)])
  scale_rows(rows_v, toff, _TAIL)
  pltpu.sync_copy(rows_v.at[pl.ds(0, _TAIL)],
                  agg_sh.at[sidx_v.at[0, pl.ds(0, _TAIL)]], add=True)

  plsc.subcore_barrier()

  for ro, nr in ich:
    r0 = sid * _RPW + ro
    pltpu.sync_copy(agg_sh.at[pl.ds(r0, nr)], rows_v.at[pl.ds(0, nr)])
    pltpu.sync_copy(rows_v.at[pl.ds(0, nr)], out_hbm.at[cid, pl.ds(r0, nr)])


_GT = 25
_GPW = _BS // _GT


@functools.partial(
    pl.kernel,
    out_type=(
        jax.ShapeDtypeStruct((_BS, _D), jnp.float32),
        jax.ShapeDtypeStruct((_BS, 64), jnp.float32),
        jax.ShapeDtypeStruct((_BS,), jnp.float32),
        jax.ShapeDtypeStruct((_BS, _H), jnp.float32),
        jax.ShapeDtypeStruct((_BS,), jnp.float32),
    ),
    mesh=_mesh(),
    scratch_types=[
        pltpu.VMEM((_Q,), jnp.int32),
        pltpu.VMEM((_Q,), jnp.int32),
        pltpu.VMEM((_N,), jnp.float32),
        pltpu.VMEM((_GPW,), jnp.int32),
        pltpu.VMEM((_GPW,), jnp.int32),
        pltpu.VMEM((1, _GPW), jnp.int32),
        pltpu.VMEM((1, _GPW), jnp.int32),
        pltpu.VMEM((1, _GPW), jnp.int32),
        pltpu.VMEM((_GPW,), jnp.float32),
        pltpu.VMEM((_GPW,), jnp.float32),
        pltpu.VMEM((_GPW, _D), jnp.float32),
        pltpu.VMEM((_GPW, _D), jnp.float32),
        pltpu.VMEM((_GPW, 64), jnp.float32),
        pltpu.VMEM((_GPW, _H), jnp.float32),
    ],
    compiler_params=pltpu.CompilerParams(
        needs_layout_passes=False, use_tc_tiling_on_sc=False),
)
def _gather_kernel(qi_hbm, qf_hbm, qr_hbm, agg0_hbm, agg1_hbm, diff_hbm,
                   embd_hbm, b3_hbm, w3_hbm,
                   conc_hbm, dif_hbm, dqs_hbm, w3g_hbm, b3g_hbm,
                   qi_v, diff_v, b3_v, q_v, qr_v, idx_v, idxs_v, dr_v,
                   dqs_v, b3g_v, ca_v, cb_v, db_v, wb_v):
  wid = lax.axis_index("s") * _NC + lax.axis_index("c")

  @pl.when(wid < _GT)
  def _():
    base = wid * _GPW
    pltpu.sync_copy(qi_hbm, qi_v)
    pltpu.sync_copy(diff_hbm, diff_v)
    pltpu.sync_copy(b3_hbm, b3_v)
    pltpu.sync_copy(qf_hbm.at[pl.ds(base, _GPW)], q_v)
    pltpu.sync_copy(qr_hbm.at[pl.ds(base, _GPW)], qr_v)

    @plsc.parallel_loop(0, _GPW // 16, unroll=2)
    def ib(i):
      sl = pl.ds(i * 16, 16)
      qv = q_v[sl]
      qs = qr_v[sl]
      n16 = plsc.load_gather(qi_v, [qv])
      ns16 = plsc.load_gather(qi_v, [qs])
      idx_v[0, sl] = n16
      idxs_v[0, sl] = ns16
      dr_v[0, sl] = plsc.load_gather(diff_v, [qv])
      dqs_v[sl] = plsc.load_gather(diff_v, [qs]).astype(jnp.float32)
      b3g_v[sl] = plsc.load_gather(b3_v, [ns16])

    pltpu.sync_copy(agg0_hbm.at[idx_v.at[0]], ca_v)
    pltpu.sync_copy(agg1_hbm.at[idx_v.at[0]], cb_v)

    @plsc.parallel_loop(0, _GPW, unroll=4)
    def ab(j):
      for t in range(_D // 16):
        sl = pl.ds(t * 16, 16)
        ca_v[j, sl] = ca_v[j, sl] + cb_v[j, sl]

    pltpu.sync_copy(embd_hbm.at[dr_v.at[0]], db_v)
    pltpu.sync_copy(w3_hbm.at[idxs_v.at[0]], wb_v)

    pltpu.sync_copy(ca_v, conc_hbm.at[pl.ds(base, _GPW)])
    pltpu.sync_copy(db_v, dif_hbm.at[pl.ds(base, _GPW)])
    pltpu.sync_copy(wb_v, w3g_hbm.at[pl.ds(base, _GPW)])
    pltpu.sync_copy(dqs_v, dqs_hbm.at[pl.ds(base, _GPW)])
    pltpu.sync_copy(b3g_v, b3g_hbm.at[pl.ds(base, _GPW)])


def _dense_tc(conc, dif, yf, ea, W1, b1, W2, b2, Wih, Whh, bih, bhh, w3g3,
              b3g2, dqs2):
  def body(conc_ref, dif_ref, yf_ref, ea_ref, W1_ref, b1_ref, W2_ref, b2_ref,
           Wih_ref, Whh_ref, bih_ref, bhh_ref, w3g_ref, b3g_ref, dqs_ref,
           out_ref, res_ref):
    x1 = jnp.concatenate([conc_ref[...], dif_ref[...]], axis=1)
    text = lax.dot_general(x1, W1_ref[...], (((1,), (1,)), ((), ())),
                           preferred_element_type=jnp.float32) + b1_ref[...]
    a0 = ea_ref[0:1, :]
    a1 = ea_ref[1:2, :]
    ans = a0 + yf_ref[...] * (a1 - a0)
    x2 = jnp.concatenate([text, ans], axis=1)
    X = lax.dot_general(x2, W2_ref[...], (((1,), (1,)), ((), ())),
                        preferred_element_type=jnp.float32) + b2_ref[...]
    bgv = bih_ref[...] + bhh_ref[...]
    GX = lax.dot_general(X, Wih_ref[...], (((1,), (1,)), ((), ())),
                         preferred_element_type=jnp.float32) + bgv
    Whh = Whh_ref[...]

    h = jnp.zeros((50, _H), jnp.float32)
    c = jnp.zeros((50, _H), jnp.float32)
    hs = []
    for t in range(32):
      g = (GX[t * 50:(t + 1) * 50, :]
           + lax.dot_general(h, Whh, (((1,), (1,)), ((), ())),
                             preferred_element_type=jnp.float32))
      i_ = jax.nn.sigmoid(g[:, 0:_H])
      f_ = jax.nn.sigmoid(g[:, _H:2 * _H])
      gg = jnp.tanh(g[:, 2 * _H:3 * _H])
      o_ = jax.nn.sigmoid(g[:, 3 * _H:4 * _H])
      c = f_ * c + i_ * gg
      h = o_ * jnp.tanh(c)
      hs.append(h)

    outc = jnp.concatenate(hs, axis=0)
    out_ref[...] = outc
    pr = jnp.sum(outc * w3g_ref[...], axis=1, keepdims=True) + b3g_ref[...]
    ev = jax.nn.sigmoid(pr)
    res_ref[...] = jax.nn.sigmoid(ev - (dqs_ref[...] * 0.2 + 0.2))

  return pl.pallas_call(
      body,
      out_shape=(
          jax.ShapeDtypeStruct((_BS, _H), jnp.float32),
          jax.ShapeDtypeStruct((_BS, 1), jnp.float32),
      ),
  )(conc, dif, yf, ea, W1, b1, W2, b2, Wih, Whh, bih, bhh, w3g3, b3g2, dqs2)


_CT = 1024


def _e_tc(out, W3, b3r):
  def body(o_ref, w_ref, b_ref, e_ref):
    blk = jax.nn.sigmoid(
        lax.dot_general(o_ref[...], w_ref[...], (((1,), (1,)), ((), ())),
                        preferred_element_type=jnp.float32) + b_ref[...])
    for b in range(32):
      e_ref[b] = blk[b * 50:(b + 1) * 50, :]

  grid = pl.cdiv(_N, _CT)
  return pl.pallas_call(
      body,
      grid=(grid,),
      in_specs=[
          pl.BlockSpec((_BS, _H), lambda j: (0, 0)),
          pl.BlockSpec((_CT, _H), lambda j: (j, 0)),
          pl.BlockSpec((1, _CT), lambda j: (0, j)),
      ],
      out_specs=pl.BlockSpec((32, 50, _CT), lambda j: (0, 0, j)),
      out_shape=jax.ShapeDtypeStruct((32, 50, _N), jnp.float32),
  )(out, W3, b3r)


def kernel(Q_info, edge_index, edge_type, q, y, diff, device, rgcn_weight,
           rgcn_root, rgcn_bias, emb_diff, emb_answer, W1, b1, W2, b2,
           W_ih, W_hh, b_ih, b_hh, W3, b3):
  src = edge_index[0].astype(jnp.int32)
  dst = edge_index[1].astype(jnp.int32)
  rel = edge_type.astype(jnp.int32)

  cnt_parts = _count_kernel(dst, rel)
  inv = _inv_tc(cnt_parts)

  s_all = _scale_kernel(dst, rel, inv)
  wflat = rgcn_weight.reshape(_SEG, _D)
  agg = _scatter_kernel(src, dst, rel, wflat, s_all, rgcn_root, rgcn_bias)

  qf = q.reshape(-1).astype(jnp.int32)
  qr = jnp.roll(qf, -1)
  conc, dif, dqs, w3g, b3g = _gather_kernel(
      Q_info.astype(jnp.int32), qf, qr, agg[0], agg[1],
      diff.astype(jnp.int32), emb_diff, b3, W3)

  yf = y.reshape(_BS, 1).astype(jnp.float32)
  out2, res_full = _dense_tc(
      conc, dif, yf, emb_answer, W1, b1.reshape(1, -1), W2, b2.reshape(1, -1),
      W_ih, W_hh, b_ih.reshape(1, -1), b_hh.reshape(1, -1),
      w3g, b3g.reshape(_BS, 1), dqs.reshape(_BS, 1))

  e = _e_tc(out2, W3, b3.reshape(1, _N))

  res = res_full.reshape(32, 50)[:, :49]
  return (res, e)


# speedup vs baseline: 8.4340x; 1.0325x over previous
import functools

import jax
import jax.numpy as jnp
from jax import lax
from jax.experimental import pallas as pl
from jax.experimental.pallas import tpu as pltpu
from jax.experimental.pallas import tpu_sc as plsc

_N = 10000
_R = 4
_D = 128
_E = 160000
_SEG = _N * _R
_NC = 2
_NS = 16
_NW = _NC * _NS
_EPW = _E // _NW
_Q = 20000
_BS = 1600
_H = 256
_RPW = _N // _NS


def _mesh():
  return plsc.VectorSubcoreMesh(
      core_axis_name="c", subcore_axis_name="s",
      num_cores=_NC, num_subcores=_NS)


@functools.partial(
    pl.kernel,
    out_type=jax.ShapeDtypeStruct((_NW, _SEG), jnp.float32),
    mesh=_mesh(),
    scratch_types=[
        pltpu.VMEM((_EPW,), jnp.int32),
        pltpu.VMEM((_EPW,), jnp.int32),
        pltpu.VMEM((_SEG,), jnp.float32),
    ],
    compiler_params=pltpu.CompilerParams(
        needs_layout_passes=False, use_tc_tiling_on_sc=False),
)
def _count_kernel(dst_hbm, rel_hbm, out_hbm, dst_v, comb_v, cnt_v):
  wid = lax.axis_index("s") * _NC + lax.axis_index("c")
  base = wid * _EPW
  pltpu.sync_copy(dst_hbm.at[pl.ds(base, _EPW)], dst_v)
  pltpu.sync_copy(rel_hbm.at[pl.ds(base, _EPW)], comb_v)

  def zbody(i, _):
    cnt_v[pl.ds(i * 16, 16)] = jnp.zeros((16,), jnp.float32)
    return 0
  lax.fori_loop(0, _SEG // 16, zbody, 0)

  def cbody(i, _):
    sl = pl.ds(i * 16, 16)
    comb_v[sl] = dst_v[sl] * _R + comb_v[sl]
    return 0
  lax.fori_loop(0, _EPW // 16, cbody, 0)

  def hbody(i, _):
    c16 = comb_v[pl.ds(i * 16, 16)]
    cnts, lastm = plsc.scan_count(c16)
    plsc.addupdate_scatter(cnt_v, [c16], cnts.astype(jnp.float32), mask=lastm)
    return 0
  lax.fori_loop(0, _EPW // 16, hbody, 0)

  pltpu.sync_copy(cnt_v, out_hbm.at[wid])


def _inv_tc(cnt_parts):
  def body(c_ref, o_ref):
    s = jnp.sum(c_ref[...], axis=0, keepdims=True)
    o_ref[...] = 1.0 / jnp.maximum(s, 1.0)

  out = pl.pallas_call(
      body,
      out_shape=jax.ShapeDtypeStruct((1, _SEG), jnp.float32),
  )(cnt_parts)
  return out.reshape(_SEG)


@functools.partial(
    pl.kernel,
    out_type=jax.ShapeDtypeStruct((_E,), jnp.float32),
    mesh=_mesh(),
    scratch_types=[
        pltpu.VMEM((_EPW,), jnp.int32),
        pltpu.VMEM((_EPW,), jnp.int32),
        pltpu.VMEM((_SEG,), jnp.float32),
        pltpu.VMEM((_EPW,), jnp.float32),
    ],
    compiler_params=pltpu.CompilerParams(
        needs_layout_passes=False, use_tc_tiling_on_sc=False),
)
def _scale_kernel(dst_hbm, rel_hbm, inv_hbm, out_hbm, dst_v, rel_v, inv_v,
                  s_v):
  wid = lax.axis_index("s") * _NC + lax.axis_index("c")
  base = wid * _EPW
  pltpu.sync_copy(dst_hbm.at[pl.ds(base, _EPW)], dst_v)
  pltpu.sync_copy(rel_hbm.at[pl.ds(base, _EPW)], rel_v)
  pltpu.sync_copy(inv_hbm, inv_v)

  def body(i, _):
    sl = pl.ds(i * 16, 16)
    comb = dst_v[sl] * _R + rel_v[sl]
    s_v[sl] = plsc.load_gather(inv_v, [comb])
    return 0
  lax.fori_loop(0, _EPW // 16, body, 0)

  pltpu.sync_copy(s_v, out_hbm.at[pl.ds(base, _EPW)])


_CH = 96
_NFULL = (_EPW // _CH)
_TAIL = _EPW - _NFULL * _CH
_NPAIR = _NFULL // 2


@functools.partial(
    pl.kernel,
    out_type=jax.ShapeDtypeStruct((_NC, _N, _D), jnp.float32),
    mesh=_mesh(),
    scratch_types=[
        pltpu.VMEM((_EPW + 16,), jnp.int32),
        pltpu.VMEM((_EPW + 16,), jnp.int32),
        pltpu.VMEM((_EPW + 16,), jnp.int32),
        pltpu.VMEM((_EPW + 16,), jnp.float32),
        pltpu.VMEM((2, _CH), jnp.int32),
        pltpu.VMEM((2, _CH), jnp.int32),
        pltpu.VMEM((_CH, _D), jnp.float32),
        pltpu.VMEM((_CH, _D), jnp.float32),
        pltpu.VMEM((_D,), jnp.float32),
        pltpu.SemaphoreType.DMA,
        pltpu.SemaphoreType.DMA,
        pltpu.VMEM_SHARED((_N, _D), jnp.float32),
    ],
    compiler_params=pltpu.CompilerParams(
        needs_layout_passes=False, use_tc_tiling_on_sc=False),
)
def _scatter_kernel(src_hbm, dst_hbm, rel_hbm, w_hbm, s_hbm, root_hbm,
                    bias_hbm, out_hbm, src_v, dst_v, rel_v, s_v, gidx_v,
                    sidx_v, rows_v, rows1_v, bias_v, g0_sem, g1_sem, agg_sh):
  cid = lax.axis_index("c")
  sid = lax.axis_index("s")
  wid = sid * _NC + cid
  base = wid * _EPW

  d_src = pltpu.async_copy(src_hbm.at[pl.ds(base, _EPW)],
                           src_v.at[pl.ds(0, _EPW)], g0_sem)
  d_dst = pltpu.async_copy(dst_hbm.at[pl.ds(base, _EPW)],
                           dst_v.at[pl.ds(0, _EPW)], g0_sem)
  d_rel = pltpu.async_copy(rel_hbm.at[pl.ds(base, _EPW)],
                           rel_v.at[pl.ds(0, _EPW)], g1_sem)
  d_s = pltpu.async_copy(s_hbm.at[pl.ds(base, _EPW)],
                         s_v.at[pl.ds(0, _EPW)], g1_sem)
  pltpu.sync_copy(bias_hbm, bias_v)

  zrow = jnp.zeros((16,), jnp.float32)
  ich = [(_CH * k, _CH) for k in range(_RPW // _CH)]
  ich.append((_CH * (_RPW // _CH), _RPW - _CH * (_RPW // _CH)))

  def zero_rows(nrows):
    @plsc.parallel_loop(0, nrows, unroll=4)
    def zb(j):
      for t in range(_D // 16):
        rows_v[j, pl.ds(t * 16, 16)] = zrow

  @pl.when(cid == 1)
  def _():
    zero_rows(_CH)
    for ro, nr in ich:
      r0 = sid * _RPW + ro
      pltpu.sync_copy(rows_v.at[pl.ds(0, nr)], agg_sh.at[pl.ds(r0, nr)])

  @pl.when(cid == 0)
  def _():
    for ro, nr in ich:
      r0 = sid * _RPW + ro
      pltpu.sync_copy(root_hbm.at[pl.ds(r0, nr)], rows_v.at[pl.ds(0, nr)])

      @plsc.parallel_loop(0, nr, unroll=4)
      def ab(j):
        for t in range(_D // 16):
          sl = pl.ds(t * 16, 16)
          rows_v[j, sl] = rows_v[j, sl] + bias_v[sl]
      pltpu.sync_copy(rows_v.at[pl.ds(0, nr)], agg_sh.at[pl.ds(r0, nr)])

  d_src.wait()
  d_dst.wait()
  d_rel.wait()
  d_s.wait()
  plsc.subcore_barrier()

  def build_idx(off, bsel):
    @plsc.parallel_loop(0, _CH // 16, unroll=3)
    def ib(i):
      sl = pl.ds(off + i * 16, 16)
      so = pl.ds(i * 16, 16)
      gidx_v[bsel, so] = rel_v[sl] * _N + src_v[sl]
      sidx_v[bsel, so] = dst_v[sl]

  def scale_rows(rv, off, nedges):
    @plsc.parallel_loop(0, nedges, unroll=4)
    def sbody(j):
      s = s_v[pl.ds(off + j, 16)][0]
      bv = jnp.full((16,), s, jnp.float32)
      for t in range(_D // 16):
        sl = pl.ds(t * 16, 16)
        rv[j, sl] = rv[j, sl] * bv

  def pair(k, _):
    off0 = (2 * k) * _CH
    off1 = (2 * k + 1) * _CH
    build_idx(off0, 0)
    d0 = pltpu.async_copy(w_hbm.at[gidx_v.at[0]], rows_v, g0_sem)
    build_idx(off1, 1)
    d1 = pltpu.async_copy(w_hbm.at[gidx_v.at[1]], rows1_v, g1_sem)
    d0.wait()
    scale_rows(rows_v, off0, _CH)
    s0 = pltpu.async_copy(rows_v, agg_sh.at[sidx_v.at[0]], g0_sem, add=True)
    d1.wait()
    scale_rows(rows1_v, off1, _CH)
    s1 = pltpu.async_copy(rows1_v, agg_sh.at[sidx_v.at[1]], g1_sem, add=True)
    s0.wait()
    s1.wait()
    return 0
  lax.fori_loop(0, _NPAIR, pair, 0)

  toff = _NFULL * _CH
  gidx_v[0, pl.ds(0, 16)] = rel_v[pl.ds(toff, 16)] * _N + src_v[pl.ds(toff, 16)]
  sidx_v[0, pl.ds(0, 16)] = dst_v[pl.ds(toff, 16)]
  pltpu.sync_copy(w_hbm.at[gidx_v.at[0, pl.ds(0, _TAIL)]],
                  rows_v.at[pl.ds(0, _TAIL)])
  scale_rows(rows_v, toff, _TAIL)
  pltpu.sync_copy(rows_v.at[pl.ds(0, _TAIL)],
                  agg_sh.at[sidx_v.at[0, pl.ds(0, _TAIL)]], add=True)

  plsc.subcore_barrier()

  for ro, nr in ich:
    r0 = sid * _RPW + ro
    pltpu.sync_copy(agg_sh.at[pl.ds(r0, nr)], rows_v.at[pl.ds(0, nr)])
    pltpu.sync_copy(rows_v.at[pl.ds(0, nr)], out_hbm.at[cid, pl.ds(r0, nr)])


_GT = 25
_GPW = _BS // _GT


@functools.partial(
    pl.kernel,
    out_type=(
        jax.ShapeDtypeStruct((_BS, _D), jnp.float32),
        jax.ShapeDtypeStruct((_BS, 64), jnp.float32),
        jax.ShapeDtypeStruct((_BS,), jnp.float32),
        jax.ShapeDtypeStruct((_BS, _H), jnp.float32),
        jax.ShapeDtypeStruct((_BS,), jnp.float32),
    ),
    mesh=_mesh(),
    scratch_types=[
        pltpu.VMEM((_Q,), jnp.int32),
        pltpu.VMEM((_Q,), jnp.int32),
        pltpu.VMEM((_N,), jnp.float32),
        pltpu.VMEM((_GPW,), jnp.int32),
        pltpu.VMEM((_GPW,), jnp.int32),
        pltpu.VMEM((1, _GPW), jnp.int32),
        pltpu.VMEM((1, _GPW), jnp.int32),
        pltpu.VMEM((1, _GPW), jnp.int32),
        pltpu.VMEM((_GPW,), jnp.float32),
        pltpu.VMEM((_GPW,), jnp.float32),
        pltpu.VMEM((_GPW, _D), jnp.float32),
        pltpu.VMEM((_GPW, _D), jnp.float32),
        pltpu.VMEM((_GPW, 64), jnp.float32),
        pltpu.VMEM((_GPW, _H), jnp.float32),
    ],
    compiler_params=pltpu.CompilerParams(
        needs_layout_passes=False, use_tc_tiling_on_sc=False),
)
def _gather_kernel(qi_hbm, qf_hbm, qr_hbm, agg0_hbm, agg1_hbm, diff_hbm,
                   embd_hbm, b3_hbm, w3_hbm,
                   conc_hbm, dif_hbm, dqs_hbm, w3g_hbm, b3g_hbm,
                   qi_v, diff_v, b3_v, q_v, qr_v, idx_v, idxs_v, dr_v,
                   dqs_v, b3g_v, ca_v, cb_v, db_v, wb_v):
  wid = lax.axis_index("s") * _NC + lax.axis_index("c")

  @pl.when(wid < _GT)
  def _():
    base = wid * _GPW
    pltpu.sync_copy(qi_hbm, qi_v)
    pltpu.sync_copy(diff_hbm, diff_v)
    pltpu.sync_copy(b3_hbm, b3_v)
    pltpu.sync_copy(qf_hbm.at[pl.ds(base, _GPW)], q_v)
    pltpu.sync_copy(qr_hbm.at[pl.ds(base, _GPW)], qr_v)

    @plsc.parallel_loop(0, _GPW // 16, unroll=2)
    def ib(i):
      sl = pl.ds(i * 16, 16)
      qv = q_v[sl]
      qs = qr_v[sl]
      n16 = plsc.load_gather(qi_v, [qv])
      ns16 = plsc.load_gather(qi_v, [qs])
      idx_v[0, sl] = n16
      idxs_v[0, sl] = ns16
      dr_v[0, sl] = plsc.load_gather(diff_v, [qv])
      dqs_v[sl] = plsc.load_gather(diff_v, [qs]).astype(jnp.float32)
      b3g_v[sl] = plsc.load_gather(b3_v, [ns16])

    pltpu.sync_copy(agg0_hbm.at[idx_v.at[0]], ca_v)
    pltpu.sync_copy(agg1_hbm.at[idx_v.at[0]], cb_v)

    @plsc.parallel_loop(0, _GPW, unroll=4)
    def ab(j):
      for t in range(_D // 16):
        sl = pl.ds(t * 16, 16)
        ca_v[j, sl] = ca_v[j, sl] + cb_v[j, sl]

    pltpu.sync_copy(embd_hbm.at[dr_v.at[0]], db_v)
    pltpu.sync_copy(w3_hbm.at[idxs_v.at[0]], wb_v)

    pltpu.sync_copy(ca_v, conc_hbm.at[pl.ds(base, _GPW)])
    pltpu.sync_copy(db_v, dif_hbm.at[pl.ds(base, _GPW)])
    pltpu.sync_copy(wb_v, w3g_hbm.at[pl.ds(base, _GPW)])
    pltpu.sync_copy(dqs_v, dqs_hbm.at[pl.ds(base, _GPW)])
    pltpu.sync_copy(b3g_v, b3g_hbm.at[pl.ds(base, _GPW)])


def _dense_tc(conc, dif, yf, ea, W1, b1, W2, b2, Wih, Whh, bih, bhh, w3g3,
              b3g2, dqs2):
  def body(conc_ref, dif_ref, yf_ref, ea_ref, W1_ref, b1_ref, W2_ref, b2_ref,
           Wih_ref, Whh_ref, bih_ref, bhh_ref, w3g_ref, b3g_ref, dqs_ref,
           out_ref, res_ref):
    x1 = jnp.concatenate([conc_ref[...], dif_ref[...]], axis=1)
    text = lax.dot_general(x1, W1_ref[...], (((1,), (1,)), ((), ())),
                           preferred_element_type=jnp.float32) + b1_ref[...]
    a0 = ea_ref[0:1, :]
    a1 = ea_ref[1:2, :]
    ans = a0 + yf_ref[...] * (a1 - a0)
    x2 = jnp.concatenate([text, ans], axis=1)
    X = lax.dot_general(x2, W2_ref[...], (((1,), (1,)), ((), ())),
                        preferred_element_type=jnp.float32) + b2_ref[...]
    bgv = bih_ref[...] + bhh_ref[...]
    GX = lax.dot_general(X, Wih_ref[...], (((1,), (1,)), ((), ())),
                         preferred_element_type=jnp.float32) + bgv
    Whh = Whh_ref[...]

    h = jnp.zeros((50, _H), jnp.float32)
    c = jnp.zeros((50, _H), jnp.float32)
    hs = []
    for t in range(32):
      g = (GX[t * 50:(t + 1) * 50, :]
           + lax.dot_general(h, Whh, (((1,), (1,)), ((), ())),
                             preferred_element_type=jnp.float32))
      i_ = jax.nn.sigmoid(g[:, 0:_H])
      f_ = jax.nn.sigmoid(g[:, _H:2 * _H])
      gg = jnp.tanh(g[:, 2 * _H:3 * _H])
      o_ = jax.nn.sigmoid(g[:, 3 * _H:4 * _H])
      c = f_ * c + i_ * gg
      h = o_ * jnp.tanh(c)
      hs.append(h)

    outc = jnp.concatenate(hs, axis=0)
    out_ref[...] = outc
    pr = jnp.sum(outc * w3g_ref[...], axis=1, keepdims=True) + b3g_ref[...]
    ev = jax.nn.sigmoid(pr)
    res_ref[...] = jax.nn.sigmoid(ev - (dqs_ref[...] * 0.2 + 0.2))

  return pl.pallas_call(
      body,
      out_shape=(
          jax.ShapeDtypeStruct((_BS, _H), jnp.float32),
          jax.ShapeDtypeStruct((_BS, 1), jnp.float32),
      ),
  )(conc, dif, yf, ea, W1, b1, W2, b2, Wih, Whh, bih, bhh, w3g3, b3g2, dqs2)


_CT = 1024


def _e_tc(out, W3, b3r):
  def body(o_ref, w_ref, b_ref, e_ref):
    blk = jax.nn.sigmoid(
        lax.dot_general(o_ref[...], w_ref[...], (((1,), (1,)), ((), ())),
                        preferred_element_type=jnp.float32) + b_ref[...])
    for b in range(32):
      e_ref[b] = blk[b * 50:(b + 1) * 50, :]

  grid = pl.cdiv(_N, _CT)
  return pl.pallas_call(
      body,
      grid=(grid,),
      in_specs=[
          pl.BlockSpec((_BS, _H), lambda j: (0, 0)),
          pl.BlockSpec((_CT, _H), lambda j: (j, 0)),
          pl.BlockSpec((1, _CT), lambda j: (0, j)),
      ],
      out_specs=pl.BlockSpec((32, 50, _CT), lambda j: (0, 0, j)),
      out_shape=jax.ShapeDtypeStruct((32, 50, _N), jnp.float32),
  )(out, W3, b3r)


def kernel(Q_info, edge_index, edge_type, q, y, diff, device, rgcn_weight,
           rgcn_root, rgcn_bias, emb_diff, emb_answer, W1, b1, W2, b2,
           W_ih, W_hh, b_ih, b_hh, W3, b3):
  src = edge_index[0].astype(jnp.int32)
  dst = edge_index[1].astype(jnp.int32)
  rel = edge_type.astype(jnp.int32)

  cnt_parts = _count_kernel(dst, rel)
  inv = _inv_tc(cnt_parts)

  s_all = _scale_kernel(dst, rel, inv)
  wflat = rgcn_weight.reshape(_SEG, _D)
  agg = _scatter_kernel(src, dst, rel, wflat, s_all, rgcn_root, rgcn_bias)

  qf = q.reshape(-1).astype(jnp.int32)
  qr = jnp.roll(qf, -1)
  conc, dif, dqs, w3g, b3g = _gather_kernel(
      Q_info.astype(jnp.int32), qf, qr, agg[0], agg[1],
      diff.astype(jnp.int32), emb_diff, b3, W3)

  yf = y.reshape(_BS, 1).astype(jnp.float32)
  out2, res_full = _dense_tc(
      conc, dif, yf, emb_answer, W1, b1.reshape(1, -1), W2, b2.reshape(1, -1),
      W_ih, W_hh, b_ih.reshape(1, -1), b_hh.reshape(1, -1),
      w3g, b3g.reshape(_BS, 1), dqs.reshape(_BS, 1))

  e = _e_tc(out2, W3, b3.reshape(1, _N))

  res = res_full.reshape(32, 50)[:, :49]
  return (res, e)


# speedup vs baseline: 10.2056x; 1.2101x over previous
import functools

import jax
import jax.numpy as jnp
from jax import lax
from jax.experimental import pallas as pl
from jax.experimental.pallas import tpu as pltpu
from jax.experimental.pallas import tpu_sc as plsc

_N = 10000
_R = 4
_D = 128
_E = 160000
_SEG = _N * _R
_NC = 2
_NS = 16
_NW = _NC * _NS
_EPW = _E // _NW
_Q = 20000
_BS = 1600
_H = 256
_RPW = _N // _NS


def _mesh():
  return plsc.VectorSubcoreMesh(
      core_axis_name="c", subcore_axis_name="s",
      num_cores=_NC, num_subcores=_NS)


@functools.partial(
    pl.kernel,
    out_type=jax.ShapeDtypeStruct((_NW, _SEG), jnp.float32),
    mesh=_mesh(),
    scratch_types=[
        pltpu.VMEM((_EPW,), jnp.int32),
        pltpu.VMEM((_EPW,), jnp.int32),
        pltpu.VMEM((_SEG,), jnp.float32),
    ],
    compiler_params=pltpu.CompilerParams(
        needs_layout_passes=False, use_tc_tiling_on_sc=False),
)
def _count_kernel(dst_hbm, rel_hbm, out_hbm, dst_v, comb_v, cnt_v):
  wid = lax.axis_index("s") * _NC + lax.axis_index("c")
  base = wid * _EPW
  pltpu.sync_copy(dst_hbm.at[pl.ds(base, _EPW)], dst_v)
  pltpu.sync_copy(rel_hbm.at[pl.ds(base, _EPW)], comb_v)

  def zbody(i, _):
    cnt_v[pl.ds(i * 16, 16)] = jnp.zeros((16,), jnp.float32)
    return 0
  lax.fori_loop(0, _SEG // 16, zbody, 0)

  def cbody(i, _):
    sl = pl.ds(i * 16, 16)
    comb_v[sl] = dst_v[sl] * _R + comb_v[sl]
    return 0
  lax.fori_loop(0, _EPW // 16, cbody, 0)

  def hbody(i, _):
    c16 = comb_v[pl.ds(i * 16, 16)]
    cnts, lastm = plsc.scan_count(c16)
    plsc.addupdate_scatter(cnt_v, [c16], cnts.astype(jnp.float32), mask=lastm)
    return 0
  lax.fori_loop(0, _EPW // 16, hbody, 0)

  pltpu.sync_copy(cnt_v, out_hbm.at[wid])


def _inv_tc(cnt_parts):
  def body(c_ref, o_ref):
    s = jnp.sum(c_ref[...], axis=0, keepdims=True)
    o_ref[...] = 1.0 / jnp.maximum(s, 1.0)

  out = pl.pallas_call(
      body,
      out_shape=jax.ShapeDtypeStruct((1, _SEG), jnp.float32),
  )(cnt_parts)
  return out.reshape(_SEG)


@functools.partial(
    pl.kernel,
    out_type=jax.ShapeDtypeStruct((_E,), jnp.float32),
    mesh=_mesh(),
    scratch_types=[
        pltpu.VMEM((_EPW,), jnp.int32),
        pltpu.VMEM((_EPW,), jnp.int32),
        pltpu.VMEM((_SEG,), jnp.float32),
        pltpu.VMEM((_EPW,), jnp.float32),
    ],
    compiler_params=pltpu.CompilerParams(
        needs_layout_passes=False, use_tc_tiling_on_sc=False),
)
def _scale_kernel(dst_hbm, rel_hbm, inv_hbm, out_hbm, dst_v, rel_v, inv_v,
                  s_v):
  wid = lax.axis_index("s") * _NC + lax.axis_index("c")
  base = wid * _EPW
  pltpu.sync_copy(dst_hbm.at[pl.ds(base, _EPW)], dst_v)
  pltpu.sync_copy(rel_hbm.at[pl.ds(base, _EPW)], rel_v)
  pltpu.sync_copy(inv_hbm, inv_v)

  def body(i, _):
    sl = pl.ds(i * 16, 16)
    comb = dst_v[sl] * _R + rel_v[sl]
    s_v[sl] = plsc.load_gather(inv_v, [comb])
    return 0
  lax.fori_loop(0, _EPW // 16, body, 0)

  pltpu.sync_copy(s_v, out_hbm.at[pl.ds(base, _EPW)])


_CH = 96
_NFULL = (_EPW // _CH)
_TAIL = _EPW - _NFULL * _CH
_NPAIR = _NFULL // 2


@functools.partial(
    pl.kernel,
    out_type=jax.ShapeDtypeStruct((_NC, _N, _D), jnp.float32),
    mesh=_mesh(),
    scratch_types=[
        pltpu.VMEM((_EPW + 16,), jnp.int32),
        pltpu.VMEM((_EPW + 16,), jnp.int32),
        pltpu.VMEM((_EPW + 16,), jnp.int32),
        pltpu.VMEM((_EPW + 16,), jnp.float32),
        pltpu.VMEM((2, _CH), jnp.int32),
        pltpu.VMEM((2, _CH), jnp.int32),
        pltpu.VMEM((_CH, _D), jnp.float32),
        pltpu.VMEM((_CH, _D), jnp.float32),
        pltpu.VMEM((_D,), jnp.float32),
        pltpu.SemaphoreType.DMA,
        pltpu.SemaphoreType.DMA,
        pltpu.VMEM_SHARED((_N, _D), jnp.float32),
    ],
    compiler_params=pltpu.CompilerParams(
        needs_layout_passes=False, use_tc_tiling_on_sc=False),
)
def _scatter_kernel(src_hbm, dst_hbm, rel_hbm, w_hbm, s_hbm, root_hbm,
                    bias_hbm, out_hbm, src_v, dst_v, rel_v, s_v, gidx_v,
                    sidx_v, rows_v, rows1_v, bias_v, g0_sem, g1_sem, agg_sh):
  cid = lax.axis_index("c")
  sid = lax.axis_index("s")
  wid = sid * _NC + cid
  base = wid * _EPW

  d_src = pltpu.async_copy(src_hbm.at[pl.ds(base, _EPW)],
                           src_v.at[pl.ds(0, _EPW)], g0_sem)
  d_dst = pltpu.async_copy(dst_hbm.at[pl.ds(base, _EPW)],
                           dst_v.at[pl.ds(0, _EPW)], g0_sem)
  d_rel = pltpu.async_copy(rel_hbm.at[pl.ds(base, _EPW)],
                           rel_v.at[pl.ds(0, _EPW)], g1_sem)
  d_s = pltpu.async_copy(s_hbm.at[pl.ds(base, _EPW)],
                         s_v.at[pl.ds(0, _EPW)], g1_sem)
  pltpu.sync_copy(bias_hbm, bias_v)

  zrow = jnp.zeros((16,), jnp.float32)
  ich = [(_CH * k, _CH) for k in range(_RPW // _CH)]
  ich.append((_CH * (_RPW // _CH), _RPW - _CH * (_RPW // _CH)))

  def zero_rows(nrows):
    @plsc.parallel_loop(0, nrows, unroll=4)
    def zb(j):
      for t in range(_D // 16):
        rows_v[j, pl.ds(t * 16, 16)] = zrow

  @pl.when(cid == 1)
  def _():
    zero_rows(_CH)
    for ro, nr in ich:
      r0 = sid * _RPW + ro
      pltpu.sync_copy(rows_v.at[pl.ds(0, nr)], agg_sh.at[pl.ds(r0, nr)])

  @pl.when(cid == 0)
  def _():
    for ro, nr in ich:
      r0 = sid * _RPW + ro
      pltpu.sync_copy(root_hbm.at[pl.ds(r0, nr)], rows_v.at[pl.ds(0, nr)])

      @plsc.parallel_loop(0, nr, unroll=4)
      def ab(j):
        for t in range(_D // 16):
          sl = pl.ds(t * 16, 16)
          rows_v[j, sl] = rows_v[j, sl] + bias_v[sl]
      pltpu.sync_copy(rows_v.at[pl.ds(0, nr)], agg_sh.at[pl.ds(r0, nr)])

  d_src.wait()
  d_dst.wait()
  d_rel.wait()
  d_s.wait()
  plsc.subcore_barrier()

  def build_idx(off, bsel):
    @plsc.parallel_loop(0, _CH // 16, unroll=3)
    def ib(i):
      sl = pl.ds(off + i * 16, 16)
      so = pl.ds(i * 16, 16)
      gidx_v[bsel, so] = rel_v[sl] * _N + src_v[sl]
      sidx_v[bsel, so] = dst_v[sl]

  def scale_rows(rv, off, nedges):
    @plsc.parallel_loop(0, nedges, unroll=4)
    def sbody(j):
      s = s_v[pl.ds(off + j, 16)][0]
      bv = jnp.full((16,), s, jnp.float32)
      for t in range(_D // 16):
        sl = pl.ds(t * 16, 16)
        rv[j, sl] = rv[j, sl] * bv

  def pair(k, _):
    off0 = (2 * k) * _CH
    off1 = (2 * k + 1) * _CH
    build_idx(off0, 0)
    d0 = pltpu.async_copy(w_hbm.at[gidx_v.at[0]], rows_v, g0_sem)
    build_idx(off1, 1)
    d1 = pltpu.async_copy(w_hbm.at[gidx_v.at[1]], rows1_v, g1_sem)
    d0.wait()
    scale_rows(rows_v, off0, _CH)
    s0 = pltpu.async_copy(rows_v, agg_sh.at[sidx_v.at[0]], g0_sem, add=True)
    d1.wait()
    scale_rows(rows1_v, off1, _CH)
    s1 = pltpu.async_copy(rows1_v, agg_sh.at[sidx_v.at[1]], g1_sem, add=True)
    s0.wait()
    s1.wait()
    return 0
  lax.fori_loop(0, _NPAIR, pair, 0)

  toff = _NFULL * _CH
  gidx_v[0, pl.ds(0, 16)] = rel_v[pl.ds(toff, 16)] * _N + src_v[pl.ds(toff, 16)]
  sidx_v[0, pl.ds(0, 16)] = dst_v[pl.ds(toff, 16)]
  pltpu.sync_copy(w_hbm.at[gidx_v.at[0, pl.ds(0, _TAIL)]],
                  rows_v.at[pl.ds(0, _TAIL)])
  scale_rows(rows_v, toff, _TAIL)
  pltpu.sync_copy(rows_v.at[pl.ds(0, _TAIL)],
                  agg_sh.at[sidx_v.at[0, pl.ds(0, _TAIL)]], add=True)

  plsc.subcore_barrier()

  for ro, nr in ich:
    r0 = sid * _RPW + ro
    pltpu.sync_copy(agg_sh.at[pl.ds(r0, nr)], rows_v.at[pl.ds(0, nr)])
    pltpu.sync_copy(rows_v.at[pl.ds(0, nr)], out_hbm.at[cid, pl.ds(r0, nr)])


_GT = 25
_GPW = _BS // _GT


@functools.partial(
    pl.kernel,
    out_type=(
        jax.ShapeDtypeStruct((_BS, _D), jnp.float32),
        jax.ShapeDtypeStruct((_BS, 64), jnp.float32),
        jax.ShapeDtypeStruct((_BS,), jnp.float32),
        jax.ShapeDtypeStruct((_BS, _H), jnp.float32),
        jax.ShapeDtypeStruct((_BS,), jnp.float32),
    ),
    mesh=_mesh(),
    scratch_types=[
        pltpu.VMEM((_Q,), jnp.int32),
        pltpu.VMEM((_Q,), jnp.int32),
        pltpu.VMEM((_N,), jnp.float32),
        pltpu.VMEM((_GPW,), jnp.int32),
        pltpu.VMEM((_GPW,), jnp.int32),
        pltpu.VMEM((1, _GPW), jnp.int32),
        pltpu.VMEM((1, _GPW), jnp.int32),
        pltpu.VMEM((1, _GPW), jnp.int32),
        pltpu.VMEM((_GPW,), jnp.float32),
        pltpu.VMEM((_GPW,), jnp.float32),
        pltpu.VMEM((_GPW, _D), jnp.float32),
        pltpu.VMEM((_GPW, _D), jnp.float32),
        pltpu.VMEM((_GPW, 64), jnp.float32),
        pltpu.VMEM((_GPW, _H), jnp.float32),
    ],
    compiler_params=pltpu.CompilerParams(
        needs_layout_passes=False, use_tc_tiling_on_sc=False),
)
def _gather_kernel(qi_hbm, qf_hbm, qr_hbm, agg0_hbm, agg1_hbm, diff_hbm,
                   embd_hbm, b3_hbm, w3_hbm,
                   conc_hbm, dif_hbm, dqs_hbm, w3g_hbm, b3g_hbm,
                   qi_v, diff_v, b3_v, q_v, qr_v, idx_v, idxs_v, dr_v,
                   dqs_v, b3g_v, ca_v, cb_v, db_v, wb_v):
  wid = lax.axis_index("s") * _NC + lax.axis_index("c")

  @pl.when(wid < _GT)
  def _():
    base = wid * _GPW
    pltpu.sync_copy(qi_hbm, qi_v)
    pltpu.sync_copy(diff_hbm, diff_v)
    pltpu.sync_copy(b3_hbm, b3_v)
    pltpu.sync_copy(qf_hbm.at[pl.ds(base, _GPW)], q_v)
    pltpu.sync_copy(qr_hbm.at[pl.ds(base, _GPW)], qr_v)

    @plsc.parallel_loop(0, _GPW // 16, unroll=2)
    def ib(i):
      sl = pl.ds(i * 16, 16)
      qv = q_v[sl]
      qs = qr_v[sl]
      n16 = plsc.load_gather(qi_v, [qv])
      ns16 = plsc.load_gather(qi_v, [qs])
      idx_v[0, sl] = n16
      idxs_v[0, sl] = ns16
      dr_v[0, sl] = plsc.load_gather(diff_v, [qv])
      dqs_v[sl] = plsc.load_gather(diff_v, [qs]).astype(jnp.float32)
      b3g_v[sl] = plsc.load_gather(b3_v, [ns16])

    pltpu.sync_copy(agg0_hbm.at[idx_v.at[0]], ca_v)
    pltpu.sync_copy(agg1_hbm.at[idx_v.at[0]], cb_v)

    @plsc.parallel_loop(0, _GPW, unroll=4)
    def ab(j):
      for t in range(_D // 16):
        sl = pl.ds(t * 16, 16)
        ca_v[j, sl] = ca_v[j, sl] + cb_v[j, sl]

    pltpu.sync_copy(embd_hbm.at[dr_v.at[0]], db_v)
    pltpu.sync_copy(w3_hbm.at[idxs_v.at[0]], wb_v)

    pltpu.sync_copy(ca_v, conc_hbm.at[pl.ds(base, _GPW)])
    pltpu.sync_copy(db_v, dif_hbm.at[pl.ds(base, _GPW)])
    pltpu.sync_copy(wb_v, w3g_hbm.at[pl.ds(base, _GPW)])
    pltpu.sync_copy(dqs_v, dqs_hbm.at[pl.ds(base, _GPW)])
    pltpu.sync_copy(b3g_v, b3g_hbm.at[pl.ds(base, _GPW)])


def _dense_tc(conc, dif, yf, ea, W1, b1, W2, b2, Wih, Whh, bih, bhh, w3g3,
              b3g2, dqs2):
  def body(conc_ref, dif_ref, yf_ref, ea_ref, W1_ref, b1_ref, W2_ref, b2_ref,
           Wih_ref, Whh_ref, bih_ref, bhh_ref, w3g_ref, b3g_ref, dqs_ref,
           out_ref, res_ref):
    x1 = jnp.concatenate([conc_ref[...], dif_ref[...]], axis=1)
    text = lax.dot_general(x1, W1_ref[...], (((1,), (1,)), ((), ())),
                           preferred_element_type=jnp.float32) + b1_ref[...]
    a0 = ea_ref[0:1, :]
    a1 = ea_ref[1:2, :]
    ans = a0 + yf_ref[...] * (a1 - a0)
    x2 = jnp.concatenate([text, ans], axis=1)
    X = lax.dot_general(x2, W2_ref[...], (((1,), (1,)), ((), ())),
                        preferred_element_type=jnp.float32) + b2_ref[...]
    bgv = bih_ref[...] + bhh_ref[...]
    GX = lax.dot_general(X, Wih_ref[...], (((1,), (1,)), ((), ())),
                         preferred_element_type=jnp.float32) + bgv
    Whh = Whh_ref[...]

    h = jnp.zeros((50, _H), jnp.float32)
    c = jnp.zeros((50, _H), jnp.float32)
    hs = []
    for t in range(32):
      g = (GX[t * 50:(t + 1) * 50, :]
           + lax.dot_general(h, Whh, (((1,), (1,)), ((), ())),
                             preferred_element_type=jnp.float32))
      i_ = jax.nn.sigmoid(g[:, 0:_H])
      f_ = jax.nn.sigmoid(g[:, _H:2 * _H])
      gg = jnp.tanh(g[:, 2 * _H:3 * _H])
      o_ = jax.nn.sigmoid(g[:, 3 * _H:4 * _H])
      c = f_ * c + i_ * gg
      h = o_ * jnp.tanh(c)
      hs.append(h)

    outv = jnp.stack(hs, axis=0)
    out_ref[...] = jnp.swapaxes(outv, 0, 1)
    outc = jnp.concatenate(hs, axis=0)
    pr = jnp.sum(outc * w3g_ref[...], axis=1, keepdims=True) + b3g_ref[...]
    ev = jax.nn.sigmoid(pr)
    res_ref[...] = jax.nn.sigmoid(ev - (dqs_ref[...] * 0.2 + 0.2))

  return pl.pallas_call(
      body,
      out_shape=(
          jax.ShapeDtypeStruct((50, 32, _H), jnp.float32),
          jax.ShapeDtypeStruct((_BS, 1), jnp.float32),
      ),
  )(conc, dif, yf, ea, W1, b1, W2, b2, Wih, Whh, bih, bhh, w3g3, b3g2, dqs2)


_CT = 1024


def _e_tc(out, W3, b3r):
  def body(o_ref, w_ref, b_ref, e_ref):
    e_ref[...] = jax.nn.sigmoid(
        lax.dot_general(o_ref[...], w_ref[...], (((1,), (1,)), ((), ())),
                        preferred_element_type=jnp.float32) + b_ref[...])

  grid = pl.cdiv(_N, _CT)
  return pl.pallas_call(
      body,
      grid=(grid,),
      in_specs=[
          pl.BlockSpec((_BS, _H), lambda j: (0, 0)),
          pl.BlockSpec((_CT, _H), lambda j: (j, 0)),
          pl.BlockSpec((1, _CT), lambda j: (0, j)),
      ],
      out_specs=pl.BlockSpec((_BS, _CT), lambda j: (0, j)),
      out_shape=jax.ShapeDtypeStruct((_BS, _N), jnp.float32),
  )(out, W3, b3r)


def kernel(Q_info, edge_index, edge_type, q, y, diff, device, rgcn_weight,
           rgcn_root, rgcn_bias, emb_diff, emb_answer, W1, b1, W2, b2,
           W_ih, W_hh, b_ih, b_hh, W3, b3):
  src = edge_index[0].astype(jnp.int32)
  dst = edge_index[1].astype(jnp.int32)
  rel = edge_type.astype(jnp.int32)

  cnt_parts = _count_kernel(dst, rel)
  inv = _inv_tc(cnt_parts)

  s_all = _scale_kernel(dst, rel, inv)
  wflat = rgcn_weight.reshape(_SEG, _D)
  agg = _scatter_kernel(src, dst, rel, wflat, s_all, rgcn_root, rgcn_bias)

  qf = q.reshape(-1).astype(jnp.int32)
  qr = jnp.roll(qf, -1)
  conc, dif, dqs, w3g, b3g = _gather_kernel(
      Q_info.astype(jnp.int32), qf, qr, agg[0], agg[1],
      diff.astype(jnp.int32), emb_diff, b3, W3)

  yf = y.reshape(_BS, 1).astype(jnp.float32)
  outsm, res_full = _dense_tc(
      conc, dif, yf, emb_answer, W1, b1.reshape(1, -1), W2, b2.reshape(1, -1),
      W_ih, W_hh, b_ih.reshape(1, -1), b_hh.reshape(1, -1),
      w3g, b3g.reshape(_BS, 1), dqs.reshape(_BS, 1))

  e2 = _e_tc(outsm.reshape(_BS, _H), W3, b3.reshape(1, _N))
  e = jnp.swapaxes(e2.reshape(50, 32, _N), 0, 1)

  res = res_full.reshape(32, 50)[:, :49]
  return (res, e)


# speedup vs baseline: 10.2975x; 1.0090x over previous
import functools

import jax
import jax.numpy as jnp
from jax import lax
from jax.experimental import pallas as pl
from jax.experimental.pallas import tpu as pltpu
from jax.experimental.pallas import tpu_sc as plsc

_N = 10000
_R = 4
_D = 128
_E = 160000
_SEG = _N * _R
_NC = 2
_NS = 16
_NW = _NC * _NS
_EPW = _E // _NW
_Q = 20000
_BS = 1600
_H = 256
_RPW = _N // _NS


def _mesh():
  return plsc.VectorSubcoreMesh(
      core_axis_name="c", subcore_axis_name="s",
      num_cores=_NC, num_subcores=_NS)


@functools.partial(
    pl.kernel,
    out_type=jax.ShapeDtypeStruct((_NW, _SEG), jnp.float32),
    mesh=_mesh(),
    scratch_types=[
        pltpu.VMEM((_EPW,), jnp.int32),
        pltpu.VMEM((_EPW,), jnp.int32),
        pltpu.VMEM((_SEG,), jnp.float32),
    ],
    compiler_params=pltpu.CompilerParams(
        needs_layout_passes=False, use_tc_tiling_on_sc=False),
)
def _count_kernel(dst_hbm, rel_hbm, out_hbm, dst_v, comb_v, cnt_v):
  wid = lax.axis_index("s") * _NC + lax.axis_index("c")
  base = wid * _EPW
  pltpu.sync_copy(dst_hbm.at[pl.ds(base, _EPW)], dst_v)
  pltpu.sync_copy(rel_hbm.at[pl.ds(base, _EPW)], comb_v)

  def zbody(i, _):
    cnt_v[pl.ds(i * 16, 16)] = jnp.zeros((16,), jnp.float32)
    return 0
  lax.fori_loop(0, _SEG // 16, zbody, 0)

  def cbody(i, _):
    sl = pl.ds(i * 16, 16)
    comb_v[sl] = dst_v[sl] * _R + comb_v[sl]
    return 0
  lax.fori_loop(0, _EPW // 16, cbody, 0)

  def hbody(i, _):
    c16 = comb_v[pl.ds(i * 16, 16)]
    cnts, lastm = plsc.scan_count(c16)
    plsc.addupdate_scatter(cnt_v, [c16], cnts.astype(jnp.float32), mask=lastm)
    return 0
  lax.fori_loop(0, _EPW // 16, hbody, 0)

  pltpu.sync_copy(cnt_v, out_hbm.at[wid])


def _inv_tc(cnt_parts):
  def body(c_ref, o_ref):
    s = jnp.sum(c_ref[...], axis=0, keepdims=True)
    o_ref[...] = 1.0 / jnp.maximum(s, 1.0)

  out = pl.pallas_call(
      body,
      out_shape=jax.ShapeDtypeStruct((1, _SEG), jnp.float32),
  )(cnt_parts)
  return out.reshape(_SEG)


@functools.partial(
    pl.kernel,
    out_type=jax.ShapeDtypeStruct((_E,), jnp.float32),
    mesh=_mesh(),
    scratch_types=[
        pltpu.VMEM((_EPW,), jnp.int32),
        pltpu.VMEM((_EPW,), jnp.int32),
        pltpu.VMEM((_SEG,), jnp.float32),
        pltpu.VMEM((_EPW,), jnp.float32),
    ],
    compiler_params=pltpu.CompilerParams(
        needs_layout_passes=False, use_tc_tiling_on_sc=False),
)
def _scale_kernel(dst_hbm, rel_hbm, inv_hbm, out_hbm, dst_v, rel_v, inv_v,
                  s_v):
  wid = lax.axis_index("s") * _NC + lax.axis_index("c")
  base = wid * _EPW
  pltpu.sync_copy(dst_hbm.at[pl.ds(base, _EPW)], dst_v)
  pltpu.sync_copy(rel_hbm.at[pl.ds(base, _EPW)], rel_v)
  pltpu.sync_copy(inv_hbm, inv_v)

  def body(i, _):
    sl = pl.ds(i * 16, 16)
    comb = dst_v[sl] * _R + rel_v[sl]
    s_v[sl] = plsc.load_gather(inv_v, [comb])
    return 0
  lax.fori_loop(0, _EPW // 16, body, 0)

  pltpu.sync_copy(s_v, out_hbm.at[pl.ds(base, _EPW)])


_CH = 96
_NFULL = (_EPW // _CH)
_TAIL = _EPW - _NFULL * _CH
_NPAIR = _NFULL // 2


@functools.partial(
    pl.kernel,
    out_type=(
        jax.ShapeDtypeStruct((_N, _D), jnp.float32),
        jax.ShapeDtypeStruct((_N, _D), jnp.float32),
    ),
    mesh=_mesh(),
    scratch_types=[
        pltpu.VMEM((_EPW + 16,), jnp.int32),
        pltpu.VMEM((_EPW + 16,), jnp.int32),
        pltpu.VMEM((_EPW + 16,), jnp.int32),
        pltpu.VMEM((_EPW + 16,), jnp.float32),
        pltpu.VMEM((2, _CH), jnp.int32),
        pltpu.VMEM((2, _CH), jnp.int32),
        pltpu.VMEM((_CH, _D), jnp.float32),
        pltpu.VMEM((_CH, _D), jnp.float32),
        pltpu.VMEM((_D,), jnp.float32),
        pltpu.SemaphoreType.DMA,
        pltpu.SemaphoreType.DMA,
        pltpu.VMEM_SHARED((_N, _D), jnp.float32),
    ],
    compiler_params=pltpu.CompilerParams(
        needs_layout_passes=False, use_tc_tiling_on_sc=False),
)
def _scatter_kernel(src_hbm, dst_hbm, rel_hbm, w_hbm, s_hbm, root_hbm,
                    bias_hbm, out0_hbm, out1_hbm, src_v, dst_v, rel_v, s_v,
                    gidx_v, sidx_v, rows_v, rows1_v, bias_v, g0_sem, g1_sem,
                    agg_sh):
  cid = lax.axis_index("c")
  sid = lax.axis_index("s")
  wid = sid * _NC + cid
  base = wid * _EPW

  d_src = pltpu.async_copy(src_hbm.at[pl.ds(base, _EPW)],
                           src_v.at[pl.ds(0, _EPW)], g0_sem)
  d_dst = pltpu.async_copy(dst_hbm.at[pl.ds(base, _EPW)],
                           dst_v.at[pl.ds(0, _EPW)], g0_sem)
  d_rel = pltpu.async_copy(rel_hbm.at[pl.ds(base, _EPW)],
                           rel_v.at[pl.ds(0, _EPW)], g1_sem)
  d_s = pltpu.async_copy(s_hbm.at[pl.ds(base, _EPW)],
                         s_v.at[pl.ds(0, _EPW)], g1_sem)
  pltpu.sync_copy(bias_hbm, bias_v)

  zrow = jnp.zeros((16,), jnp.float32)
  ich = [(_CH * k, _CH) for k in range(_RPW // _CH)]
  ich.append((_CH * (_RPW // _CH), _RPW - _CH * (_RPW // _CH)))

  def zero_rows(nrows):
    @plsc.parallel_loop(0, nrows, unroll=4)
    def zb(j):
      for t in range(_D // 16):
        rows_v[j, pl.ds(t * 16, 16)] = zrow

  @pl.when(cid == 1)
  def _():
    zero_rows(_CH)
    for ro, nr in ich:
      r0 = sid * _RPW + ro
      pltpu.sync_copy(rows_v.at[pl.ds(0, nr)], agg_sh.at[pl.ds(r0, nr)])

  @pl.when(cid == 0)
  def _():
    for ro, nr in ich:
      r0 = sid * _RPW + ro
      pltpu.sync_copy(root_hbm.at[pl.ds(r0, nr)], rows_v.at[pl.ds(0, nr)])

      @plsc.parallel_loop(0, nr, unroll=4)
      def ab(j):
        for t in range(_D // 16):
          sl = pl.ds(t * 16, 16)
          rows_v[j, sl] = rows_v[j, sl] + bias_v[sl]
      pltpu.sync_copy(rows_v.at[pl.ds(0, nr)], agg_sh.at[pl.ds(r0, nr)])

  d_src.wait()
  d_dst.wait()
  d_rel.wait()
  d_s.wait()
  plsc.subcore_barrier()

  def build_idx(off, bsel):
    @plsc.parallel_loop(0, _CH // 16, unroll=3)
    def ib(i):
      sl = pl.ds(off + i * 16, 16)
      so = pl.ds(i * 16, 16)
      gidx_v[bsel, so] = rel_v[sl] * _N + src_v[sl]
      sidx_v[bsel, so] = dst_v[sl]

  def scale_rows(rv, off, nedges):
    @plsc.parallel_loop(0, nedges, unroll=8)
    def sbody(j):
      s = s_v[pl.ds(off + j, 16)][0]
      bv = jnp.full((16,), s, jnp.float32)
      for t in range(_D // 16):
        sl = pl.ds(t * 16, 16)
        rv[j, sl] = rv[j, sl] * bv

  def pair(k, _):
    off0 = (2 * k) * _CH
    off1 = (2 * k + 1) * _CH
    build_idx(off0, 0)
    d0 = pltpu.async_copy(w_hbm.at[gidx_v.at[0]], rows_v, g0_sem)
    build_idx(off1, 1)
    d1 = pltpu.async_copy(w_hbm.at[gidx_v.at[1]], rows1_v, g1_sem)
    d0.wait()
    scale_rows(rows_v, off0, _CH)
    s0 = pltpu.async_copy(rows_v, agg_sh.at[sidx_v.at[0]], g0_sem, add=True)
    d1.wait()
    scale_rows(rows1_v, off1, _CH)
    s1 = pltpu.async_copy(rows1_v, agg_sh.at[sidx_v.at[1]], g1_sem, add=True)
    s0.wait()
    s1.wait()
    return 0
  lax.fori_loop(0, _NPAIR, pair, 0)

  toff = _NFULL * _CH
  gidx_v[0, pl.ds(0, 16)] = rel_v[pl.ds(toff, 16)] * _N + src_v[pl.ds(toff, 16)]
  sidx_v[0, pl.ds(0, 16)] = dst_v[pl.ds(toff, 16)]
  pltpu.sync_copy(w_hbm.at[gidx_v.at[0, pl.ds(0, _TAIL)]],
                  rows_v.at[pl.ds(0, _TAIL)])
  scale_rows(rows_v, toff, _TAIL)
  pltpu.sync_copy(rows_v.at[pl.ds(0, _TAIL)],
                  agg_sh.at[sidx_v.at[0, pl.ds(0, _TAIL)]], add=True)

  plsc.subcore_barrier()

  @pl.when(cid == 0)
  def _():
    for ro, nr in ich:
      r0 = sid * _RPW + ro
      pltpu.sync_copy(agg_sh.at[pl.ds(r0, nr)], rows_v.at[pl.ds(0, nr)])
      pltpu.sync_copy(rows_v.at[pl.ds(0, nr)], out0_hbm.at[pl.ds(r0, nr)])

  @pl.when(cid == 1)
  def _():
    for ro, nr in ich:
      r0 = sid * _RPW + ro
      pltpu.sync_copy(agg_sh.at[pl.ds(r0, nr)], rows_v.at[pl.ds(0, nr)])
      pltpu.sync_copy(rows_v.at[pl.ds(0, nr)], out1_hbm.at[pl.ds(r0, nr)])


_GT = 25
_GPW = _BS // _GT


@functools.partial(
    pl.kernel,
    out_type=(
        jax.ShapeDtypeStruct((_BS, _D), jnp.float32),
        jax.ShapeDtypeStruct((_BS, 64), jnp.float32),
        jax.ShapeDtypeStruct((_BS,), jnp.float32),
        jax.ShapeDtypeStruct((_BS, _H), jnp.float32),
        jax.ShapeDtypeStruct((_BS,), jnp.float32),
    ),
    mesh=_mesh(),
    scratch_types=[
        pltpu.VMEM((_Q,), jnp.int32),
        pltpu.VMEM((_Q,), jnp.int32),
        pltpu.VMEM((_N,), jnp.float32),
        pltpu.VMEM((_GPW,), jnp.int32),
        pltpu.VMEM((_GPW,), jnp.int32),
        pltpu.VMEM((1, _GPW), jnp.int32),
        pltpu.VMEM((1, _GPW), jnp.int32),
        pltpu.VMEM((1, _GPW), jnp.int32),
        pltpu.VMEM((_GPW,), jnp.float32),
        pltpu.VMEM((_GPW,), jnp.float32),
        pltpu.VMEM((_GPW, _D), jnp.float32),
        pltpu.VMEM((_GPW, _D), jnp.float32),
        pltpu.VMEM((_GPW, 64), jnp.float32),
        pltpu.VMEM((_GPW, _H), jnp.float32),
    ],
    compiler_params=pltpu.CompilerParams(
        needs_layout_passes=False, use_tc_tiling_on_sc=False),
)
def _gather_kernel(qi_hbm, qf_hbm, qr_hbm, agg0_hbm, agg1_hbm, diff_hbm,
                   embd_hbm, b3_hbm, w3_hbm,
                   conc_hbm, dif_hbm, dqs_hbm, w3g_hbm, b3g_hbm,
                   qi_v, diff_v, b3_v, q_v, qr_v, idx_v, idxs_v, dr_v,
                   dqs_v, b3g_v, ca_v, cb_v, db_v, wb_v):
  wid = lax.axis_index("s") * _NC + lax.axis_index("c")

  @pl.when(wid < _GT)
  def _():
    base = wid * _GPW
    pltpu.sync_copy(qi_hbm, qi_v)
    pltpu.sync_copy(diff_hbm, diff_v)
    pltpu.sync_copy(b3_hbm, b3_v)
    pltpu.sync_copy(qf_hbm.at[pl.ds(base, _GPW)], q_v)
    pltpu.sync_copy(qr_hbm.at[pl.ds(base, _GPW)], qr_v)

    @plsc.parallel_loop(0, _GPW // 16, unroll=2)
    def ib(i):
      sl = pl.ds(i * 16, 16)
      qv = q_v[sl]
      qs = qr_v[sl]
      n16 = plsc.load_gather(qi_v, [qv])
      ns16 = plsc.load_gather(qi_v, [qs])
      idx_v[0, sl] = n16
      idxs_v[0, sl] = ns16
      dr_v[0, sl] = plsc.load_gather(diff_v, [qv])
      dqs_v[sl] = plsc.load_gather(diff_v, [qs]).astype(jnp.float32)
      b3g_v[sl] = plsc.load_gather(b3_v, [ns16])

    pltpu.sync_copy(agg0_hbm.at[idx_v.at[0]], ca_v)
    pltpu.sync_copy(agg1_hbm.at[idx_v.at[0]], cb_v)

    @plsc.parallel_loop(0, _GPW, unroll=4)
    def ab(j):
      for t in range(_D // 16):
        sl = pl.ds(t * 16, 16)
        ca_v[j, sl] = ca_v[j, sl] + cb_v[j, sl]

    pltpu.sync_copy(embd_hbm.at[dr_v.at[0]], db_v)
    pltpu.sync_copy(w3_hbm.at[idxs_v.at[0]], wb_v)

    pltpu.sync_copy(ca_v, conc_hbm.at[pl.ds(base, _GPW)])
    pltpu.sync_copy(db_v, dif_hbm.at[pl.ds(base, _GPW)])
    pltpu.sync_copy(wb_v, w3g_hbm.at[pl.ds(base, _GPW)])
    pltpu.sync_copy(dqs_v, dqs_hbm.at[pl.ds(base, _GPW)])
    pltpu.sync_copy(b3g_v, b3g_hbm.at[pl.ds(base, _GPW)])


def _dense_tc(conc, dif, yf, ea, W1, b1, W2, b2, Wih, Whh, bih, bhh, w3g3,
              b3g2, dqs2):
  def body(conc_ref, dif_ref, yf_ref, ea_ref, W1_ref, b1_ref, W2_ref, b2_ref,
           Wih_ref, Whh_ref, bih_ref, bhh_ref, w3g_ref, b3g_ref, dqs_ref,
           out_ref, res_ref):
    x1 = jnp.concatenate([conc_ref[...], dif_ref[...]], axis=1)
    text = lax.dot_general(x1, W1_ref[...], (((1,), (1,)), ((), ())),
                           preferred_element_type=jnp.float32) + b1_ref[...]
    a0 = ea_ref[0:1, :]
    a1 = ea_ref[1:2, :]
    ans = a0 + yf_ref[...] * (a1 - a0)
    x2 = jnp.concatenate([text, ans], axis=1)
    X = lax.dot_general(x2, W2_ref[...], (((1,), (1,)), ((), ())),
                        preferred_element_type=jnp.float32) + b2_ref[...]
    bgv = bih_ref[...] + bhh_ref[...]
    GX = lax.dot_general(X, Wih_ref[...], (((1,), (1,)), ((), ())),
                         preferred_element_type=jnp.float32) + bgv
    Whh = Whh_ref[...]

    h = jnp.zeros((50, _H), jnp.float32)
    c = jnp.zeros((50, _H), jnp.float32)
    hs = []
    for t in range(32):
      g = (GX[t * 50:(t + 1) * 50, :]
           + lax.dot_general(h, Whh, (((1,), (1,)), ((), ())),
                             preferred_element_type=jnp.float32))
      i_ = jax.nn.sigmoid(g[:, 0:_H])
      f_ = jax.nn.sigmoid(g[:, _H:2 * _H])
      gg = jnp.tanh(g[:, 2 * _H:3 * _H])
      o_ = jax.nn.sigmoid(g[:, 3 * _H:4 * _H])
      c = f_ * c + i_ * gg
      h = o_ * jnp.tanh(c)
      hs.append(h)

    outv = jnp.stack(hs, axis=0)
    out_ref[...] = jnp.swapaxes(outv, 0, 1)
    outc = jnp.concatenate(hs, axis=0)
    pr = jnp.sum(outc * w3g_ref[...], axis=1, keepdims=True) + b3g_ref[...]
    ev = jax.nn.sigmoid(pr)
    res_ref[...] = jax.nn.sigmoid(ev - (dqs_ref[...] * 0.2 + 0.2))

  return pl.pallas_call(
      body,
      out_shape=(
          jax.ShapeDtypeStruct((50, 32, _H), jnp.float32),
          jax.ShapeDtypeStruct((_BS, 1), jnp.float32),
      ),
  )(conc, dif, yf, ea, W1, b1, W2, b2, Wih, Whh, bih, bhh, w3g3, b3g2, dqs2)


_CT = 1024


def _e_tc(out, W3, b3r):
  def body(o_ref, w_ref, b_ref, e_ref):
    e_ref[...] = jax.nn.sigmoid(
        lax.dot_general(o_ref[...], w_ref[...], (((1,), (1,)), ((), ())),
                        preferred_element_type=jnp.float32) + b_ref[...])

  grid = pl.cdiv(_N, _CT)
  return pl.pallas_call(
      body,
      grid=(grid,),
      in_specs=[
          pl.BlockSpec((_BS, _H), lambda j: (0, 0)),
          pl.BlockSpec((_CT, _H), lambda j: (j, 0)),
          pl.BlockSpec((1, _CT), lambda j: (0, j)),
      ],
      out_specs=pl.BlockSpec((_BS, _CT), lambda j: (0, j)),
      out_shape=jax.ShapeDtypeStruct((_BS, _N), jnp.float32),
  )(out, W3, b3r)


def kernel(Q_info, edge_index, edge_type, q, y, diff, device, rgcn_weight,
           rgcn_root, rgcn_bias, emb_diff, emb_answer, W1, b1, W2, b2,
           W_ih, W_hh, b_ih, b_hh, W3, b3):
  src = edge_index[0].astype(jnp.int32)
  dst = edge_index[1].astype(jnp.int32)
  rel = edge_type.astype(jnp.int32)

  cnt_parts = _count_kernel(dst, rel)
  inv = _inv_tc(cnt_parts)

  s_all = _scale_kernel(dst, rel, inv)
  wflat = rgcn_weight.reshape(_SEG, _D)
  agg0, agg1 = _scatter_kernel(src, dst, rel, wflat, s_all, rgcn_root,
                               rgcn_bias)

  qf = q.reshape(-1).astype(jnp.int32)
  qr = jnp.roll(qf, -1)
  conc, dif, dqs, w3g, b3g = _gather_kernel(
      Q_info.astype(jnp.int32), qf, qr, agg0, agg1,
      diff.astype(jnp.int32), emb_diff, b3, W3)

  yf = y.reshape(_BS, 1).astype(jnp.float32)
  outsm, res_full = _dense_tc(
      conc, dif, yf, emb_answer, W1, b1.reshape(1, -1), W2, b2.reshape(1, -1),
      W_ih, W_hh, b_ih.reshape(1, -1), b_hh.reshape(1, -1),
      w3g, b3g.reshape(_BS, 1), dqs.reshape(_BS, 1))

  e2 = _e_tc(outsm.reshape(_BS, _H), W3, b3.reshape(1, _N))
  e = jnp.swapaxes(e2.reshape(50, 32, _N), 0, 1)

  res = res_full.reshape(32, 50)[:, :49]
  return (res, e)


# speedup vs baseline: 10.6537x; 1.0346x over previous
import functools

import jax
import jax.numpy as jnp
from jax import lax
from jax.experimental import pallas as pl
from jax.experimental.pallas import tpu as pltpu
from jax.experimental.pallas import tpu_sc as plsc

_N = 10000
_R = 4
_D = 128
_E = 160000
_SEG = _N * _R
_NC = 2
_NS = 16
_NW = _NC * _NS
_EPW = _E // _NW
_Q = 20000
_BS = 1600
_H = 256
_RPW = _N // _NS


def _mesh():
  return plsc.VectorSubcoreMesh(
      core_axis_name="c", subcore_axis_name="s",
      num_cores=_NC, num_subcores=_NS)


@functools.partial(
    pl.kernel,
    out_type=jax.ShapeDtypeStruct((_NW, _SEG), jnp.float32),
    mesh=_mesh(),
    scratch_types=[
        pltpu.VMEM((_EPW,), jnp.int32),
        pltpu.VMEM((_EPW,), jnp.int32),
        pltpu.VMEM((_SEG,), jnp.float32),
    ],
    compiler_params=pltpu.CompilerParams(
        needs_layout_passes=False, use_tc_tiling_on_sc=False),
)
def _count_kernel(dst_hbm, rel_hbm, out_hbm, dst_v, comb_v, cnt_v):
  wid = lax.axis_index("s") * _NC + lax.axis_index("c")
  base = wid * _EPW
  pltpu.sync_copy(dst_hbm.at[pl.ds(base, _EPW)], dst_v)
  pltpu.sync_copy(rel_hbm.at[pl.ds(base, _EPW)], comb_v)

  def zbody(i, _):
    cnt_v[pl.ds(i * 16, 16)] = jnp.zeros((16,), jnp.float32)
    return 0
  lax.fori_loop(0, _SEG // 16, zbody, 0)

  def cbody(i, _):
    sl = pl.ds(i * 16, 16)
    comb_v[sl] = dst_v[sl] * _R + comb_v[sl]
    return 0
  lax.fori_loop(0, _EPW // 16, cbody, 0)

  def hbody(i, _):
    c16 = comb_v[pl.ds(i * 16, 16)]
    cnts, lastm = plsc.scan_count(c16)
    plsc.addupdate_scatter(cnt_v, [c16], cnts.astype(jnp.float32), mask=lastm)
    return 0
  lax.fori_loop(0, _EPW // 16, hbody, 0)

  pltpu.sync_copy(cnt_v, out_hbm.at[wid])


def _inv_tc(cnt_parts):
  def body(c_ref, o_ref):
    s = jnp.sum(c_ref[...], axis=0, keepdims=True)
    o_ref[...] = 1.0 / jnp.maximum(s, 1.0)

  out = pl.pallas_call(
      body,
      out_shape=jax.ShapeDtypeStruct((1, _SEG), jnp.float32),
  )(cnt_parts)
  return out.reshape(_SEG)


@functools.partial(
    pl.kernel,
    out_type=jax.ShapeDtypeStruct((_E,), jnp.float32),
    mesh=_mesh(),
    scratch_types=[
        pltpu.VMEM((_EPW,), jnp.int32),
        pltpu.VMEM((_EPW,), jnp.int32),
        pltpu.VMEM((_SEG,), jnp.float32),
        pltpu.VMEM((_EPW,), jnp.float32),
    ],
    compiler_params=pltpu.CompilerParams(
        needs_layout_passes=False, use_tc_tiling_on_sc=False),
)
def _scale_kernel(dst_hbm, rel_hbm, inv_hbm, out_hbm, dst_v, rel_v, inv_v,
                  s_v):
  wid = lax.axis_index("s") * _NC + lax.axis_index("c")
  base = wid * _EPW
  pltpu.sync_copy(dst_hbm.at[pl.ds(base, _EPW)], dst_v)
  pltpu.sync_copy(rel_hbm.at[pl.ds(base, _EPW)], rel_v)
  pltpu.sync_copy(inv_hbm, inv_v)

  def body(i, _):
    sl = pl.ds(i * 16, 16)
    comb = dst_v[sl] * _R + rel_v[sl]
    s_v[sl] = plsc.load_gather(inv_v, [comb])
    return 0
  lax.fori_loop(0, _EPW // 16, body, 0)

  pltpu.sync_copy(s_v, out_hbm.at[pl.ds(base, _EPW)])


_CH = 96
_NFULL = (_EPW // _CH)
_TAIL = _EPW - _NFULL * _CH
_NPAIR = _NFULL // 2


@functools.partial(
    pl.kernel,
    out_type=(
        jax.ShapeDtypeStruct((_N, _D), jnp.float32),
        jax.ShapeDtypeStruct((_N, _D), jnp.float32),
    ),
    mesh=_mesh(),
    scratch_types=[
        pltpu.VMEM((_EPW + 16,), jnp.int32),
        pltpu.VMEM((_EPW + 16,), jnp.int32),
        pltpu.VMEM((_EPW + 16,), jnp.int32),
        pltpu.VMEM((_EPW + 16,), jnp.float32),
        pltpu.VMEM((2, _CH), jnp.int32),
        pltpu.VMEM((2, _CH), jnp.int32),
        pltpu.VMEM((_CH, _D), jnp.float32),
        pltpu.VMEM((_CH, _D), jnp.float32),
        pltpu.VMEM((_D,), jnp.float32),
        pltpu.SemaphoreType.DMA,
        pltpu.SemaphoreType.DMA,
        pltpu.VMEM_SHARED((_N, _D), jnp.float32),
    ],
    compiler_params=pltpu.CompilerParams(
        needs_layout_passes=False, use_tc_tiling_on_sc=False),
)
def _scatter_kernel(src_hbm, dst_hbm, rel_hbm, w_hbm, s_hbm, root_hbm,
                    bias_hbm, out0_hbm, out1_hbm, src_v, dst_v, rel_v, s_v,
                    gidx_v, sidx_v, rows_v, rows1_v, bias_v, g0_sem, g1_sem,
                    agg_sh):
  cid = lax.axis_index("c")
  sid = lax.axis_index("s")
  wid = sid * _NC + cid
  base = wid * _EPW

  d_src = pltpu.async_copy(src_hbm.at[pl.ds(base, _EPW)],
                           src_v.at[pl.ds(0, _EPW)], g0_sem)
  d_dst = pltpu.async_copy(dst_hbm.at[pl.ds(base, _EPW)],
                           dst_v.at[pl.ds(0, _EPW)], g0_sem)
  d_rel = pltpu.async_copy(rel_hbm.at[pl.ds(base, _EPW)],
                           rel_v.at[pl.ds(0, _EPW)], g1_sem)
  d_s = pltpu.async_copy(s_hbm.at[pl.ds(base, _EPW)],
                         s_v.at[pl.ds(0, _EPW)], g1_sem)
  pltpu.sync_copy(bias_hbm, bias_v)

  zrow = jnp.zeros((16,), jnp.float32)
  ich = [(_CH * k, _CH) for k in range(_RPW // _CH)]
  ich.append((_CH * (_RPW // _CH), _RPW - _CH * (_RPW // _CH)))

  def zero_rows(nrows):
    @plsc.parallel_loop(0, nrows, unroll=4)
    def zb(j):
      for t in range(_D // 16):
        rows_v[j, pl.ds(t * 16, 16)] = zrow

  @pl.when(cid == 1)
  def _():
    zero_rows(_CH)
    for ro, nr in ich:
      r0 = sid * _RPW + ro
      pltpu.sync_copy(rows_v.at[pl.ds(0, nr)], agg_sh.at[pl.ds(r0, nr)])

  @pl.when(cid == 0)
  def _():
    bufs = (rows_v, rows1_v)
    sems = (g0_sem, g1_sem)
    dins = {}
    r00 = sid * _RPW + ich[0][0]
    dins[0] = pltpu.async_copy(root_hbm.at[pl.ds(r00, ich[0][1])],
                               bufs[0].at[pl.ds(0, ich[0][1])], sems[0])
    for k, (ro, nr) in enumerate(ich):
      b = k % 2
      buf = bufs[b]
      r0 = sid * _RPW + ro
      dins[k].wait()
      if k + 1 < len(ich):
        ro2, nr2 = ich[k + 1]
        r02 = sid * _RPW + ro2
        dins[k + 1] = pltpu.async_copy(
            root_hbm.at[pl.ds(r02, nr2)],
            bufs[(k + 1) % 2].at[pl.ds(0, nr2)], sems[(k + 1) % 2])

      @plsc.parallel_loop(0, nr, unroll=4)
      def ab(j):
        for t in range(_D // 16):
          sl = pl.ds(t * 16, 16)
          buf[j, sl] = buf[j, sl] + bias_v[sl]
      pltpu.sync_copy(buf.at[pl.ds(0, nr)], agg_sh.at[pl.ds(r0, nr)])

  d_src.wait()
  d_dst.wait()
  d_rel.wait()
  d_s.wait()
  plsc.subcore_barrier()

  def build_idx(off, bsel):
    @plsc.parallel_loop(0, _CH // 16, unroll=3)
    def ib(i):
      sl = pl.ds(off + i * 16, 16)
      so = pl.ds(i * 16, 16)
      gidx_v[bsel, so] = rel_v[sl] * _N + src_v[sl]
      sidx_v[bsel, so] = dst_v[sl]

  def scale_rows(rv, off, nedges):
    @plsc.parallel_loop(0, nedges, unroll=8)
    def sbody(j):
      s = s_v[pl.ds(off + j, 16)][0]
      bv = jnp.full((16,), s, jnp.float32)
      for t in range(_D // 16):
        sl = pl.ds(t * 16, 16)
        rv[j, sl] = rv[j, sl] * bv

  def pair(k, _):
    off0 = (2 * k) * _CH
    off1 = (2 * k + 1) * _CH
    build_idx(off0, 0)
    d0 = pltpu.async_copy(w_hbm.at[gidx_v.at[0]], rows_v, g0_sem)
    build_idx(off1, 1)
    d1 = pltpu.async_copy(w_hbm.at[gidx_v.at[1]], rows1_v, g1_sem)
    d0.wait()
    scale_rows(rows_v, off0, _CH)
    s0 = pltpu.async_copy(rows_v, agg_sh.at[sidx_v.at[0]], g0_sem, add=True)
    d1.wait()
    scale_rows(rows1_v, off1, _CH)
    s1 = pltpu.async_copy(rows1_v, agg_sh.at[sidx_v.at[1]], g1_sem, add=True)
    s0.wait()
    s1.wait()
    return 0
  lax.fori_loop(0, _NPAIR, pair, 0)

  toff = _NFULL * _CH
  gidx_v[0, pl.ds(0, 16)] = rel_v[pl.ds(toff, 16)] * _N + src_v[pl.ds(toff, 16)]
  sidx_v[0, pl.ds(0, 16)] = dst_v[pl.ds(toff, 16)]
  pltpu.sync_copy(w_hbm.at[gidx_v.at[0, pl.ds(0, _TAIL)]],
                  rows_v.at[pl.ds(0, _TAIL)])
  scale_rows(rows_v, toff, _TAIL)
  pltpu.sync_copy(rows_v.at[pl.ds(0, _TAIL)],
                  agg_sh.at[sidx_v.at[0, pl.ds(0, _TAIL)]], add=True)

  plsc.subcore_barrier()

  def drain(out_hbm):
    bufs = (rows_v, rows1_v)
    sems = (g0_sem, g1_sem)
    dins = {}
    douts = {}
    r00 = sid * _RPW + ich[0][0]
    dins[0] = pltpu.async_copy(agg_sh.at[pl.ds(r00, ich[0][1])],
                               bufs[0].at[pl.ds(0, ich[0][1])], sems[0])
    nk = len(ich)
    for k, (ro, nr) in enumerate(ich):
      b = k % 2
      r0 = sid * _RPW + ro
      dins[k].wait()
      douts[k] = pltpu.async_copy(bufs[b].at[pl.ds(0, nr)],
                                  out_hbm.at[pl.ds(r0, nr)], sems[b])
      if k + 1 < nk:
        ob = (k + 1) % 2
        if k - 1 >= 0:
          douts[k - 1].wait()
        ro2, nr2 = ich[k + 1]
        r02 = sid * _RPW + ro2
        dins[k + 1] = pltpu.async_copy(agg_sh.at[pl.ds(r02, nr2)],
                                       bufs[ob].at[pl.ds(0, nr2)], sems[ob])
    douts[nk - 2].wait()
    douts[nk - 1].wait()

  @pl.when(cid == 0)
  def _():
    drain(out0_hbm)

  @pl.when(cid == 1)
  def _():
    drain(out1_hbm)


_GT = 25
_GPW = _BS // _GT


@functools.partial(
    pl.kernel,
    out_type=(
        jax.ShapeDtypeStruct((_BS, _D), jnp.float32),
        jax.ShapeDtypeStruct((_BS, 64), jnp.float32),
        jax.ShapeDtypeStruct((_BS,), jnp.float32),
        jax.ShapeDtypeStruct((_BS, _H), jnp.float32),
        jax.ShapeDtypeStruct((_BS,), jnp.float32),
    ),
    mesh=_mesh(),
    scratch_types=[
        pltpu.VMEM((_Q,), jnp.int32),
        pltpu.VMEM((_Q,), jnp.int32),
        pltpu.VMEM((_N,), jnp.float32),
        pltpu.VMEM((_GPW,), jnp.int32),
        pltpu.VMEM((_GPW,), jnp.int32),
        pltpu.VMEM((1, _GPW), jnp.int32),
        pltpu.VMEM((1, _GPW), jnp.int32),
        pltpu.VMEM((1, _GPW), jnp.int32),
        pltpu.VMEM((_GPW,), jnp.float32),
        pltpu.VMEM((_GPW,), jnp.float32),
        pltpu.VMEM((_GPW, _D), jnp.float32),
        pltpu.VMEM((_GPW, _D), jnp.float32),
        pltpu.VMEM((_GPW, 64), jnp.float32),
        pltpu.VMEM((_GPW, _H), jnp.float32),
    ],
    compiler_params=pltpu.CompilerParams(
        needs_layout_passes=False, use_tc_tiling_on_sc=False),
)
def _gather_kernel(qi_hbm, qf_hbm, qr_hbm, agg0_hbm, agg1_hbm, diff_hbm,
                   embd_hbm, b3_hbm, w3_hbm,
                   conc_hbm, dif_hbm, dqs_hbm, w3g_hbm, b3g_hbm,
                   qi_v, diff_v, b3_v, q_v, qr_v, idx_v, idxs_v, dr_v,
                   dqs_v, b3g_v, ca_v, cb_v, db_v, wb_v):
  wid = lax.axis_index("s") * _NC + lax.axis_index("c")

  @pl.when(wid < _GT)
  def _():
    base = wid * _GPW
    pltpu.sync_copy(qi_hbm, qi_v)
    pltpu.sync_copy(diff_hbm, diff_v)
    pltpu.sync_copy(b3_hbm, b3_v)
    pltpu.sync_copy(qf_hbm.at[pl.ds(base, _GPW)], q_v)
    pltpu.sync_copy(qr_hbm.at[pl.ds(base, _GPW)], qr_v)

    @plsc.parallel_loop(0, _GPW // 16, unroll=2)
    def ib(i):
      sl = pl.ds(i * 16, 16)
      qv = q_v[sl]
      qs = qr_v[sl]
      n16 = plsc.load_gather(qi_v, [qv])
      ns16 = plsc.load_gather(qi_v, [qs])
      idx_v[0, sl] = n16
      idxs_v[0, sl] = ns16
      dr_v[0, sl] = plsc.load_gather(diff_v, [qv])
      dqs_v[sl] = plsc.load_gather(diff_v, [qs]).astype(jnp.float32)
      b3g_v[sl] = plsc.load_gather(b3_v, [ns16])

    pltpu.sync_copy(agg0_hbm.at[idx_v.at[0]], ca_v)
    pltpu.sync_copy(agg1_hbm.at[idx_v.at[0]], cb_v)

    @plsc.parallel_loop(0, _GPW, unroll=4)
    def ab(j):
      for t in range(_D // 16):
        sl = pl.ds(t * 16, 16)
        ca_v[j, sl] = ca_v[j, sl] + cb_v[j, sl]

    pltpu.sync_copy(embd_hbm.at[dr_v.at[0]], db_v)
    pltpu.sync_copy(w3_hbm.at[idxs_v.at[0]], wb_v)

    pltpu.sync_copy(ca_v, conc_hbm.at[pl.ds(base, _GPW)])
    pltpu.sync_copy(db_v, dif_hbm.at[pl.ds(base, _GPW)])
    pltpu.sync_copy(wb_v, w3g_hbm.at[pl.ds(base, _GPW)])
    pltpu.sync_copy(dqs_v, dqs_hbm.at[pl.ds(base, _GPW)])
    pltpu.sync_copy(b3g_v, b3g_hbm.at[pl.ds(base, _GPW)])


def _dense_tc(conc, dif, yf, ea, W1, b1, W2, b2, Wih, Whh, bih, bhh, w3g3,
              b3g2, dqs2):
  def body(conc_ref, dif_ref, yf_ref, ea_ref, W1_ref, b1_ref, W2_ref, b2_ref,
           Wih_ref, Whh_ref, bih_ref, bhh_ref, w3g_ref, b3g_ref, dqs_ref,
           out_ref, res_ref):
    x1 = jnp.concatenate([conc_ref[...], dif_ref[...]], axis=1)
    text = lax.dot_general(x1, W1_ref[...], (((1,), (1,)), ((), ())),
                           preferred_element_type=jnp.float32) + b1_ref[...]
    a0 = ea_ref[0:1, :]
    a1 = ea_ref[1:2, :]
    ans = a0 + yf_ref[...] * (a1 - a0)
    x2 = jnp.concatenate([text, ans], axis=1)
    X = lax.dot_general(x2, W2_ref[...], (((1,), (1,)), ((), ())),
                        preferred_element_type=jnp.float32) + b2_ref[...]
    bgv = bih_ref[...] + bhh_ref[...]
    GX = lax.dot_general(X, Wih_ref[...], (((1,), (1,)), ((), ())),
                         preferred_element_type=jnp.float32) + bgv
    Whh = Whh_ref[...]

    h = jnp.zeros((50, _H), jnp.float32)
    c = jnp.zeros((50, _H), jnp.float32)
    hs = []
    for t in range(32):
      g = (GX[t * 50:(t + 1) * 50, :]
           + lax.dot_general(h, Whh, (((1,), (1,)), ((), ())),
                             preferred_element_type=jnp.float32))
      i_ = jax.nn.sigmoid(g[:, 0:_H])
      f_ = jax.nn.sigmoid(g[:, _H:2 * _H])
      gg = jnp.tanh(g[:, 2 * _H:3 * _H])
      o_ = jax.nn.sigmoid(g[:, 3 * _H:4 * _H])
      c = f_ * c + i_ * gg
      h = o_ * jnp.tanh(c)
      hs.append(h)

    outv = jnp.stack(hs, axis=0)
    out_ref[...] = jnp.swapaxes(outv, 0, 1)
    outc = jnp.concatenate(hs, axis=0)
    pr = jnp.sum(outc * w3g_ref[...], axis=1, keepdims=True) + b3g_ref[...]
    ev = jax.nn.sigmoid(pr)
    res_ref[...] = jax.nn.sigmoid(ev - (dqs_ref[...] * 0.2 + 0.2))

  return pl.pallas_call(
      body,
      out_shape=(
          jax.ShapeDtypeStruct((50, 32, _H), jnp.float32),
          jax.ShapeDtypeStruct((_BS, 1), jnp.float32),
      ),
  )(conc, dif, yf, ea, W1, b1, W2, b2, Wih, Whh, bih, bhh, w3g3, b3g2, dqs2)


_CT = 1024


def _e_tc(out, W3, b3r):
  def body(o_ref, w_ref, b_ref, e_ref):
    e_ref[...] = jax.nn.sigmoid(
        lax.dot_general(o_ref[...], w_ref[...], (((1,), (1,)), ((), ())),
                        preferred_element_type=jnp.float32) + b_ref[...])

  grid = pl.cdiv(_N, _CT)
  return pl.pallas_call(
      body,
      grid=(grid,),
      in_specs=[
          pl.BlockSpec((_BS, _H), lambda j: (0, 0)),
          pl.BlockSpec((_CT, _H), lambda j: (j, 0)),
          pl.BlockSpec((1, _CT), lambda j: (0, j)),
      ],
      out_specs=pl.BlockSpec((_BS, _CT), lambda j: (0, j)),
      out_shape=jax.ShapeDtypeStruct((_BS, _N), jnp.float32),
  )(out, W3, b3r)


def kernel(Q_info, edge_index, edge_type, q, y, diff, device, rgcn_weight,
           rgcn_root, rgcn_bias, emb_diff, emb_answer, W1, b1, W2, b2,
           W_ih, W_hh, b_ih, b_hh, W3, b3):
  src = edge_index[0].astype(jnp.int32)
  dst = edge_index[1].astype(jnp.int32)
  rel = edge_type.astype(jnp.int32)

  cnt_parts = _count_kernel(dst, rel)
  inv = _inv_tc(cnt_parts)

  s_all = _scale_kernel(dst, rel, inv)
  wflat = rgcn_weight.reshape(_SEG, _D)
  agg0, agg1 = _scatter_kernel(src, dst, rel, wflat, s_all, rgcn_root,
                               rgcn_bias)

  qf = q.reshape(-1).astype(jnp.int32)
  qr = jnp.roll(qf, -1)
  conc, dif, dqs, w3g, b3g = _gather_kernel(
      Q_info.astype(jnp.int32), qf, qr, agg0, agg1,
      diff.astype(jnp.int32), emb_diff, b3, W3)

  yf = y.reshape(_BS, 1).astype(jnp.float32)
  outsm, res_full = _dense_tc(
      conc, dif, yf, emb_answer, W1, b1.reshape(1, -1), W2, b2.reshape(1, -1),
      W_ih, W_hh, b_ih.reshape(1, -1), b_hh.reshape(1, -1),
      w3g, b3g.reshape(_BS, 1), dqs.reshape(_BS, 1))

  e2 = _e_tc(outsm.reshape(_BS, _H), W3, b3.reshape(1, _N))
  e = jnp.swapaxes(e2.reshape(50, 32, _N), 0, 1)

  res = res_full.reshape(32, 50)[:, :49]
  return (res, e)


# speedup vs baseline: 11.1512x; 1.0467x over previous
import functools

import jax
import jax.numpy as jnp
from jax import lax
from jax.experimental import pallas as pl
from jax.experimental.pallas import tpu as pltpu
from jax.experimental.pallas import tpu_sc as plsc

_N = 10000
_R = 4
_D = 128
_E = 160000
_SEG = _N * _R
_NC = 2
_NS = 16
_NW = _NC * _NS
_EPW = _E // _NW
_Q = 20000
_BS = 1600
_H = 256
_RPW = _N // _NS


def _mesh():
  return plsc.VectorSubcoreMesh(
      core_axis_name="c", subcore_axis_name="s",
      num_cores=_NC, num_subcores=_NS)


@functools.partial(
    pl.kernel,
    out_type=jax.ShapeDtypeStruct((_NW, _SEG), jnp.float32),
    mesh=_mesh(),
    scratch_types=[
        pltpu.VMEM((_EPW,), jnp.int32),
        pltpu.VMEM((_EPW,), jnp.int32),
        pltpu.VMEM((_SEG,), jnp.float32),
        pltpu.SemaphoreType.DMA,
        pltpu.SemaphoreType.DMA,
    ],
    compiler_params=pltpu.CompilerParams(
        needs_layout_passes=False, use_tc_tiling_on_sc=False),
)
def _count_kernel(dst_hbm, rel_hbm, out_hbm, dst_v, comb_v, cnt_v, s0, s1):
  wid = lax.axis_index("s") * _NC + lax.axis_index("c")
  base = wid * _EPW
  d0 = pltpu.async_copy(dst_hbm.at[pl.ds(base, _EPW)], dst_v, s0)
  d1 = pltpu.async_copy(rel_hbm.at[pl.ds(base, _EPW)], comb_v, s1)

  zrow16 = jnp.zeros((16,), jnp.float32)

  @plsc.parallel_loop(0, _SEG // 16, unroll=8)
  def zbody(i):
    cnt_v[pl.ds(i * 16, 16)] = zrow16

  d0.wait()
  d1.wait()

  @plsc.parallel_loop(0, _EPW // 16, unroll=4)
  def cbody(i):
    sl = pl.ds(i * 16, 16)
    comb_v[sl] = dst_v[sl] * _R + comb_v[sl]

  def hbody(i, _):
    c16 = comb_v[pl.ds(i * 16, 16)]
    cnts, lastm = plsc.scan_count(c16)
    plsc.addupdate_scatter(cnt_v, [c16], cnts.astype(jnp.float32), mask=lastm)
    return 0
  lax.fori_loop(0, _EPW // 16, hbody, 0)

  pltpu.sync_copy(cnt_v, out_hbm.at[wid])


def _inv_tc(cnt_parts):
  def body(c_ref, o_ref):
    s = jnp.sum(c_ref[...], axis=0, keepdims=True)
    o_ref[...] = 1.0 / jnp.maximum(s, 1.0)

  out = pl.pallas_call(
      body,
      out_shape=jax.ShapeDtypeStruct((1, _SEG), jnp.float32),
  )(cnt_parts)
  return out.reshape(_SEG)


@functools.partial(
    pl.kernel,
    out_type=jax.ShapeDtypeStruct((_E,), jnp.float32),
    mesh=_mesh(),
    scratch_types=[
        pltpu.VMEM((_EPW,), jnp.int32),
        pltpu.VMEM((_EPW,), jnp.int32),
        pltpu.VMEM((_SEG,), jnp.float32),
        pltpu.VMEM((_EPW,), jnp.float32),
    ],
    compiler_params=pltpu.CompilerParams(
        needs_layout_passes=False, use_tc_tiling_on_sc=False),
)
def _scale_kernel(dst_hbm, rel_hbm, inv_hbm, out_hbm, dst_v, rel_v, inv_v,
                  s_v):
  wid = lax.axis_index("s") * _NC + lax.axis_index("c")
  base = wid * _EPW
  pltpu.sync_copy(dst_hbm.at[pl.ds(base, _EPW)], dst_v)
  pltpu.sync_copy(rel_hbm.at[pl.ds(base, _EPW)], rel_v)
  pltpu.sync_copy(inv_hbm, inv_v)

  def body(i, _):
    sl = pl.ds(i * 16, 16)
    comb = dst_v[sl] * _R + rel_v[sl]
    s_v[sl] = plsc.load_gather(inv_v, [comb])
    return 0
  lax.fori_loop(0, _EPW // 16, body, 0)

  pltpu.sync_copy(s_v, out_hbm.at[pl.ds(base, _EPW)])


_CH = 96
_NFULL = (_EPW // _CH)
_TAIL = _EPW - _NFULL * _CH
_NPAIR = _NFULL // 2


@functools.partial(
    pl.kernel,
    out_type=(
        jax.ShapeDtypeStruct((_N, _D), jnp.float32),
        jax.ShapeDtypeStruct((_N, _D), jnp.float32),
    ),
    mesh=_mesh(),
    scratch_types=[
        pltpu.VMEM((_EPW + 16,), jnp.int32),
        pltpu.VMEM((_EPW + 16,), jnp.int32),
        pltpu.VMEM((_EPW + 16,), jnp.int32),
        pltpu.VMEM((_EPW + 16,), jnp.float32),
        pltpu.VMEM((2, _CH), jnp.int32),
        pltpu.VMEM((2, _CH), jnp.int32),
        pltpu.VMEM((_CH, _D), jnp.float32),
        pltpu.VMEM((_CH, _D), jnp.float32),
        pltpu.VMEM((_D,), jnp.float32),
        pltpu.SemaphoreType.DMA,
        pltpu.SemaphoreType.DMA,
        pltpu.VMEM_SHARED((_N, _D), jnp.float32),
    ],
    compiler_params=pltpu.CompilerParams(
        needs_layout_passes=False, use_tc_tiling_on_sc=False),
)
def _scatter_kernel(src_hbm, dst_hbm, rel_hbm, w_hbm, s_hbm, root_hbm,
                    bias_hbm, out0_hbm, out1_hbm, src_v, dst_v, rel_v, s_v,
                    gidx_v, sidx_v, rows_v, rows1_v, bias_v, g0_sem, g1_sem,
                    agg_sh):
  cid = lax.axis_index("c")
  sid = lax.axis_index("s")
  wid = sid * _NC + cid
  base = wid * _EPW

  d_src = pltpu.async_copy(src_hbm.at[pl.ds(base, _EPW)],
                           src_v.at[pl.ds(0, _EPW)], g0_sem)
  d_dst = pltpu.async_copy(dst_hbm.at[pl.ds(base, _EPW)],
                           dst_v.at[pl.ds(0, _EPW)], g0_sem)
  d_rel = pltpu.async_copy(rel_hbm.at[pl.ds(base, _EPW)],
                           rel_v.at[pl.ds(0, _EPW)], g1_sem)
  d_s = pltpu.async_copy(s_hbm.at[pl.ds(base, _EPW)],
                         s_v.at[pl.ds(0, _EPW)], g1_sem)
  pltpu.sync_copy(bias_hbm, bias_v)

  zrow = jnp.zeros((16,), jnp.float32)
  ich = [(_CH * k, _CH) for k in range(_RPW // _CH)]
  ich.append((_CH * (_RPW // _CH), _RPW - _CH * (_RPW // _CH)))

  def zero_rows(nrows):
    @plsc.parallel_loop(0, nrows, unroll=4)
    def zb(j):
      for t in range(_D // 16):
        rows_v[j, pl.ds(t * 16, 16)] = zrow

  @pl.when(cid == 1)
  def _():
    zero_rows(_CH)
    for ro, nr in ich:
      r0 = sid * _RPW + ro
      pltpu.sync_copy(rows_v.at[pl.ds(0, nr)], agg_sh.at[pl.ds(r0, nr)])

  @pl.when(cid == 0)
  def _():
    bufs = (rows_v, rows1_v)
    sems = (g0_sem, g1_sem)
    dins = {}
    r00 = sid * _RPW + ich[0][0]
    dins[0] = pltpu.async_copy(root_hbm.at[pl.ds(r00, ich[0][1])],
                               bufs[0].at[pl.ds(0, ich[0][1])], sems[0])
    for k, (ro, nr) in enumerate(ich):
      b = k % 2
      buf = bufs[b]
      r0 = sid * _RPW + ro
      dins[k].wait()
      if k + 1 < len(ich):
        ro2, nr2 = ich[k + 1]
        r02 = sid * _RPW + ro2
        dins[k + 1] = pltpu.async_copy(
            root_hbm.at[pl.ds(r02, nr2)],
            bufs[(k + 1) % 2].at[pl.ds(0, nr2)], sems[(k + 1) % 2])

      @plsc.parallel_loop(0, nr, unroll=4)
      def ab(j):
        for t in range(_D // 16):
          sl = pl.ds(t * 16, 16)
          buf[j, sl] = buf[j, sl] + bias_v[sl]
      pltpu.sync_copy(buf.at[pl.ds(0, nr)], agg_sh.at[pl.ds(r0, nr)])

  d_src.wait()
  d_dst.wait()
  d_rel.wait()
  d_s.wait()
  plsc.subcore_barrier()

  def build_idx(off, bsel):
    @plsc.parallel_loop(0, _CH // 16, unroll=3)
    def ib(i):
      sl = pl.ds(off + i * 16, 16)
      so = pl.ds(i * 16, 16)
      gidx_v[bsel, so] = rel_v[sl] * _N + src_v[sl]
      sidx_v[bsel, so] = dst_v[sl]

  def scale_rows(rv, off, nedges):
    @plsc.parallel_loop(0, nedges, unroll=8)
    def sbody(j):
      s = s_v[pl.ds(off + j, 16)][0]
      bv = jnp.full((16,), s, jnp.float32)
      for t in range(_D // 16):
        sl = pl.ds(t * 16, 16)
        rv[j, sl] = rv[j, sl] * bv

  def pair(k, _):
    off0 = (2 * k) * _CH
    off1 = (2 * k + 1) * _CH
    build_idx(off0, 0)
    d0 = pltpu.async_copy(w_hbm.at[gidx_v.at[0]], rows_v, g0_sem)
    build_idx(off1, 1)
    d1 = pltpu.async_copy(w_hbm.at[gidx_v.at[1]], rows1_v, g1_sem)
    d0.wait()
    scale_rows(rows_v, off0, _CH)
    s0 = pltpu.async_copy(rows_v, agg_sh.at[sidx_v.at[0]], g0_sem, add=True)
    d1.wait()
    scale_rows(rows1_v, off1, _CH)
    s1 = pltpu.async_copy(rows1_v, agg_sh.at[sidx_v.at[1]], g1_sem, add=True)
    s0.wait()
    s1.wait()
    return 0
  lax.fori_loop(0, _NPAIR, pair, 0)

  toff = _NFULL * _CH
  gidx_v[0, pl.ds(0, 16)] = rel_v[pl.ds(toff, 16)] * _N + src_v[pl.ds(toff, 16)]
  sidx_v[0, pl.ds(0, 16)] = dst_v[pl.ds(toff, 16)]
  pltpu.sync_copy(w_hbm.at[gidx_v.at[0, pl.ds(0, _TAIL)]],
                  rows_v.at[pl.ds(0, _TAIL)])
  scale_rows(rows_v, toff, _TAIL)
  pltpu.sync_copy(rows_v.at[pl.ds(0, _TAIL)],
                  agg_sh.at[sidx_v.at[0, pl.ds(0, _TAIL)]], add=True)

  plsc.subcore_barrier()

  def drain(out_hbm):
    bufs = (rows_v, rows1_v)
    sems = (g0_sem, g1_sem)
    dins = {}
    douts = {}
    r00 = sid * _RPW + ich[0][0]
    dins[0] = pltpu.async_copy(agg_sh.at[pl.ds(r00, ich[0][1])],
                               bufs[0].at[pl.ds(0, ich[0][1])], sems[0])
    nk = len(ich)
    for k, (ro, nr) in enumerate(ich):
      b = k % 2
      r0 = sid * _RPW + ro
      dins[k].wait()
      douts[k] = pltpu.async_copy(bufs[b].at[pl.ds(0, nr)],
                                  out_hbm.at[pl.ds(r0, nr)], sems[b])
      if k + 1 < nk:
        ob = (k + 1) % 2
        if k - 1 >= 0:
          douts[k - 1].wait()
        ro2, nr2 = ich[k + 1]
        r02 = sid * _RPW + ro2
        dins[k + 1] = pltpu.async_copy(agg_sh.at[pl.ds(r02, nr2)],
                                       bufs[ob].at[pl.ds(0, nr2)], sems[ob])
    douts[nk - 2].wait()
    douts[nk - 1].wait()

  @pl.when(cid == 0)
  def _():
    drain(out0_hbm)

  @pl.when(cid == 1)
  def _():
    drain(out1_hbm)


_GT = 25
_GPW = _BS // _GT


@functools.partial(
    pl.kernel,
    out_type=(
        jax.ShapeDtypeStruct((_BS, _D), jnp.float32),
        jax.ShapeDtypeStruct((_BS, 64), jnp.float32),
        jax.ShapeDtypeStruct((_BS,), jnp.float32),
        jax.ShapeDtypeStruct((_BS, _H), jnp.float32),
        jax.ShapeDtypeStruct((_BS,), jnp.float32),
    ),
    mesh=_mesh(),
    scratch_types=[
        pltpu.VMEM((_Q,), jnp.int32),
        pltpu.VMEM((_Q,), jnp.int32),
        pltpu.VMEM((_N,), jnp.float32),
        pltpu.VMEM((_GPW,), jnp.int32),
        pltpu.VMEM((_GPW,), jnp.int32),
        pltpu.VMEM((1, _GPW), jnp.int32),
        pltpu.VMEM((1, _GPW), jnp.int32),
        pltpu.VMEM((1, _GPW), jnp.int32),
        pltpu.VMEM((_GPW,), jnp.float32),
        pltpu.VMEM((_GPW,), jnp.float32),
        pltpu.VMEM((_GPW, _D), jnp.float32),
        pltpu.VMEM((_GPW, _D), jnp.float32),
        pltpu.VMEM((_GPW, 64), jnp.float32),
        pltpu.VMEM((_GPW, _H), jnp.float32),
    ],
    compiler_params=pltpu.CompilerParams(
        needs_layout_passes=False, use_tc_tiling_on_sc=False),
)
def _gather_kernel(qi_hbm, qf_hbm, qr_hbm, agg0_hbm, agg1_hbm, diff_hbm,
                   embd_hbm, b3_hbm, w3_hbm,
                   conc_hbm, dif_hbm, dqs_hbm, w3g_hbm, b3g_hbm,
                   qi_v, diff_v, b3_v, q_v, qr_v, idx_v, idxs_v, dr_v,
                   dqs_v, b3g_v, ca_v, cb_v, db_v, wb_v):
  wid = lax.axis_index("s") * _NC + lax.axis_index("c")

  @pl.when(wid < _GT)
  def _():
    base = wid * _GPW
    pltpu.sync_copy(qi_hbm, qi_v)
    pltpu.sync_copy(diff_hbm, diff_v)
    pltpu.sync_copy(b3_hbm, b3_v)
    pltpu.sync_copy(qf_hbm.at[pl.ds(base, _GPW)], q_v)
    pltpu.sync_copy(qr_hbm.at[pl.ds(base, _GPW)], qr_v)

    @plsc.parallel_loop(0, _GPW // 16, unroll=2)
    def ib(i):
      sl = pl.ds(i * 16, 16)
      qv = q_v[sl]
      qs = qr_v[sl]
      n16 = plsc.load_gather(qi_v, [qv])
      ns16 = plsc.load_gather(qi_v, [qs])
      idx_v[0, sl] = n16
      idxs_v[0, sl] = ns16
      dr_v[0, sl] = plsc.load_gather(diff_v, [qv])
      dqs_v[sl] = plsc.load_gather(diff_v, [qs]).astype(jnp.float32)
      b3g_v[sl] = plsc.load_gather(b3_v, [ns16])

    pltpu.sync_copy(agg0_hbm.at[idx_v.at[0]], ca_v)
    pltpu.sync_copy(agg1_hbm.at[idx_v.at[0]], cb_v)

    @plsc.parallel_loop(0, _GPW, unroll=4)
    def ab(j):
      for t in range(_D // 16):
        sl = pl.ds(t * 16, 16)
        ca_v[j, sl] = ca_v[j, sl] + cb_v[j, sl]

    pltpu.sync_copy(embd_hbm.at[dr_v.at[0]], db_v)
    pltpu.sync_copy(w3_hbm.at[idxs_v.at[0]], wb_v)

    pltpu.sync_copy(ca_v, conc_hbm.at[pl.ds(base, _GPW)])
    pltpu.sync_copy(db_v, dif_hbm.at[pl.ds(base, _GPW)])
    pltpu.sync_copy(wb_v, w3g_hbm.at[pl.ds(base, _GPW)])
    pltpu.sync_copy(dqs_v, dqs_hbm.at[pl.ds(base, _GPW)])
    pltpu.sync_copy(b3g_v, b3g_hbm.at[pl.ds(base, _GPW)])


def _dense_tc(conc, dif, yf, ea, W1, b1, W2, b2, Wih, Whh, bih, bhh, w3g3,
              b3g2, dqs2):
  def body(conc_ref, dif_ref, yf_ref, ea_ref, W1_ref, b1_ref, W2_ref, b2_ref,
           Wih_ref, Whh_ref, bih_ref, bhh_ref, w3g_ref, b3g_ref, dqs_ref,
           out_ref, res_ref):
    x1 = jnp.concatenate([conc_ref[...], dif_ref[...]], axis=1)
    text = lax.dot_general(x1, W1_ref[...], (((1,), (1,)), ((), ())),
                           preferred_element_type=jnp.float32) + b1_ref[...]
    a0 = ea_ref[0:1, :]
    a1 = ea_ref[1:2, :]
    ans = a0 + yf_ref[...] * (a1 - a0)
    x2 = jnp.concatenate([text, ans], axis=1)
    X = lax.dot_general(x2, W2_ref[...], (((1,), (1,)), ((), ())),
                        preferred_element_type=jnp.float32) + b2_ref[...]
    bgv = bih_ref[...] + bhh_ref[...]
    GX = lax.dot_general(X, Wih_ref[...], (((1,), (1,)), ((), ())),
                         preferred_element_type=jnp.float32) + bgv
    Whh = Whh_ref[...]

    h = jnp.zeros((50, _H), jnp.float32)
    c = jnp.zeros((50, _H), jnp.float32)
    hs = []
    for t in range(32):
      g = (GX[t * 50:(t + 1) * 50, :]
           + lax.dot_general(h, Whh, (((1,), (1,)), ((), ())),
                             preferred_element_type=jnp.float32))
      i_ = jax.nn.sigmoid(g[:, 0:_H])
      f_ = jax.nn.sigmoid(g[:, _H:2 * _H])
      gg = jnp.tanh(g[:, 2 * _H:3 * _H])
      o_ = jax.nn.sigmoid(g[:, 3 * _H:4 * _H])
      c = f_ * c + i_ * gg
      h = o_ * jnp.tanh(c)
      hs.append(h)

    outv = jnp.stack(hs, axis=0)
    out_ref[...] = jnp.swapaxes(outv, 0, 1)
    outc = jnp.concatenate(hs, axis=0)
    pr = jnp.sum(outc * w3g_ref[...], axis=1, keepdims=True) + b3g_ref[...]
    ev = jax.nn.sigmoid(pr)
    res_ref[...] = jax.nn.sigmoid(ev - (dqs_ref[...] * 0.2 + 0.2))

  return pl.pallas_call(
      body,
      out_shape=(
          jax.ShapeDtypeStruct((50, 32, _H), jnp.float32),
          jax.ShapeDtypeStruct((_BS, 1), jnp.float32),
      ),
  )(conc, dif, yf, ea, W1, b1, W2, b2, Wih, Whh, bih, bhh, w3g3, b3g2, dqs2)


_CT = 1024


def _e_tc(out, W3, b3r):
  def body(o_ref, w_ref, b_ref, e_ref):
    e_ref[...] = jax.nn.sigmoid(
        lax.dot_general(o_ref[...], w_ref[...], (((1,), (1,)), ((), ())),
                        preferred_element_type=jnp.float32) + b_ref[...])

  grid = pl.cdiv(_N, _CT)
  return pl.pallas_call(
      body,
      grid=(grid,),
      in_specs=[
          pl.BlockSpec((_BS, _H), lambda j: (0, 0)),
          pl.BlockSpec((_CT, _H), lambda j: (j, 0)),
          pl.BlockSpec((1, _CT), lambda j: (0, j)),
      ],
      out_specs=pl.BlockSpec((_BS, _CT), lambda j: (0, j)),
      out_shape=jax.ShapeDtypeStruct((_BS, _N), jnp.float32),
  )(out, W3, b3r)


def kernel(Q_info, edge_index, edge_type, q, y, diff, device, rgcn_weight,
           rgcn_root, rgcn_bias, emb_diff, emb_answer, W1, b1, W2, b2,
           W_ih, W_hh, b_ih, b_hh, W3, b3):
  src = edge_index[0].astype(jnp.int32)
  dst = edge_index[1].astype(jnp.int32)
  rel = edge_type.astype(jnp.int32)

  cnt_parts = _count_kernel(dst, rel)
  inv = _inv_tc(cnt_parts)

  s_all = _scale_kernel(dst, rel, inv)
  wflat = rgcn_weight.reshape(_SEG, _D)
  agg0, agg1 = _scatter_kernel(src, dst, rel, wflat, s_all, rgcn_root,
                               rgcn_bias)

  qf = q.reshape(-1).astype(jnp.int32)
  qr = jnp.roll(qf, -1)
  conc, dif, dqs, w3g, b3g = _gather_kernel(
      Q_info.astype(jnp.int32), qf, qr, agg0, agg1,
      diff.astype(jnp.int32), emb_diff, b3, W3)

  yf = y.reshape(_BS, 1).astype(jnp.float32)
  outsm, res_full = _dense_tc(
      conc, dif, yf, emb_answer, W1, b1.reshape(1, -1), W2, b2.reshape(1, -1),
      W_ih, W_hh, b_ih.reshape(1, -1), b_hh.reshape(1, -1),
      w3g, b3g.reshape(_BS, 1), dqs.reshape(_BS, 1))

  e2 = _e_tc(outsm.reshape(_BS, _H), W3, b3.reshape(1, _N))
  e = jnp.swapaxes(e2.reshape(50, 32, _N), 0, 1)

  res = res_full.reshape(32, 50)[:, :49]
  return (res, e)
